# baseline (speedup 1.0000x reference)
import functools
import jax
import jax.numpy as jnp
from jax.experimental import pallas as pl
from jax.experimental.pallas import tpu as pltpu

N = 10000
E = 160000
F_IN = 256
HID = 256
NC = 128

MBLK = 400


def _mm_body(x_ref, w_ref, o_ref):
    o_ref[...] = jnp.dot(x_ref[...], w_ref[...],
                         preferred_element_type=jnp.float32)


def _mm(x, w):
    m, k = x.shape
    n = w.shape[1]
    grid = (m // MBLK,)
    return pl.pallas_call(
        _mm_body,
        grid=grid,
        in_specs=[
            pl.BlockSpec((MBLK, k), lambda i: (i, 0)),
            pl.BlockSpec((k, n), lambda i: (0, 0)),
        ],
        out_specs=pl.BlockSpec((MBLK, n), lambda i: (i, 0)),
        out_shape=jax.ShapeDtypeStruct((m, n), jnp.float32),
    )(x, w)


def _segment_softmax(alpha, dst, n):
    amax = jax.ops.segment_max(alpha, dst, num_segments=n)
    amax = jnp.where(jnp.isfinite(amax), amax, 0.0)
    ex = jnp.exp(alpha - amax[dst])
    denom = jax.ops.segment_sum(ex, dst, num_segments=n)
    return ex / (denom[dst] + 1e-16)


def _gat_conv(x, ei, W, att_src, att_dst, bias, heads, out_ch, concat):
    n = x.shape[0]
    h = _mm(x, W).reshape(n, heads, out_ch)
    a_src = (h * att_src[None]).sum(-1)
    a_dst = (h * att_dst[None]).sum(-1)
    loop = jnp.arange(n, dtype=ei.dtype)
    ei = jnp.concatenate([ei, jnp.stack([loop, loop])], axis=1)
    src, dst = ei[0], ei[1]
    alpha = jax.nn.leaky_relu(a_src[src] + a_dst[dst], 0.2)
    alpha = _segment_softmax(alpha, dst, n)
    msg = h[src] * alpha[..., None]
    out = jax.ops.segment_sum(msg, dst, num_segments=n)
    if concat:
        out = out.reshape(n, heads * out_ch)
    else:
        out = out.mean(axis=1)
    return out + bias


def _transformer_conv(x, ei, Wq, bq, Wk, bk, Wv, bv, Wskip, bskip, heads, out_ch):
    n = x.shape[0]
    q = (_mm(x, Wq) + bq).reshape(n, heads, out_ch)
    k = (_mm(x, Wk) + bk).reshape(n, heads, out_ch)
    v = (_mm(x, Wv) + bv).reshape(n, heads, out_ch)
    src, dst = ei[0], ei[1]
    alpha = (q[dst] * k[src]).sum(-1) / jnp.sqrt(float(out_ch))
    alpha = _segment_softmax(alpha, dst, n)
    msg = v[src] * alpha[..., None]
    out = jax.ops.segment_sum(msg, dst, num_segments=n).reshape(n, heads * out_ch)
    return out + (_mm(x, Wskip) + bskip)


def kernel(x, edge_index, W1, att_src1, att_dst1, b1, W2, att_src2, att_dst2,
           b2, Wq, bq, Wk, bk, Wv, bv, Wskip, bskip, Wres, bres):
    x1 = jax.nn.elu(_gat_conv(x, edge_index, W1, att_src1, att_dst1, b1,
                              8, HID // 8, True))
    x2 = jax.nn.elu(_gat_conv(x1, edge_index, W2, att_src2, att_dst2, b2,
                              1, HID, False))
    x3 = _transformer_conv(x2, edge_index, Wq, bq, Wk, bk, Wv, bv,
                           Wskip, bskip, 1, NC)
    res = _mm(x, Wres) + bres
    return jnp.tanh(x3 + res)


# speedup vs baseline: 9.9712x; 9.9712x over previous
import jax
import jax.numpy as jnp
from jax import lax
from jax.experimental import pallas as pl
from jax.experimental.pallas import tpu as pltpu
from jax.experimental.pallas import tpu_sc as plsc

N = 10000
E = 160000
F_IN = 256
HID = 256
NC = 128

MBLK = 400
GRID = N // MBLK
CH = 128
CH8 = CH // 8
NCHUNK = E // CH
NSUB = 16
NCORE = 2
NW = NCORE * NSUB
RA = 640
RATAIL = N - (NSUB - 1) * RA
CHA = 64
CHA8 = CHA // 8
NCHUNKA = E // CHA
EX_ROWS = E // 8

_mesh = plsc.VectorSubcoreMesh(core_axis_name="c", subcore_axis_name="s")
_sc_params = pltpu.CompilerParams(needs_layout_passes=False)


def _k1_body(x_ref, w_ref, a_ref, ha_ref, hb_ref, a1_ref):
    h = jnp.dot(x_ref[...], w_ref[...], preferred_element_type=jnp.float32)
    ha_ref[...] = h[:, :128]
    hb_ref[...] = h[:, 128:]
    a1_ref[...] = jnp.dot(h, a_ref[...], preferred_element_type=jnp.float32)


def _k1(x, W1, A1):
    return pl.pallas_call(
        _k1_body,
        grid=(GRID,),
        in_specs=[
            pl.BlockSpec((MBLK, F_IN), lambda i: (i, 0)),
            pl.BlockSpec((F_IN, HID), lambda i: (0, 0)),
            pl.BlockSpec((HID, 128), lambda i: (0, 0)),
        ],
        out_specs=[
            pl.BlockSpec((MBLK, 128), lambda i: (i, 0)),
            pl.BlockSpec((MBLK, 128), lambda i: (i, 0)),
            pl.BlockSpec((MBLK, 128), lambda i: (i, 0)),
        ],
        out_shape=[
            jax.ShapeDtypeStruct((N, 128), jnp.float32),
            jax.ShapeDtypeStruct((N, 128), jnp.float32),
            jax.ShapeDtypeStruct((N, 128), jnp.float32),
        ],
    )(x, W1, A1)


def _elu(v):
    return jnp.where(v > 0, v, jnp.exp(v) - 1.0)


def _k2_body(sa_ref, sb_ref, ha_ref, hb_ref, exs_ref, inv_ref, b_ref,
             w_ref, a_ref, h2a_ref, h2b_ref, a2_ref):
    scat = jnp.concatenate([sa_ref[...], sb_ref[...]], axis=1)
    h = jnp.concatenate([ha_ref[...], hb_ref[...]], axis=1)
    x1 = _elu((scat + h * exs_ref[...]) * inv_ref[...] + b_ref[0:1, :])
    h2 = jnp.dot(x1, w_ref[...], preferred_element_type=jnp.float32)
    h2a_ref[...] = h2[:, :128]
    h2b_ref[...] = h2[:, 128:]
    a2_ref[...] = jnp.dot(h2, a_ref[...], preferred_element_type=jnp.float32)


def _k2(sa, sb, ha, hb, exsx, invx, bpad, W2, A2):
    return pl.pallas_call(
        _k2_body,
        grid=(GRID,),
        in_specs=[
            pl.BlockSpec((MBLK, 128), lambda i: (i, 0)),
            pl.BlockSpec((MBLK, 128), lambda i: (i, 0)),
            pl.BlockSpec((MBLK, 128), lambda i: (i, 0)),
            pl.BlockSpec((MBLK, 128), lambda i: (i, 0)),
            pl.BlockSpec((MBLK, HID), lambda i: (i, 0)),
            pl.BlockSpec((MBLK, HID), lambda i: (i, 0)),
            pl.BlockSpec((8, HID), lambda i: (0, 0)),
            pl.BlockSpec((HID, HID), lambda i: (0, 0)),
            pl.BlockSpec((HID, 128), lambda i: (0, 0)),
        ],
        out_specs=[
            pl.BlockSpec((MBLK, 128), lambda i: (i, 0)),
            pl.BlockSpec((MBLK, 128), lambda i: (i, 0)),
            pl.BlockSpec((MBLK, 128), lambda i: (i, 0)),
        ],
        out_shape=[
            jax.ShapeDtypeStruct((N, 128), jnp.float32),
            jax.ShapeDtypeStruct((N, 128), jnp.float32),
            jax.ShapeDtypeStruct((N, 128), jnp.float32),
        ],
    )(sa, sb, ha, hb, exsx, invx, bpad, W2, A2)


def _k3_body(sa_ref, sb_ref, ha_ref, hb_ref, exs_ref, inv_ref, b_ref,
             w_ref, bc_ref, q_ref, k_ref, v_ref, skip_ref):
    scat = jnp.concatenate([sa_ref[...], sb_ref[...]], axis=1)
    h = jnp.concatenate([ha_ref[...], hb_ref[...]], axis=1)
    x2 = _elu((scat + h * exs_ref[...]) * inv_ref[...] + b_ref[0:1, :])
    y = jnp.dot(x2, w_ref[...], preferred_element_type=jnp.float32)
    y = y + bc_ref[0:1, :]
    q_ref[...] = y[:, 0:128]
    k_ref[...] = y[:, 128:256]
    v_ref[...] = y[:, 256:384]
    skip_ref[...] = y[:, 384:512]


def _k3(sa, sb, ha, hb, exsx, invx, bpad, Wcat, bcat):
    return pl.pallas_call(
        _k3_body,
        grid=(GRID,),
        in_specs=[
            pl.BlockSpec((MBLK, 128), lambda i: (i, 0)),
            pl.BlockSpec((MBLK, 128), lambda i: (i, 0)),
            pl.BlockSpec((MBLK, 128), lambda i: (i, 0)),
            pl.BlockSpec((MBLK, 128), lambda i: (i, 0)),
            pl.BlockSpec((MBLK, HID), lambda i: (i, 0)),
            pl.BlockSpec((MBLK, HID), lambda i: (i, 0)),
            pl.BlockSpec((8, HID), lambda i: (0, 0)),
            pl.BlockSpec((HID, 512), lambda i: (0, 0)),
            pl.BlockSpec((8, 512), lambda i: (0, 0)),
        ],
        out_specs=[pl.BlockSpec((MBLK, 128), lambda i: (i, 0))] * 4,
        out_shape=[jax.ShapeDtypeStruct((N, 128), jnp.float32)] * 4,
    )(sa, sb, ha, hb, exsx, invx, bpad, Wcat, bcat)


def _k4_body(sa_ref, sb_ref, inv_ref, skip_ref, x_ref, w_ref, b_ref, o_ref):
    acc = jnp.dot(x_ref[...], w_ref[...], preferred_element_type=jnp.float32)
    x3 = jnp.concatenate([sa_ref[...][:, 0:64], sb_ref[...][:, 0:64]], axis=1)
    o_ref[...] = jnp.tanh(x3 * inv_ref[...]
                          + skip_ref[...] + acc + b_ref[0:1, :])


def _k4(sa, sb, invx, skip, x, Wres, bpad):
    return pl.pallas_call(
        _k4_body,
        grid=(GRID,),
        in_specs=[
            pl.BlockSpec((MBLK, 128), lambda i: (i, 0)),
            pl.BlockSpec((MBLK, 128), lambda i: (i, 0)),
            pl.BlockSpec((MBLK, 128), lambda i: (i, 0)),
            pl.BlockSpec((MBLK, 128), lambda i: (i, 0)),
            pl.BlockSpec((MBLK, F_IN), lambda i: (i, 0)),
            pl.BlockSpec((F_IN, 128), lambda i: (0, 0)),
            pl.BlockSpec((8, 128), lambda i: (0, 0)),
        ],
        out_specs=pl.BlockSpec((MBLK, 128), lambda i: (i, 0)),
        out_shape=jax.ShapeDtypeStruct((N, 128), jnp.float32),
    )(sa, sb, invx, skip, x, Wres, bpad)


def _zero_vbuf(vbuf, rows):
    z = jnp.zeros((16,), jnp.float32)
    nv = vbuf.shape[1] // 16

    def zr(r, _):
        for j in range(nv):
            vbuf[r, pl.ds(j * 16, 16)] = z
        return 0

    lax.fori_loop(0, rows, zr, 0)


def _zero_acc(zb, acc_sh, sub):
    zr = zb.shape[0]
    _zero_vbuf(zb, zr)

    @pl.when(sub < NSUB - 1)
    def _():
        for k in range(RA // zr):
            pltpu.sync_copy(zb, acc_sh.at[pl.ds(sub * RA + k * zr, zr)])

    @pl.when(sub == NSUB - 1)
    def _():
        for k in range(RATAIL // zr):
            pltpu.sync_copy(zb, acc_sh.at[pl.ds((NSUB - 1) * RA + k * zr,
                                                zr)])
        rem = RATAIL % zr
        if rem:
            pltpu.sync_copy(zb.at[pl.ds(0, rem)],
                            acc_sh.at[pl.ds(N - rem, rem)])


def _drain_acc(acc_sh, out_hbm, core, sub, sem):
    @pl.when(sub < NSUB - 1)
    def _():
        pltpu.async_copy(acc_sh.at[pl.ds(sub * RA, RA)],
                         out_hbm.at[core, pl.ds(sub * RA, RA)], sem).wait()

    @pl.when(sub == NSUB - 1)
    def _():
        pltpu.async_copy(acc_sh.at[pl.ds((NSUB - 1) * RA, RATAIL)],
                         out_hbm.at[core, pl.ds((NSUB - 1) * RA, RATAIL)],
                         sem).wait()


def _sc_a_body(src_hbm, dst_hbm, t_hbm, ex_hbm, den_hbm,
               sidx, didx, rs, rd, exs, bufex, acc_sh, sem):
    core = lax.axis_index("c")
    sub = lax.axis_index("s")
    wid = core * NSUB + sub
    _zero_acc(rs, acc_sh, sub)
    _zero_vbuf(exs, CHA)
    plsc.subcore_barrier()

    trips = (NCHUNKA - wid + NW - 1) // NW

    def chunk(kk, _):
        ci = wid + kk * NW
        base = ci * CHA
        pltpu.async_copy(src_hbm.at[pl.ds(base, CHA)], sidx, sem).wait()
        pltpu.async_copy(dst_hbm.at[pl.ds(base, CHA)], didx, sem).wait()
        pltpu.async_copy(t_hbm.at[sidx], rs, sem).wait()
        pltpu.async_copy(t_hbm.at[didx], rd, sem).wait()

        def ew(e, _):
            t = rs[e, pl.ds(0, 16)] + rd[e, pl.ds(16, 16)]
            al = jnp.maximum(t, 0.2 * t)
            ex = jnp.exp(al - rd[e, pl.ds(32, 16)])
            exs[e, pl.ds(0, 16)] = ex
            bufex[e >> 3, pl.ds((e & 7) * 16, 16)] = ex
            return 0

        lax.fori_loop(0, CHA, ew, 0)
        pltpu.async_copy(bufex, ex_hbm.at[pl.ds(ci * CHA8, CHA8)],
                         sem).wait()
        pltpu.sync_copy(exs, acc_sh.at[didx], add=True)
        return 0

    lax.fori_loop(0, trips, chunk, 0)
    plsc.subcore_barrier()
    _drain_acc(acc_sh, den_hbm, core, sub, sem)


_sc_a = pl.kernel(
    _sc_a_body,
    out_type=[
        jax.ShapeDtypeStruct((EX_ROWS, 128), jnp.float32),
        jax.ShapeDtypeStruct((NCORE, N, 128), jnp.float32),
    ],
    mesh=_mesh,
    compiler_params=_sc_params,
    scratch_types=[
        pltpu.VMEM((CHA,), jnp.int32),
        pltpu.VMEM((CHA,), jnp.int32),
        pltpu.VMEM((CHA, 128), jnp.float32),
        pltpu.VMEM((CHA, 128), jnp.float32),
        pltpu.VMEM((CHA, 128), jnp.float32),
        pltpu.VMEM((CHA8, 128), jnp.float32),
        pltpu.VMEM_SHARED((N, 128), jnp.float32),
        pltpu.SemaphoreType.DMA,
    ],
)


def _make_sc_b(hpc):
    vph = 8 // hpc

    def body(src_hbm, dst_hbm, ha_hbm, hb_hbm, ex_hbm,
             out_hbm, sidx, didx, rows, exb, acc_sh, sem):
        core = lax.axis_index("c")
        sub = lax.axis_index("s")
        _zero_acc(rows, acc_sh, sub)
        plsc.subcore_barrier()

        trips = (NCHUNK - sub + NSUB - 1) // NSUB
        zi = jnp.zeros((16,), jnp.int32)
        hoff = hpc * core if hpc > 1 else 0

        def chunk(kk, _):
            ci = sub + kk * NSUB
            base = ci * CH
            pltpu.async_copy(src_hbm.at[pl.ds(base, CH)], sidx, sem).wait()
            pltpu.async_copy(dst_hbm.at[pl.ds(base, CH)], didx, sem).wait()

            @pl.when(core == 0)
            def _():
                pltpu.async_copy(ha_hbm.at[sidx], rows, sem).wait()

            @pl.when(core == 1)
            def _():
                pltpu.async_copy(hb_hbm.at[sidx], rows, sem).wait()

            pltpu.async_copy(ex_hbm.at[pl.ds(ci * CH8, CH8)], exb,
                             sem).wait()

            def ew(e, _):
                r8 = zi + (e >> 3)
                l0 = zi + ((e & 7) * 16 + hoff)
                for j in range(hpc):
                    wj = plsc.load_gather(exb, [r8, l0 + j])
                    for v in range(vph):
                        col = (j * vph + v) * 16
                        rows[e, pl.ds(col, 16)] = \
                            rows[e, pl.ds(col, 16)] * wj
                return 0

            lax.fori_loop(0, CH, ew, 0)
            pltpu.sync_copy(rows, acc_sh.at[didx], add=True)
            return 0

        lax.fori_loop(0, trips, chunk, 0)
        plsc.subcore_barrier()
        _drain_acc(acc_sh, out_hbm, core, sub, sem)

    return pl.kernel(
        body,
        out_type=jax.ShapeDtypeStruct((NCORE, N, 128), jnp.float32),
        mesh=_mesh,
        compiler_params=_sc_params,
        scratch_types=[
            pltpu.VMEM((CH,), jnp.int32),
            pltpu.VMEM((CH,), jnp.int32),
            pltpu.VMEM((CH, 128), jnp.float32),
            pltpu.VMEM((CH8, 128), jnp.float32),
            pltpu.VMEM_SHARED((N, 128), jnp.float32),
            pltpu.SemaphoreType.DMA,
        ],
    )


_sc_b1 = _make_sc_b(4)
_sc_b23 = _make_sc_b(1)


def _sc_a3_body(src_hbm, dst_hbm, q_hbm, k_hbm, ex_hbm, den_hbm,
                sidx, didx, qb, kb, exs, bufex, alb, acc_sh, sem):
    core = lax.axis_index("c")
    sub = lax.axis_index("s")
    wid = core * NSUB + sub
    _zero_acc(qb, acc_sh, sub)
    _zero_vbuf(exs, CHA)
    plsc.subcore_barrier()

    trips = (NCHUNKA - wid + NW - 1) // NW
    lanes = lax.iota(jnp.int32, 16)

    def chunk(kk, _):
        ci = wid + kk * NW
        base = ci * CHA
        pltpu.async_copy(src_hbm.at[pl.ds(base, CHA)], sidx, sem).wait()
        pltpu.async_copy(dst_hbm.at[pl.ds(base, CHA)], didx, sem).wait()
        pltpu.async_copy(q_hbm.at[didx], qb, sem).wait()
        pltpu.async_copy(k_hbm.at[sidx], kb, sem).wait()

        for g in range(CHA // 16):
            ridx = g * 16 + lanes

            def dot_step(f, acc):
                f0 = f * 4
                for d in range(4):
                    fcol = jnp.zeros((16,), jnp.int32) + (f0 + d)
                    qv = plsc.load_gather(qb, [ridx, fcol])
                    kv = plsc.load_gather(kb, [ridx, fcol])
                    acc = acc + qv * kv
                return acc

            acc = lax.fori_loop(0, 32, dot_step,
                                jnp.zeros((16,), jnp.float32))
            alb[pl.ds(g * 16, 16)] = acc

        for g in range(CHA // 16):
            ev = jnp.exp(alb[pl.ds(g * 16, 16)])
            plsc.store_scatter(exs, [g * 16 + lanes,
                                     jnp.zeros((16,), jnp.int32)], ev)
            plsc.store_scatter(bufex, [(g * 16 + lanes) >> 3,
                                       ((g * 16 + lanes) & 7) * 16], ev)

        pltpu.async_copy(bufex, ex_hbm.at[pl.ds(ci * CHA8, CHA8)],
                         sem).wait()
        pltpu.sync_copy(exs, acc_sh.at[didx], add=True)
        return 0

    lax.fori_loop(0, trips, chunk, 0)
    plsc.subcore_barrier()
    _drain_acc(acc_sh, den_hbm, core, sub, sem)


_sc_a3 = pl.kernel(
    _sc_a3_body,
    out_type=[
        jax.ShapeDtypeStruct((EX_ROWS, 128), jnp.float32),
        jax.ShapeDtypeStruct((NCORE, N, 128), jnp.float32),
    ],
    mesh=_mesh,
    compiler_params=_sc_params,
    scratch_types=[
        pltpu.VMEM((CHA,), jnp.int32),
        pltpu.VMEM((CHA,), jnp.int32),
        pltpu.VMEM((CHA, 128), jnp.float32),
        pltpu.VMEM((CHA, 128), jnp.float32),
        pltpu.VMEM((CHA, 128), jnp.float32),
        pltpu.VMEM((CHA8, 128), jnp.float32),
        pltpu.VMEM((CHA,), jnp.float32),
        pltpu.VMEM_SHARED((N, 128), jnp.float32),
        pltpu.SemaphoreType.DMA,
    ],
)


def _leaky(v):
    return jnp.maximum(v, 0.2 * v)


def _den_slice(denp, h):
    return denp[0, :, 0:h] + denp[1, :, 0:h]


def kernel(x, edge_index, W1, att_src1, att_dst1, b1, W2, att_src2, att_dst2,
           b2, Wq, bq, Wk, bk, Wv, bv, Wskip, bskip, Wres, bres):
    src = edge_index[0]
    dst = edge_index[1]
    f32 = jnp.float32

    heads1, ch1 = att_src1.shape
    eye1 = jnp.eye(heads1, dtype=f32)
    blk_s = (att_src1[:, :, None] * eye1[:, None, :]).reshape(HID, heads1)
    blk_d = (att_dst1[:, :, None] * eye1[:, None, :]).reshape(HID, heads1)
    A1 = jnp.zeros((HID, 128), f32)
    A1 = A1.at[:, 0:8].set(blk_s).at[:, 16:24].set(blk_d)
    A2 = jnp.zeros((HID, 128), f32)
    A2 = A2.at[:, 0:1].set(att_src2.T).at[:, 16:17].set(att_dst2.T)

    scale = 1.0 / jnp.sqrt(jnp.float32(NC))
    Wcat = jnp.concatenate([Wq * scale, Wk, Wv, Wskip], axis=1)
    bcat = jnp.tile(jnp.concatenate([bq * scale, bk, bv, bskip])[None, :],
                    (8, 1))
    b1p = jnp.tile(b1[None, :], (8, 1))
    b2p = jnp.tile(b2[None, :], (8, 1))
    bresp = jnp.tile(bres[None, :], (8, 1))

    h1a, h1b, a1 = _k1(x, W1, A1)
    asrc1 = a1[:, 0:8]
    adst1 = a1[:, 16:24]
    gmax1 = jnp.max(asrc1, axis=0, keepdims=True)
    c1 = _leaky(adst1 + gmax1)
    exs1 = jnp.exp(_leaky(asrc1 + adst1) - c1)
    T1 = a1.at[:, 32:40].set(c1)

    ex1, den1p = _sc_a(src, dst, T1)
    den1 = _den_slice(den1p, 8) + exs1
    inv1 = 1.0 / (den1 + 1e-16)
    exsx1 = jnp.repeat(exs1, ch1, axis=1)
    invx1 = jnp.repeat(inv1, ch1, axis=1)

    scat1 = _sc_b1(src, dst, h1a, h1b, ex1)

    h2a, h2b, a2 = _k2(scat1[0], scat1[1], h1a, h1b, exsx1, invx1, b1p,
                       W2, A2)
    asrc2 = a2[:, 0:1]
    adst2 = a2[:, 16:17]
    gmax2 = jnp.max(asrc2, axis=0, keepdims=True)
    c2 = _leaky(adst2 + gmax2)
    exs2 = jnp.exp(_leaky(asrc2 + adst2) - c2)
    T2 = a2.at[:, 32:33].set(c2)

    ex2, den2p = _sc_a(src, dst, T2)
    den2 = _den_slice(den2p, 1) + exs2
    inv2 = 1.0 / (den2 + 1e-16)
    exsx2 = jnp.broadcast_to(exs2, (N, HID))
    invx2 = jnp.broadcast_to(inv2, (N, HID))

    scat2 = _sc_b23(src, dst, h2a, h2b, ex2)

    q3, k3, v3, skip3 = _k3(scat2[0], scat2[1], h2a, h2b, exsx2, invx2, b2p,
                            Wcat, bcat)

    ex3, den3p = _sc_a3(src, dst, q3, k3)
    den3 = _den_slice(den3p, 1)
    inv3 = 1.0 / (den3 + 1e-16)
    invx3 = jnp.broadcast_to(inv3, (N, 128))

    v3a = jnp.pad(v3[:, 0:64], ((0, 0), (0, 64)))
    v3b = jnp.pad(v3[:, 64:128], ((0, 0), (0, 64)))
    scat3 = _sc_b23(src, dst, v3a, v3b, ex3)

    return _k4(scat3[0], scat3[1], invx3, skip3, x, Wres, bresp)


# speedup vs baseline: 12.0221x; 1.2057x over previous
import jax
import jax.numpy as jnp
from jax import lax
from jax.experimental import pallas as pl
from jax.experimental.pallas import tpu as pltpu
from jax.experimental.pallas import tpu_sc as plsc

N = 10000
E = 160000
F_IN = 256
HID = 256
NC = 128

MBLK = 400
GRID = N // MBLK
CH = 128
CH8 = CH // 8
NCHUNK = E // CH
NSUB = 16
NCORE = 2
NW = NCORE * NSUB
RA = 640
RATAIL = N - (NSUB - 1) * RA
CHA = 64
CHA8 = CHA // 8
NCHUNKA = E // CHA
EX_ROWS = E // 8

_mesh = plsc.VectorSubcoreMesh(core_axis_name="c", subcore_axis_name="s")
_sc_params = pltpu.CompilerParams(needs_layout_passes=False)


def _k1_body(x_ref, w_ref, a_ref, ha_ref, hb_ref, a1_ref):
    h = jnp.dot(x_ref[...], w_ref[...], preferred_element_type=jnp.float32)
    ha_ref[...] = h[:, :128]
    hb_ref[...] = h[:, 128:]
    a1_ref[...] = jnp.dot(h, a_ref[...], preferred_element_type=jnp.float32)


def _k1(x, W1, A1):
    return pl.pallas_call(
        _k1_body,
        grid=(GRID,),
        in_specs=[
            pl.BlockSpec((MBLK, F_IN), lambda i: (i, 0)),
            pl.BlockSpec((F_IN, HID), lambda i: (0, 0)),
            pl.BlockSpec((HID, 128), lambda i: (0, 0)),
        ],
        out_specs=[
            pl.BlockSpec((MBLK, 128), lambda i: (i, 0)),
            pl.BlockSpec((MBLK, 128), lambda i: (i, 0)),
            pl.BlockSpec((MBLK, 128), lambda i: (i, 0)),
        ],
        out_shape=[
            jax.ShapeDtypeStruct((N, 128), jnp.float32),
            jax.ShapeDtypeStruct((N, 128), jnp.float32),
            jax.ShapeDtypeStruct((N, 128), jnp.float32),
        ],
    )(x, W1, A1)


def _elu(v):
    return jnp.where(v > 0, v, jnp.exp(v) - 1.0)


def _k2_body(sa_ref, sb_ref, ha_ref, hb_ref, exs_ref, inv_ref, b_ref,
             w_ref, a_ref, h2a_ref, h2b_ref, a2_ref):
    scat = jnp.concatenate([sa_ref[...], sb_ref[...]], axis=1)
    h = jnp.concatenate([ha_ref[...], hb_ref[...]], axis=1)
    x1 = _elu((scat + h * exs_ref[...]) * inv_ref[...] + b_ref[0:1, :])
    h2 = jnp.dot(x1, w_ref[...], preferred_element_type=jnp.float32)
    h2a_ref[...] = h2[:, :128]
    h2b_ref[...] = h2[:, 128:]
    a2_ref[...] = jnp.dot(h2, a_ref[...], preferred_element_type=jnp.float32)


def _k2(sa, sb, ha, hb, exsx, invx, bpad, W2, A2):
    return pl.pallas_call(
        _k2_body,
        grid=(GRID,),
        in_specs=[
            pl.BlockSpec((MBLK, 128), lambda i: (i, 0)),
            pl.BlockSpec((MBLK, 128), lambda i: (i, 0)),
            pl.BlockSpec((MBLK, 128), lambda i: (i, 0)),
            pl.BlockSpec((MBLK, 128), lambda i: (i, 0)),
            pl.BlockSpec((MBLK, HID), lambda i: (i, 0)),
            pl.BlockSpec((MBLK, HID), lambda i: (i, 0)),
            pl.BlockSpec((8, HID), lambda i: (0, 0)),
            pl.BlockSpec((HID, HID), lambda i: (0, 0)),
            pl.BlockSpec((HID, 128), lambda i: (0, 0)),
        ],
        out_specs=[
            pl.BlockSpec((MBLK, 128), lambda i: (i, 0)),
            pl.BlockSpec((MBLK, 128), lambda i: (i, 0)),
            pl.BlockSpec((MBLK, 128), lambda i: (i, 0)),
        ],
        out_shape=[
            jax.ShapeDtypeStruct((N, 128), jnp.float32),
            jax.ShapeDtypeStruct((N, 128), jnp.float32),
            jax.ShapeDtypeStruct((N, 128), jnp.float32),
        ],
    )(sa, sb, ha, hb, exsx, invx, bpad, W2, A2)


def _k3_body(sa_ref, sb_ref, ha_ref, hb_ref, exs_ref, inv_ref, b_ref,
             w_ref, bc_ref, q_ref, k_ref, v_ref, skip_ref):
    scat = jnp.concatenate([sa_ref[...], sb_ref[...]], axis=1)
    h = jnp.concatenate([ha_ref[...], hb_ref[...]], axis=1)
    x2 = _elu((scat + h * exs_ref[...]) * inv_ref[...] + b_ref[0:1, :])
    y = jnp.dot(x2, w_ref[...], preferred_element_type=jnp.float32)
    y = y + bc_ref[0:1, :]
    q_ref[...] = y[:, 0:128]
    k_ref[...] = y[:, 128:256]
    v_ref[...] = y[:, 256:384]
    skip_ref[...] = y[:, 384:512]


def _k3(sa, sb, ha, hb, exsx, invx, bpad, Wcat, bcat):
    return pl.pallas_call(
        _k3_body,
        grid=(GRID,),
        in_specs=[
            pl.BlockSpec((MBLK, 128), lambda i: (i, 0)),
            pl.BlockSpec((MBLK, 128), lambda i: (i, 0)),
            pl.BlockSpec((MBLK, 128), lambda i: (i, 0)),
            pl.BlockSpec((MBLK, 128), lambda i: (i, 0)),
            pl.BlockSpec((MBLK, HID), lambda i: (i, 0)),
            pl.BlockSpec((MBLK, HID), lambda i: (i, 0)),
            pl.BlockSpec((8, HID), lambda i: (0, 0)),
            pl.BlockSpec((HID, 512), lambda i: (0, 0)),
            pl.BlockSpec((8, 512), lambda i: (0, 0)),
        ],
        out_specs=[pl.BlockSpec((MBLK, 128), lambda i: (i, 0))] * 4,
        out_shape=[jax.ShapeDtypeStruct((N, 128), jnp.float32)] * 4,
    )(sa, sb, ha, hb, exsx, invx, bpad, Wcat, bcat)


def _k4_body(sa_ref, sb_ref, inv_ref, skip_ref, x_ref, w_ref, b_ref, o_ref):
    acc = jnp.dot(x_ref[...], w_ref[...], preferred_element_type=jnp.float32)
    x3 = jnp.concatenate([sa_ref[...][:, 0:64], sb_ref[...][:, 0:64]], axis=1)
    o_ref[...] = jnp.tanh(x3 * inv_ref[...]
                          + skip_ref[...] + acc + b_ref[0:1, :])


def _k4(sa, sb, invx, skip, x, Wres, bpad):
    return pl.pallas_call(
        _k4_body,
        grid=(GRID,),
        in_specs=[
            pl.BlockSpec((MBLK, 128), lambda i: (i, 0)),
            pl.BlockSpec((MBLK, 128), lambda i: (i, 0)),
            pl.BlockSpec((MBLK, 128), lambda i: (i, 0)),
            pl.BlockSpec((MBLK, 128), lambda i: (i, 0)),
            pl.BlockSpec((MBLK, F_IN), lambda i: (i, 0)),
            pl.BlockSpec((F_IN, 128), lambda i: (0, 0)),
            pl.BlockSpec((8, 128), lambda i: (0, 0)),
        ],
        out_specs=pl.BlockSpec((MBLK, 128), lambda i: (i, 0)),
        out_shape=jax.ShapeDtypeStruct((N, 128), jnp.float32),
    )(sa, sb, invx, skip, x, Wres, bpad)


def _zero_vbuf(vbuf, rows):
    z = jnp.zeros((16,), jnp.float32)
    nv = vbuf.shape[1] // 16

    def zr(r, _):
        for j in range(nv):
            vbuf[r, pl.ds(j * 16, 16)] = z
        return 0

    lax.fori_loop(0, rows, zr, 0)


def _zero_acc(zb, acc_sh, sub):
    zr = zb.shape[0]
    _zero_vbuf(zb, zr)

    @pl.when(sub < NSUB - 1)
    def _():
        for k in range(RA // zr):
            pltpu.sync_copy(zb, acc_sh.at[pl.ds(sub * RA + k * zr, zr)])

    @pl.when(sub == NSUB - 1)
    def _():
        for k in range(RATAIL // zr):
            pltpu.sync_copy(zb, acc_sh.at[pl.ds((NSUB - 1) * RA + k * zr,
                                                zr)])
        rem = RATAIL % zr
        if rem:
            pltpu.sync_copy(zb.at[pl.ds(0, rem)],
                            acc_sh.at[pl.ds(N - rem, rem)])


def _drain_acc(acc_sh, out_hbm, core, sub, sem):
    @pl.when(sub < NSUB - 1)
    def _():
        pltpu.async_copy(acc_sh.at[pl.ds(sub * RA, RA)],
                         out_hbm.at[core, pl.ds(sub * RA, RA)], sem).wait()

    @pl.when(sub == NSUB - 1)
    def _():
        pltpu.async_copy(acc_sh.at[pl.ds((NSUB - 1) * RA, RATAIL)],
                         out_hbm.at[core, pl.ds((NSUB - 1) * RA, RATAIL)],
                         sem).wait()


def _sc_a_body(src_hbm, dst_hbm, t_hbm, ex_hbm, den_hbm,
               sidx, didx, rs, rd, exs, bufex, acc_sh, sem):
    core = lax.axis_index("c")
    sub = lax.axis_index("s")
    wid = core * NSUB + sub
    _zero_acc(rs, acc_sh, sub)
    _zero_vbuf(exs, CHA)
    plsc.subcore_barrier()

    trips = (NCHUNKA - wid + NW - 1) // NW

    def chunk(kk, _):
        ci = wid + kk * NW
        base = ci * CHA
        pltpu.async_copy(src_hbm.at[pl.ds(base, CHA)], sidx, sem).wait()
        pltpu.async_copy(dst_hbm.at[pl.ds(base, CHA)], didx, sem).wait()
        pltpu.async_copy(t_hbm.at[sidx], rs, sem).wait()
        pltpu.async_copy(t_hbm.at[didx], rd, sem).wait()

        def ew(e, _):
            t = rs[e, pl.ds(0, 16)] + rd[e, pl.ds(16, 16)]
            al = jnp.maximum(t, 0.2 * t)
            ex = jnp.exp(al - rd[e, pl.ds(32, 16)])
            exs[e, pl.ds(0, 16)] = ex
            bufex[e >> 3, pl.ds((e & 7) * 16, 16)] = ex
            return 0

        lax.fori_loop(0, CHA, ew, 0)
        pltpu.async_copy(bufex, ex_hbm.at[pl.ds(ci * CHA8, CHA8)],
                         sem).wait()
        pltpu.sync_copy(exs, acc_sh.at[didx], add=True)
        return 0

    lax.fori_loop(0, trips, chunk, 0)
    plsc.subcore_barrier()
    _drain_acc(acc_sh, den_hbm, core, sub, sem)


_sc_a = pl.kernel(
    _sc_a_body,
    out_type=[
        jax.ShapeDtypeStruct((EX_ROWS, 128), jnp.float32),
        jax.ShapeDtypeStruct((NCORE, N, 128), jnp.float32),
    ],
    mesh=_mesh,
    compiler_params=_sc_params,
    scratch_types=[
        pltpu.VMEM((CHA,), jnp.int32),
        pltpu.VMEM((CHA,), jnp.int32),
        pltpu.VMEM((CHA, 128), jnp.float32),
        pltpu.VMEM((CHA, 128), jnp.float32),
        pltpu.VMEM((CHA, 128), jnp.float32),
        pltpu.VMEM((CHA8, 128), jnp.float32),
        pltpu.VMEM_SHARED((N, 128), jnp.float32),
        pltpu.SemaphoreType.DMA,
    ],
)


def _make_sc_b(hpc):
    vph = 8 // hpc

    def body(src_hbm, dst_hbm, ha_hbm, hb_hbm, ex_hbm,
             out_hbm, sidx, didx, rows, exb, acc_sh, sem):
        core = lax.axis_index("c")
        sub = lax.axis_index("s")
        _zero_acc(rows, acc_sh, sub)
        plsc.subcore_barrier()

        trips = (NCHUNK - sub + NSUB - 1) // NSUB
        zi = jnp.zeros((16,), jnp.int32)
        hoff = hpc * core if hpc > 1 else 0

        def chunk(kk, _):
            ci = sub + kk * NSUB
            base = ci * CH
            pltpu.async_copy(src_hbm.at[pl.ds(base, CH)], sidx, sem).wait()
            pltpu.async_copy(dst_hbm.at[pl.ds(base, CH)], didx, sem).wait()

            @pl.when(core == 0)
            def _():
                pltpu.async_copy(ha_hbm.at[sidx], rows, sem).wait()

            @pl.when(core == 1)
            def _():
                pltpu.async_copy(hb_hbm.at[sidx], rows, sem).wait()

            pltpu.async_copy(ex_hbm.at[pl.ds(ci * CH8, CH8)], exb,
                             sem).wait()

            def ew(e, _):
                r8 = zi + (e >> 3)
                l0 = zi + ((e & 7) * 16 + hoff)
                for j in range(hpc):
                    wj = plsc.load_gather(exb, [r8, l0 + j])
                    for v in range(vph):
                        col = (j * vph + v) * 16
                        rows[e, pl.ds(col, 16)] = \
                            rows[e, pl.ds(col, 16)] * wj
                return 0

            lax.fori_loop(0, CH, ew, 0)
            pltpu.sync_copy(rows, acc_sh.at[didx], add=True)
            return 0

        lax.fori_loop(0, trips, chunk, 0)
        plsc.subcore_barrier()
        _drain_acc(acc_sh, out_hbm, core, sub, sem)

    return pl.kernel(
        body,
        out_type=jax.ShapeDtypeStruct((NCORE, N, 128), jnp.float32),
        mesh=_mesh,
        compiler_params=_sc_params,
        scratch_types=[
            pltpu.VMEM((CH,), jnp.int32),
            pltpu.VMEM((CH,), jnp.int32),
            pltpu.VMEM((CH, 128), jnp.float32),
            pltpu.VMEM((CH8, 128), jnp.float32),
            pltpu.VMEM_SHARED((N, 128), jnp.float32),
            pltpu.SemaphoreType.DMA,
        ],
    )


_sc_b1 = _make_sc_b(4)
_sc_b23 = _make_sc_b(1)


def _sc_a3_body(src_hbm, dst_hbm, q_hbm, k_hbm, ex_hbm, den_hbm,
                sidx, didx, qb, kb, exs, bufex, alb, tmp, acc_sh, sem):
    core = lax.axis_index("c")
    sub = lax.axis_index("s")
    wid = core * NSUB + sub
    _zero_acc(qb, acc_sh, sub)
    _zero_vbuf(exs, CHA)
    plsc.subcore_barrier()

    trips = (NCHUNKA - wid + NW - 1) // NW
    lanes = lax.iota(jnp.int32, 16)

    def chunk(kk, _):
        ci = wid + kk * NW
        base = ci * CHA
        pltpu.async_copy(src_hbm.at[pl.ds(base, CHA)], sidx, sem).wait()
        pltpu.async_copy(dst_hbm.at[pl.ds(base, CHA)], didx, sem).wait()
        pltpu.async_copy(q_hbm.at[didx], qb, sem).wait()
        pltpu.async_copy(k_hbm.at[sidx], kb, sem).wait()

        def dot_edge(e, _):
            p = [qb[e, pl.ds(v * 16, 16)] * kb[e, pl.ds(v * 16, 16)]
                 for v in range(8)]
            acc = ((p[0] + p[1]) + (p[2] + p[3])) + \
                  ((p[4] + p[5]) + (p[6] + p[7]))
            tmp[e & 15, :] = acc
            return 0

        for g in range(CHA // 16):
            lax.fori_loop(g * 16, g * 16 + 16, dot_edge, 0)
            red = plsc.load_gather(tmp, [lanes, jnp.zeros((16,), jnp.int32)])
            for c in range(1, 16):
                red = red + plsc.load_gather(tmp,
                                             [lanes,
                                              jnp.zeros((16,), jnp.int32) + c])
            alb[pl.ds(g * 16, 16)] = red

        for g in range(CHA // 16):
            ev = jnp.exp(alb[pl.ds(g * 16, 16)])
            plsc.store_scatter(exs, [g * 16 + lanes,
                                     jnp.zeros((16,), jnp.int32)], ev)
            plsc.store_scatter(bufex, [(g * 16 + lanes) >> 3,
                                       ((g * 16 + lanes) & 7) * 16], ev)

        pltpu.async_copy(bufex, ex_hbm.at[pl.ds(ci * CHA8, CHA8)],
                         sem).wait()
        pltpu.sync_copy(exs, acc_sh.at[didx], add=True)
        return 0

    lax.fori_loop(0, trips, chunk, 0)
    plsc.subcore_barrier()
    _drain_acc(acc_sh, den_hbm, core, sub, sem)


_sc_a3 = pl.kernel(
    _sc_a3_body,
    out_type=[
        jax.ShapeDtypeStruct((EX_ROWS, 128), jnp.float32),
        jax.ShapeDtypeStruct((NCORE, N, 128), jnp.float32),
    ],
    mesh=_mesh,
    compiler_params=_sc_params,
    scratch_types=[
        pltpu.VMEM((CHA,), jnp.int32),
        pltpu.VMEM((CHA,), jnp.int32),
        pltpu.VMEM((CHA, 128), jnp.float32),
        pltpu.VMEM((CHA, 128), jnp.float32),
        pltpu.VMEM((CHA, 128), jnp.float32),
        pltpu.VMEM((CHA8, 128), jnp.float32),
        pltpu.VMEM((CHA,), jnp.float32),
        pltpu.VMEM((16, 16), jnp.float32),
        pltpu.VMEM_SHARED((N, 128), jnp.float32),
        pltpu.SemaphoreType.DMA,
    ],
)


def _leaky(v):
    return jnp.maximum(v, 0.2 * v)


def _den_slice(denp, h):
    return denp[0, :, 0:h] + denp[1, :, 0:h]


def kernel(x, edge_index, W1, att_src1, att_dst1, b1, W2, att_src2, att_dst2,
           b2, Wq, bq, Wk, bk, Wv, bv, Wskip, bskip, Wres, bres):
    src = edge_index[0]
    dst = edge_index[1]
    f32 = jnp.float32

    heads1, ch1 = att_src1.shape
    eye1 = jnp.eye(heads1, dtype=f32)
    blk_s = (att_src1[:, :, None] * eye1[:, None, :]).reshape(HID, heads1)
    blk_d = (att_dst1[:, :, None] * eye1[:, None, :]).reshape(HID, heads1)
    A1 = jnp.zeros((HID, 128), f32)
    A1 = A1.at[:, 0:8].set(blk_s).at[:, 16:24].set(blk_d)
    A2 = jnp.zeros((HID, 128), f32)
    A2 = A2.at[:, 0:1].set(att_src2.T).at[:, 16:17].set(att_dst2.T)

    scale = 1.0 / jnp.sqrt(jnp.float32(NC))
    Wcat = jnp.concatenate([Wq * scale, Wk, Wv, Wskip], axis=1)
    bcat = jnp.tile(jnp.concatenate([bq * scale, bk, bv, bskip])[None, :],
                    (8, 1))
    b1p = jnp.tile(b1[None, :], (8, 1))
    b2p = jnp.tile(b2[None, :], (8, 1))
    bresp = jnp.tile(bres[None, :], (8, 1))

    h1a, h1b, a1 = _k1(x, W1, A1)
    asrc1 = a1[:, 0:8]
    adst1 = a1[:, 16:24]
    gmax1 = jnp.max(asrc1, axis=0, keepdims=True)
    c1 = _leaky(adst1 + gmax1)
    exs1 = jnp.exp(_leaky(asrc1 + adst1) - c1)
    T1 = a1.at[:, 32:40].set(c1)

    ex1, den1p = _sc_a(src, dst, T1)
    den1 = _den_slice(den1p, 8) + exs1
    inv1 = 1.0 / (den1 + 1e-16)
    exsx1 = jnp.repeat(exs1, ch1, axis=1)
    invx1 = jnp.repeat(inv1, ch1, axis=1)

    scat1 = _sc_b1(src, dst, h1a, h1b, ex1)

    h2a, h2b, a2 = _k2(scat1[0], scat1[1], h1a, h1b, exsx1, invx1, b1p,
                       W2, A2)
    asrc2 = a2[:, 0:1]
    adst2 = a2[:, 16:17]
    gmax2 = jnp.max(asrc2, axis=0, keepdims=True)
    c2 = _leaky(adst2 + gmax2)
    exs2 = jnp.exp(_leaky(asrc2 + adst2) - c2)
    T2 = a2.at[:, 32:33].set(c2)

    ex2, den2p = _sc_a(src, dst, T2)
    den2 = _den_slice(den2p, 1) + exs2
    inv2 = 1.0 / (den2 + 1e-16)
    exsx2 = jnp.broadcast_to(exs2, (N, HID))
    invx2 = jnp.broadcast_to(inv2, (N, HID))

    scat2 = _sc_b23(src, dst, h2a, h2b, ex2)

    q3, k3, v3, skip3 = _k3(scat2[0], scat2[1], h2a, h2b, exsx2, invx2, b2p,
                            Wcat, bcat)

    ex3, den3p = _sc_a3(src, dst, q3, k3)
    den3 = _den_slice(den3p, 1)
    inv3 = 1.0 / (den3 + 1e-16)
    invx3 = jnp.broadcast_to(inv3, (N, 128))

    v3a = jnp.pad(v3[:, 0:64], ((0, 0), (0, 64)))
    v3b = jnp.pad(v3[:, 64:128], ((0, 0), (0, 64)))
    scat3 = _sc_b23(src, dst, v3a, v3b, ex3)

    return _k4(scat3[0], scat3[1], invx3, skip3, x, Wres, bresp)


# speedup vs baseline: 13.2232x; 1.0999x over previous
import jax
import jax.numpy as jnp
from jax import lax
from jax.experimental import pallas as pl
from jax.experimental.pallas import tpu as pltpu
from jax.experimental.pallas import tpu_sc as plsc

N = 10000
E = 160000
F_IN = 256
HID = 256
NC = 128

MBLK = 400
GRID = N // MBLK
CH = 128
CH8 = CH // 8
NCHUNK = E // CH
NSUB = 16
NCORE = 2
NW = NCORE * NSUB
RA = 640
RATAIL = N - (NSUB - 1) * RA
CHA = 64
CHA8 = CHA // 8
NCHUNKA = E // CHA
EX_ROWS = E // 8

_mesh = plsc.VectorSubcoreMesh(core_axis_name="c", subcore_axis_name="s")
_sc_params = pltpu.CompilerParams(needs_layout_passes=False)


def _k1_body(x_ref, w_ref, a_ref, ha_ref, hb_ref, a1_ref):
    h = jnp.dot(x_ref[...], w_ref[...], preferred_element_type=jnp.float32)
    ha_ref[...] = h[:, :128]
    hb_ref[...] = h[:, 128:]
    a1_ref[...] = jnp.dot(h, a_ref[...], preferred_element_type=jnp.float32)


def _k1(x, W1, A1):
    return pl.pallas_call(
        _k1_body,
        grid=(GRID,),
        in_specs=[
            pl.BlockSpec((MBLK, F_IN), lambda i: (i, 0)),
            pl.BlockSpec((F_IN, HID), lambda i: (0, 0)),
            pl.BlockSpec((HID, 128), lambda i: (0, 0)),
        ],
        out_specs=[
            pl.BlockSpec((MBLK, 128), lambda i: (i, 0)),
            pl.BlockSpec((MBLK, 128), lambda i: (i, 0)),
            pl.BlockSpec((MBLK, 128), lambda i: (i, 0)),
        ],
        out_shape=[
            jax.ShapeDtypeStruct((N, 128), jnp.float32),
            jax.ShapeDtypeStruct((N, 128), jnp.float32),
            jax.ShapeDtypeStruct((N, 128), jnp.float32),
        ],
    )(x, W1, A1)


def _elu(v):
    return jnp.where(v > 0, v, jnp.exp(v) - 1.0)


def _k2_body(sa_ref, sb_ref, ha_ref, hb_ref, exs_ref, inv_ref, b_ref,
             w_ref, a_ref, h2a_ref, h2b_ref, a2_ref):
    scat = jnp.concatenate([sa_ref[...], sb_ref[...]], axis=1)
    h = jnp.concatenate([ha_ref[...], hb_ref[...]], axis=1)
    x1 = _elu((scat + h * exs_ref[...]) * inv_ref[...] + b_ref[0:1, :])
    h2 = jnp.dot(x1, w_ref[...], preferred_element_type=jnp.float32)
    h2a_ref[...] = h2[:, :128]
    h2b_ref[...] = h2[:, 128:]
    a2_ref[...] = jnp.dot(h2, a_ref[...], preferred_element_type=jnp.float32)


def _k2(sa, sb, ha, hb, exsx, invx, bpad, W2, A2):
    return pl.pallas_call(
        _k2_body,
        grid=(GRID,),
        in_specs=[
            pl.BlockSpec((MBLK, 128), lambda i: (i, 0)),
            pl.BlockSpec((MBLK, 128), lambda i: (i, 0)),
            pl.BlockSpec((MBLK, 128), lambda i: (i, 0)),
            pl.BlockSpec((MBLK, 128), lambda i: (i, 0)),
            pl.BlockSpec((MBLK, HID), lambda i: (i, 0)),
            pl.BlockSpec((MBLK, HID), lambda i: (i, 0)),
            pl.BlockSpec((8, HID), lambda i: (0, 0)),
            pl.BlockSpec((HID, HID), lambda i: (0, 0)),
            pl.BlockSpec((HID, 128), lambda i: (0, 0)),
        ],
        out_specs=[
            pl.BlockSpec((MBLK, 128), lambda i: (i, 0)),
            pl.BlockSpec((MBLK, 128), lambda i: (i, 0)),
            pl.BlockSpec((MBLK, 128), lambda i: (i, 0)),
        ],
        out_shape=[
            jax.ShapeDtypeStruct((N, 128), jnp.float32),
            jax.ShapeDtypeStruct((N, 128), jnp.float32),
            jax.ShapeDtypeStruct((N, 128), jnp.float32),
        ],
    )(sa, sb, ha, hb, exsx, invx, bpad, W2, A2)


def _k3_body(sa_ref, sb_ref, ha_ref, hb_ref, exs_ref, inv_ref, b_ref,
             w_ref, bc_ref, q_ref, k_ref, v_ref, skip_ref):
    scat = jnp.concatenate([sa_ref[...], sb_ref[...]], axis=1)
    h = jnp.concatenate([ha_ref[...], hb_ref[...]], axis=1)
    x2 = _elu((scat + h * exs_ref[...]) * inv_ref[...] + b_ref[0:1, :])
    y = jnp.dot(x2, w_ref[...], preferred_element_type=jnp.float32)
    y = y + bc_ref[0:1, :]
    q_ref[...] = y[:, 0:128]
    k_ref[...] = y[:, 128:256]
    v_ref[...] = y[:, 256:384]
    skip_ref[...] = y[:, 384:512]


def _k3(sa, sb, ha, hb, exsx, invx, bpad, Wcat, bcat):
    return pl.pallas_call(
        _k3_body,
        grid=(GRID,),
        in_specs=[
            pl.BlockSpec((MBLK, 128), lambda i: (i, 0)),
            pl.BlockSpec((MBLK, 128), lambda i: (i, 0)),
            pl.BlockSpec((MBLK, 128), lambda i: (i, 0)),
            pl.BlockSpec((MBLK, 128), lambda i: (i, 0)),
            pl.BlockSpec((MBLK, HID), lambda i: (i, 0)),
            pl.BlockSpec((MBLK, HID), lambda i: (i, 0)),
            pl.BlockSpec((8, HID), lambda i: (0, 0)),
            pl.BlockSpec((HID, 512), lambda i: (0, 0)),
            pl.BlockSpec((8, 512), lambda i: (0, 0)),
        ],
        out_specs=[pl.BlockSpec((MBLK, 128), lambda i: (i, 0))] * 4,
        out_shape=[jax.ShapeDtypeStruct((N, 128), jnp.float32)] * 4,
    )(sa, sb, ha, hb, exsx, invx, bpad, Wcat, bcat)


def _k4_body(sa_ref, sb_ref, inv_ref, skip_ref, x_ref, w_ref, b_ref, o_ref):
    acc = jnp.dot(x_ref[...], w_ref[...], preferred_element_type=jnp.float32)
    x3 = jnp.concatenate([sa_ref[...][:, 0:64], sb_ref[...][:, 0:64]], axis=1)
    o_ref[...] = jnp.tanh(x3 * inv_ref[...]
                          + skip_ref[...] + acc + b_ref[0:1, :])


def _k4(sa, sb, invx, skip, x, Wres, bpad):
    return pl.pallas_call(
        _k4_body,
        grid=(GRID,),
        in_specs=[
            pl.BlockSpec((MBLK, 128), lambda i: (i, 0)),
            pl.BlockSpec((MBLK, 128), lambda i: (i, 0)),
            pl.BlockSpec((MBLK, 128), lambda i: (i, 0)),
            pl.BlockSpec((MBLK, 128), lambda i: (i, 0)),
            pl.BlockSpec((MBLK, F_IN), lambda i: (i, 0)),
            pl.BlockSpec((F_IN, 128), lambda i: (0, 0)),
            pl.BlockSpec((8, 128), lambda i: (0, 0)),
        ],
        out_specs=pl.BlockSpec((MBLK, 128), lambda i: (i, 0)),
        out_shape=jax.ShapeDtypeStruct((N, 128), jnp.float32),
    )(sa, sb, invx, skip, x, Wres, bpad)


def _zero_vbuf(vbuf, rows):
    z = jnp.zeros((16,), jnp.float32)
    nv = vbuf.shape[1] // 16

    def zr(r, _):
        for j in range(nv):
            vbuf[r, pl.ds(j * 16, 16)] = z
        return 0

    lax.fori_loop(0, rows, zr, 0)


def _zero_acc(zb, acc_sh, sub):
    zr = zb.shape[0]
    _zero_vbuf(zb, zr)

    @pl.when(sub < NSUB - 1)
    def _():
        for k in range(RA // zr):
            pltpu.sync_copy(zb, acc_sh.at[pl.ds(sub * RA + k * zr, zr)])

    @pl.when(sub == NSUB - 1)
    def _():
        for k in range(RATAIL // zr):
            pltpu.sync_copy(zb, acc_sh.at[pl.ds((NSUB - 1) * RA + k * zr,
                                                zr)])
        rem = RATAIL % zr
        if rem:
            pltpu.sync_copy(zb.at[pl.ds(0, rem)],
                            acc_sh.at[pl.ds(N - rem, rem)])


def _drain_acc(acc_sh, out_hbm, core, sub, sem):
    @pl.when(sub < NSUB - 1)
    def _():
        pltpu.async_copy(acc_sh.at[pl.ds(sub * RA, RA)],
                         out_hbm.at[core, pl.ds(sub * RA, RA)], sem).wait()

    @pl.when(sub == NSUB - 1)
    def _():
        pltpu.async_copy(acc_sh.at[pl.ds((NSUB - 1) * RA, RATAIL)],
                         out_hbm.at[core, pl.ds((NSUB - 1) * RA, RATAIL)],
                         sem).wait()


def _sc_a_body(src_hbm, dst_hbm, t_hbm, ex_hbm, den_hbm,
               sidx, didx, rs, rd, exs, bufex, acc_sh, sem):
    core = lax.axis_index("c")
    sub = lax.axis_index("s")
    wid = core * NSUB + sub
    _zero_acc(rs, acc_sh, sub)
    _zero_vbuf(exs, CHA)
    plsc.subcore_barrier()

    trips = (NCHUNKA - wid + NW - 1) // NW

    def chunk(kk, _):
        ci = wid + kk * NW
        base = ci * CHA
        pltpu.async_copy(src_hbm.at[pl.ds(base, CHA)], sidx, sem).wait()
        pltpu.async_copy(dst_hbm.at[pl.ds(base, CHA)], didx, sem).wait()
        pltpu.async_copy(t_hbm.at[sidx], rs, sem).wait()
        pltpu.async_copy(t_hbm.at[didx], rd, sem).wait()

        def ew(i, _):
            for u in range(4):
                e = i * 4 + u
                t = rs[e, pl.ds(0, 16)] + rd[e, pl.ds(16, 16)]
                al = jnp.maximum(t, 0.2 * t)
                ex = jnp.exp(al - rd[e, pl.ds(32, 16)])
                exs[e, pl.ds(0, 16)] = ex
                bufex[e >> 3, pl.ds((e & 7) * 16, 16)] = ex
            return 0

        lax.fori_loop(0, CHA // 4, ew, 0)
        pltpu.async_copy(bufex, ex_hbm.at[pl.ds(ci * CHA8, CHA8)],
                         sem).wait()
        pltpu.sync_copy(exs, acc_sh.at[didx], add=True)
        return 0

    lax.fori_loop(0, trips, chunk, 0)
    plsc.subcore_barrier()
    _drain_acc(acc_sh, den_hbm, core, sub, sem)


_sc_a = pl.kernel(
    _sc_a_body,
    out_type=[
        jax.ShapeDtypeStruct((EX_ROWS, 128), jnp.float32),
        jax.ShapeDtypeStruct((NCORE, N, 128), jnp.float32),
    ],
    mesh=_mesh,
    compiler_params=_sc_params,
    scratch_types=[
        pltpu.VMEM((CHA,), jnp.int32),
        pltpu.VMEM((CHA,), jnp.int32),
        pltpu.VMEM((CHA, 128), jnp.float32),
        pltpu.VMEM((CHA, 128), jnp.float32),
        pltpu.VMEM((CHA, 128), jnp.float32),
        pltpu.VMEM((CHA8, 128), jnp.float32),
        pltpu.VMEM_SHARED((N, 128), jnp.float32),
        pltpu.SemaphoreType.DMA,
    ],
)


def _make_sc_b(hpc):
    vph = 8 // hpc

    def body(src_hbm, dst_hbm, ha_hbm, hb_hbm, ex_hbm,
             out_hbm, sidx, didx, rows, exb, acc_sh, sem):
        core = lax.axis_index("c")
        sub = lax.axis_index("s")
        _zero_acc(rows, acc_sh, sub)
        plsc.subcore_barrier()

        trips = (NCHUNK - sub + NSUB - 1) // NSUB
        zi = jnp.zeros((16,), jnp.int32)
        hoff = hpc * core if hpc > 1 else 0

        def chunk(kk, _):
            ci = sub + kk * NSUB
            base = ci * CH
            pltpu.async_copy(src_hbm.at[pl.ds(base, CH)], sidx, sem).wait()
            pltpu.async_copy(dst_hbm.at[pl.ds(base, CH)], didx, sem).wait()

            @pl.when(core == 0)
            def _():
                pltpu.async_copy(ha_hbm.at[sidx], rows, sem).wait()

            @pl.when(core == 1)
            def _():
                pltpu.async_copy(hb_hbm.at[sidx], rows, sem).wait()

            pltpu.async_copy(ex_hbm.at[pl.ds(ci * CH8, CH8)], exb,
                             sem).wait()

            def ew(i, _):
                ws = []
                for u in range(4):
                    e = i * 4 + u
                    r8 = zi + (e >> 3)
                    l0 = zi + ((e & 7) * 16 + hoff)
                    ws.append([plsc.load_gather(exb, [r8, l0 + j])
                               for j in range(hpc)])
                for u in range(4):
                    e = i * 4 + u
                    for j in range(hpc):
                        for v in range(vph):
                            col = (j * vph + v) * 16
                            rows[e, pl.ds(col, 16)] = \
                                rows[e, pl.ds(col, 16)] * ws[u][j]
                return 0

            lax.fori_loop(0, CH // 4, ew, 0)
            pltpu.sync_copy(rows, acc_sh.at[didx], add=True)
            return 0

        lax.fori_loop(0, trips, chunk, 0)
        plsc.subcore_barrier()
        _drain_acc(acc_sh, out_hbm, core, sub, sem)

    return pl.kernel(
        body,
        out_type=jax.ShapeDtypeStruct((NCORE, N, 128), jnp.float32),
        mesh=_mesh,
        compiler_params=_sc_params,
        scratch_types=[
            pltpu.VMEM((CH,), jnp.int32),
            pltpu.VMEM((CH,), jnp.int32),
            pltpu.VMEM((CH, 128), jnp.float32),
            pltpu.VMEM((CH8, 128), jnp.float32),
            pltpu.VMEM_SHARED((N, 128), jnp.float32),
            pltpu.SemaphoreType.DMA,
        ],
    )


_sc_b1 = _make_sc_b(4)
_sc_b23 = _make_sc_b(1)


def _sc_a3_body(src_hbm, dst_hbm, q_hbm, k_hbm, ex_hbm, den_hbm,
                sidx, didx, qb, kb, exs, bufex, alb, tmp, acc_sh, sem):
    core = lax.axis_index("c")
    sub = lax.axis_index("s")
    wid = core * NSUB + sub
    _zero_acc(qb, acc_sh, sub)
    _zero_vbuf(exs, CHA)
    plsc.subcore_barrier()

    trips = (NCHUNKA - wid + NW - 1) // NW
    lanes = lax.iota(jnp.int32, 16)

    def chunk(kk, _):
        ci = wid + kk * NW
        base = ci * CHA
        pltpu.async_copy(src_hbm.at[pl.ds(base, CHA)], sidx, sem).wait()
        pltpu.async_copy(dst_hbm.at[pl.ds(base, CHA)], didx, sem).wait()
        pltpu.async_copy(q_hbm.at[didx], qb, sem).wait()
        pltpu.async_copy(k_hbm.at[sidx], kb, sem).wait()

        def dot_edge(i, _):
            for u in range(4):
                e = i * 4 + u
                p = [qb[e, pl.ds(v * 16, 16)] * kb[e, pl.ds(v * 16, 16)]
                     for v in range(8)]
                acc = ((p[0] + p[1]) + (p[2] + p[3])) + \
                      ((p[4] + p[5]) + (p[6] + p[7]))
                tmp[e & 15, :] = acc
            return 0

        for g in range(CHA // 16):
            lax.fori_loop(g * 4, g * 4 + 4, dot_edge, 0)
            red = plsc.load_gather(tmp, [lanes, jnp.zeros((16,), jnp.int32)])
            for c in range(1, 16):
                red = red + plsc.load_gather(tmp,
                                             [lanes,
                                              jnp.zeros((16,), jnp.int32) + c])
            alb[pl.ds(g * 16, 16)] = red

        for g in range(CHA // 16):
            ev = jnp.exp(alb[pl.ds(g * 16, 16)])
            plsc.store_scatter(exs, [g * 16 + lanes,
                                     jnp.zeros((16,), jnp.int32)], ev)
            plsc.store_scatter(bufex, [(g * 16 + lanes) >> 3,
                                       ((g * 16 + lanes) & 7) * 16], ev)

        pltpu.async_copy(bufex, ex_hbm.at[pl.ds(ci * CHA8, CHA8)],
                         sem).wait()
        pltpu.sync_copy(exs, acc_sh.at[didx], add=True)
        return 0

    lax.fori_loop(0, trips, chunk, 0)
    plsc.subcore_barrier()
    _drain_acc(acc_sh, den_hbm, core, sub, sem)


_sc_a3 = pl.kernel(
    _sc_a3_body,
    out_type=[
        jax.ShapeDtypeStruct((EX_ROWS, 128), jnp.float32),
        jax.ShapeDtypeStruct((NCORE, N, 128), jnp.float32),
    ],
    mesh=_mesh,
    compiler_params=_sc_params,
    scratch_types=[
        pltpu.VMEM((CHA,), jnp.int32),
        pltpu.VMEM((CHA,), jnp.int32),
        pltpu.VMEM((CHA, 128), jnp.float32),
        pltpu.VMEM((CHA, 128), jnp.float32),
        pltpu.VMEM((CHA, 128), jnp.float32),
        pltpu.VMEM((CHA8, 128), jnp.float32),
        pltpu.VMEM((CHA,), jnp.float32),
        pltpu.VMEM((16, 16), jnp.float32),
        pltpu.VMEM_SHARED((N, 128), jnp.float32),
        pltpu.SemaphoreType.DMA,
    ],
)


def _leaky(v):
    return jnp.maximum(v, 0.2 * v)


def _den_slice(denp, h):
    return denp[0, :, 0:h] + denp[1, :, 0:h]


def kernel(x, edge_index, W1, att_src1, att_dst1, b1, W2, att_src2, att_dst2,
           b2, Wq, bq, Wk, bk, Wv, bv, Wskip, bskip, Wres, bres):
    src = edge_index[0]
    dst = edge_index[1]
    f32 = jnp.float32

    heads1, ch1 = att_src1.shape
    eye1 = jnp.eye(heads1, dtype=f32)
    blk_s = (att_src1[:, :, None] * eye1[:, None, :]).reshape(HID, heads1)
    blk_d = (att_dst1[:, :, None] * eye1[:, None, :]).reshape(HID, heads1)
    A1 = jnp.zeros((HID, 128), f32)
    A1 = A1.at[:, 0:8].set(blk_s).at[:, 16:24].set(blk_d)
    A2 = jnp.zeros((HID, 128), f32)
    A2 = A2.at[:, 0:1].set(att_src2.T).at[:, 16:17].set(att_dst2.T)

    scale = 1.0 / jnp.sqrt(jnp.float32(NC))
    Wcat = jnp.concatenate([Wq * scale, Wk, Wv, Wskip], axis=1)
    bcat = jnp.tile(jnp.concatenate([bq * scale, bk, bv, bskip])[None, :],
                    (8, 1))
    b1p = jnp.tile(b1[None, :], (8, 1))
    b2p = jnp.tile(b2[None, :], (8, 1))
    bresp = jnp.tile(bres[None, :], (8, 1))

    h1a, h1b, a1 = _k1(x, W1, A1)
    asrc1 = a1[:, 0:8]
    adst1 = a1[:, 16:24]
    gmax1 = jnp.max(asrc1, axis=0, keepdims=True)
    c1 = _leaky(adst1 + gmax1)
    exs1 = jnp.exp(_leaky(asrc1 + adst1) - c1)
    T1 = a1.at[:, 32:40].set(c1)

    ex1, den1p = _sc_a(src, dst, T1)
    den1 = _den_slice(den1p, 8) + exs1
    inv1 = 1.0 / (den1 + 1e-16)
    exsx1 = jnp.repeat(exs1, ch1, axis=1)
    invx1 = jnp.repeat(inv1, ch1, axis=1)

    scat1 = _sc_b1(src, dst, h1a, h1b, ex1)

    h2a, h2b, a2 = _k2(scat1[0], scat1[1], h1a, h1b, exsx1, invx1, b1p,
                       W2, A2)
    asrc2 = a2[:, 0:1]
    adst2 = a2[:, 16:17]
    gmax2 = jnp.max(asrc2, axis=0, keepdims=True)
    c2 = _leaky(adst2 + gmax2)
    exs2 = jnp.exp(_leaky(asrc2 + adst2) - c2)
    T2 = a2.at[:, 32:33].set(c2)

    ex2, den2p = _sc_a(src, dst, T2)
    den2 = _den_slice(den2p, 1) + exs2
    inv2 = 1.0 / (den2 + 1e-16)
    exsx2 = jnp.broadcast_to(exs2, (N, HID))
    invx2 = jnp.broadcast_to(inv2, (N, HID))

    scat2 = _sc_b23(src, dst, h2a, h2b, ex2)

    q3, k3, v3, skip3 = _k3(scat2[0], scat2[1], h2a, h2b, exsx2, invx2, b2p,
                            Wcat, bcat)

    ex3, den3p = _sc_a3(src, dst, q3, k3)
    den3 = _den_slice(den3p, 1)
    inv3 = 1.0 / (den3 + 1e-16)
    invx3 = jnp.broadcast_to(inv3, (N, 128))

    v3a = jnp.pad(v3[:, 0:64], ((0, 0), (0, 64)))
    v3b = jnp.pad(v3[:, 64:128], ((0, 0), (0, 64)))
    scat3 = _sc_b23(src, dst, v3a, v3b, ex3)

    return _k4(scat3[0], scat3[1], invx3, skip3, x, Wres, bresp)


# speedup vs baseline: 14.9749x; 1.1325x over previous
import jax
import jax.numpy as jnp
from jax import lax
from jax.experimental import pallas as pl
from jax.experimental.pallas import tpu as pltpu
from jax.experimental.pallas import tpu_sc as plsc

N = 10000
E = 160000
F_IN = 256
HID = 256
NC = 128

MBLK = 400
GRID = N // MBLK
CH = 128
CH8 = CH // 8
NCHUNK = E // CH
NSUB = 16
NCORE = 2
NW = NCORE * NSUB
RA = 640
RATAIL = N - (NSUB - 1) * RA
CHA = 64
CHA8 = CHA // 8
NCHUNKA = E // CHA
EX_ROWS = E // 8

_mesh = plsc.VectorSubcoreMesh(core_axis_name="c", subcore_axis_name="s")
_sc_params = pltpu.CompilerParams(needs_layout_passes=False)


def _k1_body(x_ref, w_ref, a_ref, ha_ref, hb_ref, a1_ref):
    h = jnp.dot(x_ref[...], w_ref[...], preferred_element_type=jnp.float32)
    ha_ref[...] = h[:, :128]
    hb_ref[...] = h[:, 128:]
    a1_ref[...] = jnp.dot(h, a_ref[...], preferred_element_type=jnp.float32)


def _k1(x, W1, A1):
    return pl.pallas_call(
        _k1_body,
        grid=(GRID,),
        in_specs=[
            pl.BlockSpec((MBLK, F_IN), lambda i: (i, 0)),
            pl.BlockSpec((F_IN, HID), lambda i: (0, 0)),
            pl.BlockSpec((HID, 128), lambda i: (0, 0)),
        ],
        out_specs=[
            pl.BlockSpec((MBLK, 128), lambda i: (i, 0)),
            pl.BlockSpec((MBLK, 128), lambda i: (i, 0)),
            pl.BlockSpec((MBLK, 128), lambda i: (i, 0)),
        ],
        out_shape=[
            jax.ShapeDtypeStruct((N, 128), jnp.float32),
            jax.ShapeDtypeStruct((N, 128), jnp.float32),
            jax.ShapeDtypeStruct((N, 128), jnp.float32),
        ],
    )(x, W1, A1)


def _elu(v):
    return jnp.where(v > 0, v, jnp.exp(v) - 1.0)


def _k2_body(sa_ref, sb_ref, ha_ref, hb_ref, exs_ref, inv_ref, b_ref,
             w_ref, a_ref, h2a_ref, h2b_ref, a2_ref):
    scat = jnp.concatenate([sa_ref[...], sb_ref[...]], axis=1)
    h = jnp.concatenate([ha_ref[...], hb_ref[...]], axis=1)
    x1 = _elu((scat + h * exs_ref[...]) * inv_ref[...] + b_ref[0:1, :])
    h2 = jnp.dot(x1, w_ref[...], preferred_element_type=jnp.float32)
    h2a_ref[...] = h2[:, :128]
    h2b_ref[...] = h2[:, 128:]
    a2_ref[...] = jnp.dot(h2, a_ref[...], preferred_element_type=jnp.float32)


def _k2(sa, sb, ha, hb, exsx, invx, bpad, W2, A2):
    return pl.pallas_call(
        _k2_body,
        grid=(GRID,),
        in_specs=[
            pl.BlockSpec((MBLK, 128), lambda i: (i, 0)),
            pl.BlockSpec((MBLK, 128), lambda i: (i, 0)),
            pl.BlockSpec((MBLK, 128), lambda i: (i, 0)),
            pl.BlockSpec((MBLK, 128), lambda i: (i, 0)),
            pl.BlockSpec((MBLK, HID), lambda i: (i, 0)),
            pl.BlockSpec((MBLK, HID), lambda i: (i, 0)),
            pl.BlockSpec((8, HID), lambda i: (0, 0)),
            pl.BlockSpec((HID, HID), lambda i: (0, 0)),
            pl.BlockSpec((HID, 128), lambda i: (0, 0)),
        ],
        out_specs=[
            pl.BlockSpec((MBLK, 128), lambda i: (i, 0)),
            pl.BlockSpec((MBLK, 128), lambda i: (i, 0)),
            pl.BlockSpec((MBLK, 128), lambda i: (i, 0)),
        ],
        out_shape=[
            jax.ShapeDtypeStruct((N, 128), jnp.float32),
            jax.ShapeDtypeStruct((N, 128), jnp.float32),
            jax.ShapeDtypeStruct((N, 128), jnp.float32),
        ],
    )(sa, sb, ha, hb, exsx, invx, bpad, W2, A2)


def _k3_body(sa_ref, sb_ref, ha_ref, hb_ref, exs_ref, inv_ref, b_ref,
             w_ref, bc_ref, q_ref, k_ref, v_ref, skip_ref):
    scat = jnp.concatenate([sa_ref[...], sb_ref[...]], axis=1)
    h = jnp.concatenate([ha_ref[...], hb_ref[...]], axis=1)
    x2 = _elu((scat + h * exs_ref[...]) * inv_ref[...] + b_ref[0:1, :])
    y = jnp.dot(x2, w_ref[...], preferred_element_type=jnp.float32)
    y = y + bc_ref[0:1, :]
    q_ref[...] = y[:, 0:128]
    k_ref[...] = y[:, 128:256]
    v_ref[...] = y[:, 256:384]
    skip_ref[...] = y[:, 384:512]


def _k3(sa, sb, ha, hb, exsx, invx, bpad, Wcat, bcat):
    return pl.pallas_call(
        _k3_body,
        grid=(GRID,),
        in_specs=[
            pl.BlockSpec((MBLK, 128), lambda i: (i, 0)),
            pl.BlockSpec((MBLK, 128), lambda i: (i, 0)),
            pl.BlockSpec((MBLK, 128), lambda i: (i, 0)),
            pl.BlockSpec((MBLK, 128), lambda i: (i, 0)),
            pl.BlockSpec((MBLK, HID), lambda i: (i, 0)),
            pl.BlockSpec((MBLK, HID), lambda i: (i, 0)),
            pl.BlockSpec((8, HID), lambda i: (0, 0)),
            pl.BlockSpec((HID, 512), lambda i: (0, 0)),
            pl.BlockSpec((8, 512), lambda i: (0, 0)),
        ],
        out_specs=[pl.BlockSpec((MBLK, 128), lambda i: (i, 0))] * 4,
        out_shape=[jax.ShapeDtypeStruct((N, 128), jnp.float32)] * 4,
    )(sa, sb, ha, hb, exsx, invx, bpad, Wcat, bcat)


def _k4_body(sa_ref, sb_ref, inv_ref, skip_ref, x_ref, w_ref, b_ref, o_ref):
    acc = jnp.dot(x_ref[...], w_ref[...], preferred_element_type=jnp.float32)
    x3 = jnp.concatenate([sa_ref[...][:, 0:64], sb_ref[...][:, 0:64]], axis=1)
    o_ref[...] = jnp.tanh(x3 * inv_ref[...]
                          + skip_ref[...] + acc + b_ref[0:1, :])


def _k4(sa, sb, invx, skip, x, Wres, bpad):
    return pl.pallas_call(
        _k4_body,
        grid=(GRID,),
        in_specs=[
            pl.BlockSpec((MBLK, 128), lambda i: (i, 0)),
            pl.BlockSpec((MBLK, 128), lambda i: (i, 0)),
            pl.BlockSpec((MBLK, 128), lambda i: (i, 0)),
            pl.BlockSpec((MBLK, 128), lambda i: (i, 0)),
            pl.BlockSpec((MBLK, F_IN), lambda i: (i, 0)),
            pl.BlockSpec((F_IN, 128), lambda i: (0, 0)),
            pl.BlockSpec((8, 128), lambda i: (0, 0)),
        ],
        out_specs=pl.BlockSpec((MBLK, 128), lambda i: (i, 0)),
        out_shape=jax.ShapeDtypeStruct((N, 128), jnp.float32),
    )(sa, sb, invx, skip, x, Wres, bpad)


def _zero_vbuf(vbuf, rows):
    z = jnp.zeros((16,), jnp.float32)
    nv = vbuf.shape[1] // 16

    def zr(r, _):
        for j in range(nv):
            vbuf[r, pl.ds(j * 16, 16)] = z
        return 0

    lax.fori_loop(0, rows, zr, 0)


def _zero_acc(zb, acc_sh, sub):
    zr = zb.shape[0]
    _zero_vbuf(zb, zr)

    @pl.when(sub < NSUB - 1)
    def _():
        for k in range(RA // zr):
            pltpu.sync_copy(zb, acc_sh.at[pl.ds(sub * RA + k * zr, zr)])

    @pl.when(sub == NSUB - 1)
    def _():
        for k in range(RATAIL // zr):
            pltpu.sync_copy(zb, acc_sh.at[pl.ds((NSUB - 1) * RA + k * zr,
                                                zr)])
        rem = RATAIL % zr
        if rem:
            pltpu.sync_copy(zb.at[pl.ds(0, rem)],
                            acc_sh.at[pl.ds(N - rem, rem)])


def _drain_acc(acc_sh, out_hbm, core, sub, sem):
    @pl.when(sub < NSUB - 1)
    def _():
        pltpu.async_copy(acc_sh.at[pl.ds(sub * RA, RA)],
                         out_hbm.at[core, pl.ds(sub * RA, RA)], sem).wait()

    @pl.when(sub == NSUB - 1)
    def _():
        pltpu.async_copy(acc_sh.at[pl.ds((NSUB - 1) * RA, RATAIL)],
                         out_hbm.at[core, pl.ds((NSUB - 1) * RA, RATAIL)],
                         sem).wait()


def _sc_a_body(src_hbm, dst_hbm, t_hbm, ex_hbm, den_hbm,
               sidx, didx, rs0, rd0, rs1, rd1, exs, bufex, acc_sh,
               sem0, sem1):
    core = lax.axis_index("c")
    sub = lax.axis_index("s")
    wid = core * NSUB + sub
    _zero_acc(rs0, acc_sh, sub)
    _zero_vbuf(exs, CHA)
    plsc.subcore_barrier()

    trips = (NCHUNKA - wid + NW - 1) // NW
    bufs = [(rs0, rd0, sem0), (rs1, rd1, sem1)]

    def load_idx(kk):
        base = (wid + kk * NW) * CHA
        pltpu.async_copy(src_hbm.at[pl.ds(base, CHA)], sidx, sem0).wait()
        pltpu.async_copy(dst_hbm.at[pl.ds(base, CHA)], didx, sem0).wait()

    def issue(b):
        rs, rd, sem = bufs[b]
        pltpu.async_copy(t_hbm.at[sidx], rs, sem)
        pltpu.async_copy(t_hbm.at[didx], rd, sem)

    load_idx(0)
    issue(0)

    def step(b, kk, _):
        rs, rd, sem = bufs[b]
        pltpu.make_async_copy(t_hbm.at[sidx], rs, sem).wait()
        pltpu.make_async_copy(t_hbm.at[didx], rd, sem).wait()

        def ew(i, _):
            for u in range(4):
                e = i * 4 + u
                t = rs[e, pl.ds(0, 16)] + rd[e, pl.ds(16, 16)]
                al = jnp.maximum(t, 0.2 * t)
                ex = jnp.exp(al - rd[e, pl.ds(32, 16)])
                exs[e, pl.ds(0, 16)] = ex
                bufex[e >> 3, pl.ds((e & 7) * 16, 16)] = ex
            return 0

        lax.fori_loop(0, CHA // 4, ew, 0)
        ci = wid + kk * NW
        pltpu.async_copy(bufex, ex_hbm.at[pl.ds(ci * CHA8, CHA8)],
                         sem0).wait()
        pltpu.sync_copy(exs, acc_sh.at[didx], add=True)

        @pl.when(kk + 1 < trips)
        def _():
            load_idx(kk + 1)
            issue(1 - b)

        return 0

    def loop(kk, _):
        @pl.when(kk % 2 == 0)
        def _():
            step(0, kk, 0)

        @pl.when(kk % 2 == 1)
        def _():
            step(1, kk, 0)

        return 0

    lax.fori_loop(0, trips, loop, 0)
    plsc.subcore_barrier()
    _drain_acc(acc_sh, den_hbm, core, sub, sem0)


_sc_a = pl.kernel(
    _sc_a_body,
    out_type=[
        jax.ShapeDtypeStruct((EX_ROWS, 128), jnp.float32),
        jax.ShapeDtypeStruct((NCORE, N, 128), jnp.float32),
    ],
    mesh=_mesh,
    compiler_params=_sc_params,
    scratch_types=[
        pltpu.VMEM((CHA,), jnp.int32),
        pltpu.VMEM((CHA,), jnp.int32),
        pltpu.VMEM((CHA, 128), jnp.float32),
        pltpu.VMEM((CHA, 128), jnp.float32),
        pltpu.VMEM((CHA, 128), jnp.float32),
        pltpu.VMEM((CHA, 128), jnp.float32),
        pltpu.VMEM((CHA, 128), jnp.float32),
        pltpu.VMEM((CHA8, 128), jnp.float32),
        pltpu.VMEM_SHARED((N, 128), jnp.float32),
        pltpu.SemaphoreType.DMA,
        pltpu.SemaphoreType.DMA,
    ],
)


def _make_sc_b(hpc):
    vph = 8 // hpc

    def body(src_hbm, dst_hbm, ha_hbm, hb_hbm, ex_hbm,
             out_hbm, sidx, didx, rows0, rows1, exb0, exb1, acc_sh,
             sem0, sem1):
        core = lax.axis_index("c")
        sub = lax.axis_index("s")
        _zero_acc(rows0, acc_sh, sub)
        plsc.subcore_barrier()

        trips = (NCHUNK - sub + NSUB - 1) // NSUB
        zi = jnp.zeros((16,), jnp.int32)
        bufs = [(rows0, exb0, sem0), (rows1, exb1, sem1)]

        def load_idx(kk):
            base = (sub + kk * NSUB) * CH
            pltpu.async_copy(src_hbm.at[pl.ds(base, CH)], sidx,
                             sem0).wait()
            pltpu.async_copy(dst_hbm.at[pl.ds(base, CH)], didx,
                             sem0).wait()

        def issue(b, kk):
            rows, exb, sem = bufs[b]

            @pl.when(core == 0)
            def _():
                pltpu.async_copy(ha_hbm.at[sidx], rows, sem)

            @pl.when(core == 1)
            def _():
                pltpu.async_copy(hb_hbm.at[sidx], rows, sem)

            ci = sub + kk * NSUB
            pltpu.async_copy(ex_hbm.at[pl.ds(ci * CH8, CH8)], exb, sem)

        load_idx(0)
        issue(0, 0)

        def step(b, kk, _):
            rows, exb, sem = bufs[b]
            pltpu.make_async_copy(ha_hbm.at[sidx], rows, sem).wait()
            ci = sub + kk * NSUB
            pltpu.make_async_copy(ex_hbm.at[pl.ds(ci * CH8, CH8)], exb,
                                  sem).wait()

            hoff = hpc * core if hpc > 1 else 0

            def ew(i, _):
                ws = []
                for u in range(4):
                    e = i * 4 + u
                    r8 = zi + (e >> 3)
                    l0 = zi + ((e & 7) * 16 + hoff)
                    ws.append([plsc.load_gather(exb, [r8, l0 + j])
                               for j in range(hpc)])
                for u in range(4):
                    e = i * 4 + u
                    for j in range(hpc):
                        for v in range(vph):
                            col = (j * vph + v) * 16
                            rows[e, pl.ds(col, 16)] = \
                                rows[e, pl.ds(col, 16)] * ws[u][j]
                return 0

            lax.fori_loop(0, CH // 4, ew, 0)
            pltpu.sync_copy(rows, acc_sh.at[didx], add=True)

            @pl.when(kk + 1 < trips)
            def _():
                load_idx(kk + 1)
                issue(1 - b, kk + 1)

            return 0

        def loop(kk, _):
            @pl.when(kk % 2 == 0)
            def _():
                step(0, kk, 0)

            @pl.when(kk % 2 == 1)
            def _():
                step(1, kk, 0)

            return 0

        lax.fori_loop(0, trips, loop, 0)
        plsc.subcore_barrier()
        _drain_acc(acc_sh, out_hbm, core, sub, sem0)

    return pl.kernel(
        body,
        out_type=jax.ShapeDtypeStruct((NCORE, N, 128), jnp.float32),
        mesh=_mesh,
        compiler_params=_sc_params,
        scratch_types=[
            pltpu.VMEM((CH,), jnp.int32),
            pltpu.VMEM((CH,), jnp.int32),
            pltpu.VMEM((CH, 128), jnp.float32),
            pltpu.VMEM((CH, 128), jnp.float32),
            pltpu.VMEM((CH8, 128), jnp.float32),
            pltpu.VMEM((CH8, 128), jnp.float32),
            pltpu.VMEM_SHARED((N, 128), jnp.float32),
            pltpu.SemaphoreType.DMA,
            pltpu.SemaphoreType.DMA,
        ],
    )


_sc_b1 = _make_sc_b(4)
_sc_b23 = _make_sc_b(1)


def _sc_a3_body(src_hbm, dst_hbm, q_hbm, k_hbm, ex_hbm, den_hbm,
                sidx, didx, qb0, kb0, qb1, kb1, exs, bufex, alb, tmp,
                acc_sh, sem0, sem1):
    core = lax.axis_index("c")
    sub = lax.axis_index("s")
    wid = core * NSUB + sub
    _zero_acc(qb0, acc_sh, sub)
    _zero_vbuf(exs, CHA)
    plsc.subcore_barrier()

    trips = (NCHUNKA - wid + NW - 1) // NW
    lanes = lax.iota(jnp.int32, 16)
    bufs = [(qb0, kb0, sem0), (qb1, kb1, sem1)]

    def load_idx(kk):
        base = (wid + kk * NW) * CHA
        pltpu.async_copy(src_hbm.at[pl.ds(base, CHA)], sidx, sem0).wait()
        pltpu.async_copy(dst_hbm.at[pl.ds(base, CHA)], didx, sem0).wait()

    def issue(b):
        qb, kb, sem = bufs[b]
        pltpu.async_copy(q_hbm.at[didx], qb, sem)
        pltpu.async_copy(k_hbm.at[sidx], kb, sem)

    load_idx(0)
    issue(0)

    def step(b, kk, _):
        qb, kb, sem = bufs[b]
        pltpu.make_async_copy(q_hbm.at[didx], qb, sem).wait()
        pltpu.make_async_copy(k_hbm.at[sidx], kb, sem).wait()

        def dot_edge(i, _):
            for u in range(4):
                e = i * 4 + u
                pr = [qb[e, pl.ds(v * 16, 16)] * kb[e, pl.ds(v * 16, 16)]
                      for v in range(8)]
                acc = ((pr[0] + pr[1]) + (pr[2] + pr[3])) + \
                      ((pr[4] + pr[5]) + (pr[6] + pr[7]))
                tmp[e & 15, :] = acc
            return 0

        for g in range(CHA // 16):
            lax.fori_loop(g * 4, g * 4 + 4, dot_edge, 0)
            red = plsc.load_gather(tmp, [lanes, jnp.zeros((16,), jnp.int32)])
            for c in range(1, 16):
                red = red + plsc.load_gather(
                    tmp, [lanes, jnp.zeros((16,), jnp.int32) + c])
            alb[pl.ds(g * 16, 16)] = red

        for g in range(CHA // 16):
            ev = jnp.exp(alb[pl.ds(g * 16, 16)])
            plsc.store_scatter(exs, [g * 16 + lanes,
                                     jnp.zeros((16,), jnp.int32)], ev)
            plsc.store_scatter(bufex, [(g * 16 + lanes) >> 3,
                                       ((g * 16 + lanes) & 7) * 16], ev)

        ci = wid + kk * NW
        pltpu.async_copy(bufex, ex_hbm.at[pl.ds(ci * CHA8, CHA8)],
                         sem0).wait()
        pltpu.sync_copy(exs, acc_sh.at[didx], add=True)

        @pl.when(kk + 1 < trips)
        def _():
            load_idx(kk + 1)
            issue(1 - b)

        return 0

    def loop(kk, _):
        @pl.when(kk % 2 == 0)
        def _():
            step(0, kk, 0)

        @pl.when(kk % 2 == 1)
        def _():
            step(1, kk, 0)

        return 0

    lax.fori_loop(0, trips, loop, 0)
    plsc.subcore_barrier()
    _drain_acc(acc_sh, den_hbm, core, sub, sem0)


_sc_a3 = pl.kernel(
    _sc_a3_body,
    out_type=[
        jax.ShapeDtypeStruct((EX_ROWS, 128), jnp.float32),
        jax.ShapeDtypeStruct((NCORE, N, 128), jnp.float32),
    ],
    mesh=_mesh,
    compiler_params=_sc_params,
    scratch_types=[
        pltpu.VMEM((CHA,), jnp.int32),
        pltpu.VMEM((CHA,), jnp.int32),
        pltpu.VMEM((CHA, 128), jnp.float32),
        pltpu.VMEM((CHA, 128), jnp.float32),
        pltpu.VMEM((CHA, 128), jnp.float32),
        pltpu.VMEM((CHA, 128), jnp.float32),
        pltpu.VMEM((CHA, 128), jnp.float32),
        pltpu.VMEM((CHA8, 128), jnp.float32),
        pltpu.VMEM((CHA,), jnp.float32),
        pltpu.VMEM((16, 16), jnp.float32),
        pltpu.VMEM_SHARED((N, 128), jnp.float32),
        pltpu.SemaphoreType.DMA,
        pltpu.SemaphoreType.DMA,
    ],
)


def _leaky(v):
    return jnp.maximum(v, 0.2 * v)


def _den_slice(denp, h):
    return denp[0, :, 0:h] + denp[1, :, 0:h]


def kernel(x, edge_index, W1, att_src1, att_dst1, b1, W2, att_src2, att_dst2,
           b2, Wq, bq, Wk, bk, Wv, bv, Wskip, bskip, Wres, bres):
    src = edge_index[0]
    dst = edge_index[1]
    f32 = jnp.float32

    heads1, ch1 = att_src1.shape
    eye1 = jnp.eye(heads1, dtype=f32)
    blk_s = (att_src1[:, :, None] * eye1[:, None, :]).reshape(HID, heads1)
    blk_d = (att_dst1[:, :, None] * eye1[:, None, :]).reshape(HID, heads1)
    A1 = jnp.zeros((HID, 128), f32)
    A1 = A1.at[:, 0:8].set(blk_s).at[:, 16:24].set(blk_d)
    A2 = jnp.zeros((HID, 128), f32)
    A2 = A2.at[:, 0:1].set(att_src2.T).at[:, 16:17].set(att_dst2.T)

    scale = 1.0 / jnp.sqrt(jnp.float32(NC))
    Wcat = jnp.concatenate([Wq * scale, Wk, Wv, Wskip], axis=1)
    bcat = jnp.tile(jnp.concatenate([bq * scale, bk, bv, bskip])[None, :],
                    (8, 1))
    b1p = jnp.tile(b1[None, :], (8, 1))
    b2p = jnp.tile(b2[None, :], (8, 1))
    bresp = jnp.tile(bres[None, :], (8, 1))

    h1a, h1b, a1 = _k1(x, W1, A1)
    asrc1 = a1[:, 0:8]
    adst1 = a1[:, 16:24]
    gmax1 = jnp.max(asrc1, axis=0, keepdims=True)
    c1 = _leaky(adst1 + gmax1)
    exs1 = jnp.exp(_leaky(asrc1 + adst1) - c1)
    T1 = a1.at[:, 32:40].set(c1)

    ex1, den1p = _sc_a(src, dst, T1)
    den1 = _den_slice(den1p, 8) + exs1
    inv1 = 1.0 / (den1 + 1e-16)
    exsx1 = jnp.repeat(exs1, ch1, axis=1)
    invx1 = jnp.repeat(inv1, ch1, axis=1)

    scat1 = _sc_b1(src, dst, h1a, h1b, ex1)

    h2a, h2b, a2 = _k2(scat1[0], scat1[1], h1a, h1b, exsx1, invx1, b1p,
                       W2, A2)
    asrc2 = a2[:, 0:1]
    adst2 = a2[:, 16:17]
    gmax2 = jnp.max(asrc2, axis=0, keepdims=True)
    c2 = _leaky(adst2 + gmax2)
    exs2 = jnp.exp(_leaky(asrc2 + adst2) - c2)
    T2 = a2.at[:, 32:33].set(c2)

    ex2, den2p = _sc_a(src, dst, T2)
    den2 = _den_slice(den2p, 1) + exs2
    inv2 = 1.0 / (den2 + 1e-16)
    exsx2 = jnp.broadcast_to(exs2, (N, HID))
    invx2 = jnp.broadcast_to(inv2, (N, HID))

    scat2 = _sc_b23(src, dst, h2a, h2b, ex2)

    q3, k3, v3, skip3 = _k3(scat2[0], scat2[1], h2a, h2b, exsx2, invx2, b2p,
                            Wcat, bcat)

    ex3, den3p = _sc_a3(src, dst, q3, k3)
    den3 = _den_slice(den3p, 1)
    inv3 = 1.0 / (den3 + 1e-16)
    invx3 = jnp.broadcast_to(inv3, (N, 128))

    v3a = jnp.pad(v3[:, 0:64], ((0, 0), (0, 64)))
    v3b = jnp.pad(v3[:, 64:128], ((0, 0), (0, 64)))
    scat3 = _sc_b23(src, dst, v3a, v3b, ex3)

    return _k4(scat3[0], scat3[1], invx3, skip3, x, Wres, bresp)


# speedup vs baseline: 16.7361x; 1.1176x over previous
import jax
import jax.numpy as jnp
from jax import lax
from jax.experimental import pallas as pl
from jax.experimental.pallas import tpu as pltpu
from jax.experimental.pallas import tpu_sc as plsc

N = 10000
E = 160000
F_IN = 256
HID = 256
NC = 128

MBLK = 400
GRID = N // MBLK
CH = 128
CH8 = CH // 8
NCHUNK = E // CH
NSUB = 16
NCORE = 2
NW = NCORE * NSUB
RA = 640
RATAIL = N - (NSUB - 1) * RA
CHA = 64
CHA8 = CHA // 8
NCHUNKA = E // CHA
EX_ROWS = E // 8

_mesh = plsc.VectorSubcoreMesh(core_axis_name="c", subcore_axis_name="s")
_sc_params = pltpu.CompilerParams(needs_layout_passes=False)


def _k1_body(x_ref, w_ref, a_ref, ha_ref, hb_ref, a1_ref):
    h = jnp.dot(x_ref[...], w_ref[...], preferred_element_type=jnp.float32)
    ha_ref[...] = h[:, :128]
    hb_ref[...] = h[:, 128:]
    a1_ref[...] = jnp.dot(h, a_ref[...], preferred_element_type=jnp.float32)


def _k1(x, W1, A1):
    return pl.pallas_call(
        _k1_body,
        grid=(GRID,),
        in_specs=[
            pl.BlockSpec((MBLK, F_IN), lambda i: (i, 0)),
            pl.BlockSpec((F_IN, HID), lambda i: (0, 0)),
            pl.BlockSpec((HID, 128), lambda i: (0, 0)),
        ],
        out_specs=[
            pl.BlockSpec((MBLK, 128), lambda i: (i, 0)),
            pl.BlockSpec((MBLK, 128), lambda i: (i, 0)),
            pl.BlockSpec((MBLK, 128), lambda i: (i, 0)),
        ],
        out_shape=[
            jax.ShapeDtypeStruct((N, 128), jnp.float32),
            jax.ShapeDtypeStruct((N, 128), jnp.float32),
            jax.ShapeDtypeStruct((N, 128), jnp.float32),
        ],
    )(x, W1, A1)


def _elu(v):
    return jnp.where(v > 0, v, jnp.exp(v) - 1.0)


def _k2_body(sa_ref, sb_ref, ha_ref, hb_ref, exs_ref, inv_ref, b_ref,
             w_ref, a_ref, h2a_ref, h2b_ref, a2_ref):
    scat = jnp.concatenate([sa_ref[...], sb_ref[...]], axis=1)
    h = jnp.concatenate([ha_ref[...], hb_ref[...]], axis=1)
    x1 = _elu((scat + h * exs_ref[...]) * inv_ref[...] + b_ref[0:1, :])
    h2 = jnp.dot(x1, w_ref[...], preferred_element_type=jnp.float32)
    h2a_ref[...] = h2[:, :128]
    h2b_ref[...] = h2[:, 128:]
    a2_ref[...] = jnp.dot(h2, a_ref[...], preferred_element_type=jnp.float32)


def _k2(sa, sb, ha, hb, exsx, invx, bpad, W2, A2):
    return pl.pallas_call(
        _k2_body,
        grid=(GRID,),
        in_specs=[
            pl.BlockSpec((MBLK, 128), lambda i: (i, 0)),
            pl.BlockSpec((MBLK, 128), lambda i: (i, 0)),
            pl.BlockSpec((MBLK, 128), lambda i: (i, 0)),
            pl.BlockSpec((MBLK, 128), lambda i: (i, 0)),
            pl.BlockSpec((MBLK, HID), lambda i: (i, 0)),
            pl.BlockSpec((MBLK, HID), lambda i: (i, 0)),
            pl.BlockSpec((8, HID), lambda i: (0, 0)),
            pl.BlockSpec((HID, HID), lambda i: (0, 0)),
            pl.BlockSpec((HID, 128), lambda i: (0, 0)),
        ],
        out_specs=[
            pl.BlockSpec((MBLK, 128), lambda i: (i, 0)),
            pl.BlockSpec((MBLK, 128), lambda i: (i, 0)),
            pl.BlockSpec((MBLK, 128), lambda i: (i, 0)),
        ],
        out_shape=[
            jax.ShapeDtypeStruct((N, 128), jnp.float32),
            jax.ShapeDtypeStruct((N, 128), jnp.float32),
            jax.ShapeDtypeStruct((N, 128), jnp.float32),
        ],
    )(sa, sb, ha, hb, exsx, invx, bpad, W2, A2)


def _k3_body(sa_ref, sb_ref, ha_ref, hb_ref, exs_ref, inv_ref, b_ref,
             w_ref, bc_ref, q_ref, k_ref, v_ref, skip_ref):
    scat = jnp.concatenate([sa_ref[...], sb_ref[...]], axis=1)
    h = jnp.concatenate([ha_ref[...], hb_ref[...]], axis=1)
    x2 = _elu((scat + h * exs_ref[...]) * inv_ref[...] + b_ref[0:1, :])
    y = jnp.dot(x2, w_ref[...], preferred_element_type=jnp.float32)
    y = y + bc_ref[0:1, :]
    q_ref[...] = y[:, 0:128]
    k_ref[...] = y[:, 128:256]
    v_ref[...] = y[:, 256:384]
    skip_ref[...] = y[:, 384:512]


def _k3(sa, sb, ha, hb, exsx, invx, bpad, Wcat, bcat):
    return pl.pallas_call(
        _k3_body,
        grid=(GRID,),
        in_specs=[
            pl.BlockSpec((MBLK, 128), lambda i: (i, 0)),
            pl.BlockSpec((MBLK, 128), lambda i: (i, 0)),
            pl.BlockSpec((MBLK, 128), lambda i: (i, 0)),
            pl.BlockSpec((MBLK, 128), lambda i: (i, 0)),
            pl.BlockSpec((MBLK, HID), lambda i: (i, 0)),
            pl.BlockSpec((MBLK, HID), lambda i: (i, 0)),
            pl.BlockSpec((8, HID), lambda i: (0, 0)),
            pl.BlockSpec((HID, 512), lambda i: (0, 0)),
            pl.BlockSpec((8, 512), lambda i: (0, 0)),
        ],
        out_specs=[pl.BlockSpec((MBLK, 128), lambda i: (i, 0))] * 4,
        out_shape=[jax.ShapeDtypeStruct((N, 128), jnp.float32)] * 4,
    )(sa, sb, ha, hb, exsx, invx, bpad, Wcat, bcat)


def _k4_body(sa_ref, sb_ref, inv_ref, skip_ref, x_ref, w_ref, b_ref, o_ref):
    acc = jnp.dot(x_ref[...], w_ref[...], preferred_element_type=jnp.float32)
    x3 = jnp.concatenate([sa_ref[...][:, 0:64], sb_ref[...][:, 0:64]], axis=1)
    o_ref[...] = jnp.tanh(x3 * inv_ref[...]
                          + skip_ref[...] + acc + b_ref[0:1, :])


def _k4(sa, sb, invx, skip, x, Wres, bpad):
    return pl.pallas_call(
        _k4_body,
        grid=(GRID,),
        in_specs=[
            pl.BlockSpec((MBLK, 128), lambda i: (i, 0)),
            pl.BlockSpec((MBLK, 128), lambda i: (i, 0)),
            pl.BlockSpec((MBLK, 128), lambda i: (i, 0)),
            pl.BlockSpec((MBLK, 128), lambda i: (i, 0)),
            pl.BlockSpec((MBLK, F_IN), lambda i: (i, 0)),
            pl.BlockSpec((F_IN, 128), lambda i: (0, 0)),
            pl.BlockSpec((8, 128), lambda i: (0, 0)),
        ],
        out_specs=pl.BlockSpec((MBLK, 128), lambda i: (i, 0)),
        out_shape=jax.ShapeDtypeStruct((N, 128), jnp.float32),
    )(sa, sb, invx, skip, x, Wres, bpad)


def _zero_vbuf(vbuf, rows):
    z = jnp.zeros((16,), jnp.float32)
    nv = vbuf.shape[1] // 16

    def zr(r, _):
        for j in range(nv):
            vbuf[r, pl.ds(j * 16, 16)] = z
        return 0

    lax.fori_loop(0, rows, zr, 0)


def _zero_acc(zb, acc_sh, sub):
    zr = zb.shape[0]
    _zero_vbuf(zb, zr)

    @pl.when(sub < NSUB - 1)
    def _():
        for k in range(RA // zr):
            pltpu.sync_copy(zb, acc_sh.at[pl.ds(sub * RA + k * zr, zr)])

    @pl.when(sub == NSUB - 1)
    def _():
        for k in range(RATAIL // zr):
            pltpu.sync_copy(zb, acc_sh.at[pl.ds((NSUB - 1) * RA + k * zr,
                                                zr)])
        rem = RATAIL % zr
        if rem:
            pltpu.sync_copy(zb.at[pl.ds(0, rem)],
                            acc_sh.at[pl.ds(N - rem, rem)])


def _drain_acc(acc_sh, out_hbm, core, sub, sem):
    @pl.when(sub < NSUB - 1)
    def _():
        pltpu.async_copy(acc_sh.at[pl.ds(sub * RA, RA)],
                         out_hbm.at[core, pl.ds(sub * RA, RA)], sem).wait()

    @pl.when(sub == NSUB - 1)
    def _():
        pltpu.async_copy(acc_sh.at[pl.ds((NSUB - 1) * RA, RATAIL)],
                         out_hbm.at[core, pl.ds((NSUB - 1) * RA, RATAIL)],
                         sem).wait()


def _sc_a_body(src_hbm, dst_hbm, t_hbm, ex_hbm, den_hbm,
               sidx, didx, rs0, rd0, rs1, rd1, exs, bufex, acc_sh,
               sem0, sem1):
    core = lax.axis_index("c")
    sub = lax.axis_index("s")
    wid = core * NSUB + sub
    _zero_acc(rs0, acc_sh, sub)
    _zero_vbuf(exs, CHA)
    plsc.subcore_barrier()

    trips = (NCHUNKA - wid + NW - 1) // NW
    bufs = [(rs0, rd0, sem0), (rs1, rd1, sem1)]

    def load_idx(kk):
        base = (wid + kk * NW) * CHA
        d1 = pltpu.async_copy(src_hbm.at[pl.ds(base, CHA)], sidx, sem0)
        d2 = pltpu.async_copy(dst_hbm.at[pl.ds(base, CHA)], didx, sem0)
        d1.wait()
        d2.wait()

    def issue(b):
        rs, rd, sem = bufs[b]
        pltpu.async_copy(t_hbm.at[sidx], rs, sem)
        pltpu.async_copy(t_hbm.at[didx], rd, sem)

    load_idx(0)
    issue(0)

    def step(b, kk, _):
        rs, rd, sem = bufs[b]
        pltpu.make_async_copy(t_hbm.at[sidx], rs, sem).wait()
        pltpu.make_async_copy(t_hbm.at[didx], rd, sem).wait()

        def ew(i, _):
            for u in range(4):
                e = i * 4 + u
                t = rs[e, pl.ds(0, 16)] + rd[e, pl.ds(16, 16)]
                al = jnp.maximum(t, 0.2 * t)
                ex = jnp.exp(al - rd[e, pl.ds(32, 16)])
                exs[e, pl.ds(0, 16)] = ex
                bufex[e >> 3, pl.ds((e & 7) * 16, 16)] = ex
            return 0

        lax.fori_loop(0, CHA // 4, ew, 0)
        ci = wid + kk * NW
        dex = pltpu.async_copy(bufex, ex_hbm.at[pl.ds(ci * CHA8, CHA8)],
                               sem0)
        pltpu.sync_copy(exs, acc_sh.at[didx], add=True)
        dex.wait()

        @pl.when(kk + 1 < trips)
        def _():
            load_idx(kk + 1)
            issue(1 - b)

        return 0

    def loop(kk, _):
        @pl.when(kk % 2 == 0)
        def _():
            step(0, kk, 0)

        @pl.when(kk % 2 == 1)
        def _():
            step(1, kk, 0)

        return 0

    lax.fori_loop(0, trips, loop, 0)
    plsc.subcore_barrier()
    _drain_acc(acc_sh, den_hbm, core, sub, sem0)


_sc_a = pl.kernel(
    _sc_a_body,
    out_type=[
        jax.ShapeDtypeStruct((EX_ROWS, 128), jnp.float32),
        jax.ShapeDtypeStruct((NCORE, N, 128), jnp.float32),
    ],
    mesh=_mesh,
    compiler_params=_sc_params,
    scratch_types=[
        pltpu.VMEM((CHA,), jnp.int32),
        pltpu.VMEM((CHA,), jnp.int32),
        pltpu.VMEM((CHA, 128), jnp.float32),
        pltpu.VMEM((CHA, 128), jnp.float32),
        pltpu.VMEM((CHA, 128), jnp.float32),
        pltpu.VMEM((CHA, 128), jnp.float32),
        pltpu.VMEM((CHA, 128), jnp.float32),
        pltpu.VMEM((CHA8, 128), jnp.float32),
        pltpu.VMEM_SHARED((N, 128), jnp.float32),
        pltpu.SemaphoreType.DMA,
        pltpu.SemaphoreType.DMA,
    ],
)


def _make_sc_b(hpc):
    vph = 8 // hpc

    def body(src_hbm, dst_hbm, ha_hbm, hb_hbm, ex_hbm,
             out_hbm, sidx, didx, rows0, rows1, exb0, exb1, acc_sh,
             sem0, sem1):
        core = lax.axis_index("c")
        sub = lax.axis_index("s")
        _zero_acc(rows0, acc_sh, sub)
        plsc.subcore_barrier()

        trips = (NCHUNK - sub + NSUB - 1) // NSUB
        zi = jnp.zeros((16,), jnp.int32)
        bufs = [(rows0, exb0, sem0), (rows1, exb1, sem1)]

        def load_idx(kk):
            base = (sub + kk * NSUB) * CH
            d1 = pltpu.async_copy(src_hbm.at[pl.ds(base, CH)], sidx, sem0)
            d2 = pltpu.async_copy(dst_hbm.at[pl.ds(base, CH)], didx, sem0)
            d1.wait()
            d2.wait()

        def issue(b, kk):
            rows, exb, sem = bufs[b]

            @pl.when(core == 0)
            def _():
                pltpu.async_copy(ha_hbm.at[sidx], rows, sem)

            @pl.when(core == 1)
            def _():
                pltpu.async_copy(hb_hbm.at[sidx], rows, sem)

            ci = sub + kk * NSUB
            pltpu.async_copy(ex_hbm.at[pl.ds(ci * CH8, CH8)], exb, sem)

        load_idx(0)
        issue(0, 0)

        def step(b, kk, _):
            rows, exb, sem = bufs[b]
            pltpu.make_async_copy(ha_hbm.at[sidx], rows, sem).wait()
            ci = sub + kk * NSUB
            pltpu.make_async_copy(ex_hbm.at[pl.ds(ci * CH8, CH8)], exb,
                                  sem).wait()

            hoff = hpc * core if hpc > 1 else 0

            def ew(i, _):
                ws = []
                for u in range(4):
                    e = i * 4 + u
                    r8 = zi + (e >> 3)
                    l0 = zi + ((e & 7) * 16 + hoff)
                    ws.append([plsc.load_gather(exb, [r8, l0 + j])
                               for j in range(hpc)])
                for u in range(4):
                    e = i * 4 + u
                    for j in range(hpc):
                        for v in range(vph):
                            col = (j * vph + v) * 16
                            rows[e, pl.ds(col, 16)] = \
                                rows[e, pl.ds(col, 16)] * ws[u][j]
                return 0

            lax.fori_loop(0, CH // 4, ew, 0)
            pltpu.sync_copy(rows, acc_sh.at[didx], add=True)

            @pl.when(kk + 1 < trips)
            def _():
                load_idx(kk + 1)
                issue(1 - b, kk + 1)

            return 0

        def loop(kk, _):
            @pl.when(kk % 2 == 0)
            def _():
                step(0, kk, 0)

            @pl.when(kk % 2 == 1)
            def _():
                step(1, kk, 0)

            return 0

        lax.fori_loop(0, trips, loop, 0)
        plsc.subcore_barrier()
        _drain_acc(acc_sh, out_hbm, core, sub, sem0)

    return pl.kernel(
        body,
        out_type=jax.ShapeDtypeStruct((NCORE, N, 128), jnp.float32),
        mesh=_mesh,
        compiler_params=_sc_params,
        scratch_types=[
            pltpu.VMEM((CH,), jnp.int32),
            pltpu.VMEM((CH,), jnp.int32),
            pltpu.VMEM((CH, 128), jnp.float32),
            pltpu.VMEM((CH, 128), jnp.float32),
            pltpu.VMEM((CH8, 128), jnp.float32),
            pltpu.VMEM((CH8, 128), jnp.float32),
            pltpu.VMEM_SHARED((N, 128), jnp.float32),
            pltpu.SemaphoreType.DMA,
            pltpu.SemaphoreType.DMA,
        ],
    )


_sc_b1 = _make_sc_b(4)
_sc_b23 = _make_sc_b(1)


def _sc_a3_body(src_hbm, dst_hbm, q_hbm, k_hbm, ex_hbm, den_hbm,
                sidx, didx, qb0, kb0, qb1, kb1, exs, bufex, alb, tmp,
                acc_sh, sem0, sem1):
    core = lax.axis_index("c")
    sub = lax.axis_index("s")
    wid = core * NSUB + sub
    _zero_acc(qb0, acc_sh, sub)
    _zero_vbuf(exs, CHA)
    plsc.subcore_barrier()

    trips = (NCHUNKA - wid + NW - 1) // NW
    lanes = lax.iota(jnp.int32, 16)
    bufs = [(qb0, kb0, sem0), (qb1, kb1, sem1)]

    def load_idx(kk):
        base = (wid + kk * NW) * CHA
        d1 = pltpu.async_copy(src_hbm.at[pl.ds(base, CHA)], sidx, sem0)
        d2 = pltpu.async_copy(dst_hbm.at[pl.ds(base, CHA)], didx, sem0)
        d1.wait()
        d2.wait()

    def issue(b):
        qb, kb, sem = bufs[b]
        pltpu.async_copy(q_hbm.at[didx], qb, sem)
        pltpu.async_copy(k_hbm.at[sidx], kb, sem)

    load_idx(0)
    issue(0)

    def step(b, kk, _):
        qb, kb, sem = bufs[b]
        pltpu.make_async_copy(q_hbm.at[didx], qb, sem).wait()
        pltpu.make_async_copy(k_hbm.at[sidx], kb, sem).wait()

        def dot_edge(i, _):
            for u in range(4):
                e = i * 4 + u
                pr = [qb[e, pl.ds(v * 16, 16)] * kb[e, pl.ds(v * 16, 16)]
                      for v in range(8)]
                acc = ((pr[0] + pr[1]) + (pr[2] + pr[3])) + \
                      ((pr[4] + pr[5]) + (pr[6] + pr[7]))
                tmp[e & 15, :] = acc
            return 0

        for g in range(CHA // 16):
            lax.fori_loop(g * 4, g * 4 + 4, dot_edge, 0)
            red = plsc.load_gather(tmp, [lanes, jnp.zeros((16,), jnp.int32)])
            for c in range(1, 16):
                red = red + plsc.load_gather(
                    tmp, [lanes, jnp.zeros((16,), jnp.int32) + c])
            alb[pl.ds(g * 16, 16)] = red

        for g in range(CHA // 16):
            ev = jnp.exp(alb[pl.ds(g * 16, 16)])
            plsc.store_scatter(exs, [g * 16 + lanes,
                                     jnp.zeros((16,), jnp.int32)], ev)
            plsc.store_scatter(bufex, [(g * 16 + lanes) >> 3,
                                       ((g * 16 + lanes) & 7) * 16], ev)

        ci = wid + kk * NW
        dex = pltpu.async_copy(bufex, ex_hbm.at[pl.ds(ci * CHA8, CHA8)],
                               sem0)
        pltpu.sync_copy(exs, acc_sh.at[didx], add=True)
        dex.wait()

        @pl.when(kk + 1 < trips)
        def _():
            load_idx(kk + 1)
            issue(1 - b)

        return 0

    def loop(kk, _):
        @pl.when(kk % 2 == 0)
        def _():
            step(0, kk, 0)

        @pl.when(kk % 2 == 1)
        def _():
            step(1, kk, 0)

        return 0

    lax.fori_loop(0, trips, loop, 0)
    plsc.subcore_barrier()
    _drain_acc(acc_sh, den_hbm, core, sub, sem0)


_sc_a3 = pl.kernel(
    _sc_a3_body,
    out_type=[
        jax.ShapeDtypeStruct((EX_ROWS, 128), jnp.float32),
        jax.ShapeDtypeStruct((NCORE, N, 128), jnp.float32),
    ],
    mesh=_mesh,
    compiler_params=_sc_params,
    scratch_types=[
        pltpu.VMEM((CHA,), jnp.int32),
        pltpu.VMEM((CHA,), jnp.int32),
        pltpu.VMEM((CHA, 128), jnp.float32),
        pltpu.VMEM((CHA, 128), jnp.float32),
        pltpu.VMEM((CHA, 128), jnp.float32),
        pltpu.VMEM((CHA, 128), jnp.float32),
        pltpu.VMEM((CHA, 128), jnp.float32),
        pltpu.VMEM((CHA8, 128), jnp.float32),
        pltpu.VMEM((CHA,), jnp.float32),
        pltpu.VMEM((16, 16), jnp.float32),
        pltpu.VMEM_SHARED((N, 128), jnp.float32),
        pltpu.SemaphoreType.DMA,
        pltpu.SemaphoreType.DMA,
    ],
)


def _leaky(v):
    return jnp.maximum(v, 0.2 * v)


def _den_slice(denp, h):
    return denp[0, :, 0:h] + denp[1, :, 0:h]


def kernel(x, edge_index, W1, att_src1, att_dst1, b1, W2, att_src2, att_dst2,
           b2, Wq, bq, Wk, bk, Wv, bv, Wskip, bskip, Wres, bres):
    src = edge_index[0]
    dst = edge_index[1]
    f32 = jnp.float32

    heads1, ch1 = att_src1.shape
    eye1 = jnp.eye(heads1, dtype=f32)
    blk_s = (att_src1[:, :, None] * eye1[:, None, :]).reshape(HID, heads1)
    blk_d = (att_dst1[:, :, None] * eye1[:, None, :]).reshape(HID, heads1)
    A1 = jnp.zeros((HID, 128), f32)
    A1 = A1.at[:, 0:8].set(blk_s).at[:, 16:24].set(blk_d)
    A2 = jnp.zeros((HID, 128), f32)
    A2 = A2.at[:, 0:1].set(att_src2.T).at[:, 16:17].set(att_dst2.T)

    scale = 1.0 / jnp.sqrt(jnp.float32(NC))
    Wcat = jnp.concatenate([Wq * scale, Wk, Wv, Wskip], axis=1)
    bcat = jnp.tile(jnp.concatenate([bq * scale, bk, bv, bskip])[None, :],
                    (8, 1))
    b1p = jnp.tile(b1[None, :], (8, 1))
    b2p = jnp.tile(b2[None, :], (8, 1))
    bresp = jnp.tile(bres[None, :], (8, 1))

    h1a, h1b, a1 = _k1(x, W1, A1)
    asrc1 = a1[:, 0:8]
    adst1 = a1[:, 16:24]
    gmax1 = jnp.max(asrc1, axis=0, keepdims=True)
    c1 = _leaky(adst1 + gmax1)
    exs1 = jnp.exp(_leaky(asrc1 + adst1) - c1)
    T1 = a1.at[:, 32:40].set(c1)

    ex1, den1p = _sc_a(src, dst, T1)
    den1 = _den_slice(den1p, 8) + exs1
    inv1 = 1.0 / (den1 + 1e-16)
    exsx1 = jnp.repeat(exs1, ch1, axis=1)
    invx1 = jnp.repeat(inv1, ch1, axis=1)

    scat1 = _sc_b1(src, dst, h1a, h1b, ex1)

    h2a, h2b, a2 = _k2(scat1[0], scat1[1], h1a, h1b, exsx1, invx1, b1p,
                       W2, A2)
    asrc2 = a2[:, 0:1]
    adst2 = a2[:, 16:17]
    gmax2 = jnp.max(asrc2, axis=0, keepdims=True)
    c2 = _leaky(adst2 + gmax2)
    exs2 = jnp.exp(_leaky(asrc2 + adst2) - c2)
    T2 = a2.at[:, 32:33].set(c2)

    ex2, den2p = _sc_a(src, dst, T2)
    den2 = _den_slice(den2p, 1) + exs2
    inv2 = 1.0 / (den2 + 1e-16)
    exsx2 = jnp.broadcast_to(exs2, (N, HID))
    invx2 = jnp.broadcast_to(inv2, (N, HID))

    scat2 = _sc_b23(src, dst, h2a, h2b, ex2)

    q3, k3, v3, skip3 = _k3(scat2[0], scat2[1], h2a, h2b, exsx2, invx2, b2p,
                            Wcat, bcat)

    ex3, den3p = _sc_a3(src, dst, q3, k3)
    den3 = _den_slice(den3p, 1)
    inv3 = 1.0 / (den3 + 1e-16)
    invx3 = jnp.broadcast_to(inv3, (N, 128))

    v3a = jnp.pad(v3[:, 0:64], ((0, 0), (0, 64)))
    v3b = jnp.pad(v3[:, 64:128], ((0, 0), (0, 64)))
    scat3 = _sc_b23(src, dst, v3a, v3b, ex3)

    return _k4(scat3[0], scat3[1], invx3, skip3, x, Wres, bresp)


# speedup vs baseline: 18.8479x; 1.1262x over previous
import jax
import jax.numpy as jnp
from jax import lax
from jax.experimental import pallas as pl
from jax.experimental.pallas import tpu as pltpu
from jax.experimental.pallas import tpu_sc as plsc

N = 10000
E = 160000
F_IN = 256
HID = 256
NC = 128

MBLK = 400
GRID = N // MBLK
CH = 128
CH8 = CH // 8
NCHUNK = E // CH
NSUB = 16
NCORE = 2
NW = NCORE * NSUB
RA = 640
RATAIL = N - (NSUB - 1) * RA
CHA = 64
CHA8 = CHA // 8
NCHUNKA = E // CHA
EX_ROWS = E // 8

_mesh = plsc.VectorSubcoreMesh(core_axis_name="c", subcore_axis_name="s")
_sc_params = pltpu.CompilerParams(needs_layout_passes=False)


def _k1_body(x_ref, w_ref, a_ref, ha_ref, hb_ref, a1_ref):
    h = jnp.dot(x_ref[...], w_ref[...], preferred_element_type=jnp.float32)
    ha_ref[...] = h[:, :128]
    hb_ref[...] = h[:, 128:]
    a1_ref[...] = jnp.dot(h, a_ref[...], preferred_element_type=jnp.float32)


def _k1(x, W1, A1):
    return pl.pallas_call(
        _k1_body,
        grid=(GRID,),
        in_specs=[
            pl.BlockSpec((MBLK, F_IN), lambda i: (i, 0)),
            pl.BlockSpec((F_IN, HID), lambda i: (0, 0)),
            pl.BlockSpec((HID, 128), lambda i: (0, 0)),
        ],
        out_specs=[
            pl.BlockSpec((MBLK, 128), lambda i: (i, 0)),
            pl.BlockSpec((MBLK, 128), lambda i: (i, 0)),
            pl.BlockSpec((MBLK, 128), lambda i: (i, 0)),
        ],
        out_shape=[
            jax.ShapeDtypeStruct((N, 128), jnp.float32),
            jax.ShapeDtypeStruct((N, 128), jnp.float32),
            jax.ShapeDtypeStruct((N, 128), jnp.float32),
        ],
    )(x, W1, A1)


def _elu(v):
    return jnp.where(v > 0, v, jnp.exp(v) - 1.0)


def _k2_body(sa_ref, sb_ref, ha_ref, hb_ref, exs_ref, inv_ref, b_ref,
             w_ref, a_ref, h2a_ref, h2b_ref, a2_ref):
    scat = jnp.concatenate([sa_ref[...], sb_ref[...]], axis=1)
    h = jnp.concatenate([ha_ref[...], hb_ref[...]], axis=1)
    x1 = _elu((scat + h * exs_ref[...]) * inv_ref[...] + b_ref[0:1, :])
    h2 = jnp.dot(x1, w_ref[...], preferred_element_type=jnp.float32)
    h2a_ref[...] = h2[:, :128]
    h2b_ref[...] = h2[:, 128:]
    a2_ref[...] = jnp.dot(h2, a_ref[...], preferred_element_type=jnp.float32)


def _k2(sa, sb, ha, hb, exsx, invx, bpad, W2, A2):
    return pl.pallas_call(
        _k2_body,
        grid=(GRID,),
        in_specs=[
            pl.BlockSpec((MBLK, 128), lambda i: (i, 0)),
            pl.BlockSpec((MBLK, 128), lambda i: (i, 0)),
            pl.BlockSpec((MBLK, 128), lambda i: (i, 0)),
            pl.BlockSpec((MBLK, 128), lambda i: (i, 0)),
            pl.BlockSpec((MBLK, HID), lambda i: (i, 0)),
            pl.BlockSpec((MBLK, HID), lambda i: (i, 0)),
            pl.BlockSpec((8, HID), lambda i: (0, 0)),
            pl.BlockSpec((HID, HID), lambda i: (0, 0)),
            pl.BlockSpec((HID, 128), lambda i: (0, 0)),
        ],
        out_specs=[
            pl.BlockSpec((MBLK, 128), lambda i: (i, 0)),
            pl.BlockSpec((MBLK, 128), lambda i: (i, 0)),
            pl.BlockSpec((MBLK, 128), lambda i: (i, 0)),
        ],
        out_shape=[
            jax.ShapeDtypeStruct((N, 128), jnp.float32),
            jax.ShapeDtypeStruct((N, 128), jnp.float32),
            jax.ShapeDtypeStruct((N, 128), jnp.float32),
        ],
    )(sa, sb, ha, hb, exsx, invx, bpad, W2, A2)


def _k3_body(sa_ref, sb_ref, ha_ref, hb_ref, exs_ref, inv_ref, b_ref,
             w_ref, bc_ref, q_ref, k_ref, v_ref, skip_ref):
    scat = jnp.concatenate([sa_ref[...], sb_ref[...]], axis=1)
    h = jnp.concatenate([ha_ref[...], hb_ref[...]], axis=1)
    x2 = _elu((scat + h * exs_ref[...]) * inv_ref[...] + b_ref[0:1, :])
    y = jnp.dot(x2, w_ref[...], preferred_element_type=jnp.float32)
    y = y + bc_ref[0:1, :]
    q_ref[...] = y[:, 0:128]
    k_ref[...] = y[:, 128:256]
    v_ref[...] = y[:, 256:384]
    skip_ref[...] = y[:, 384:512]


def _k3(sa, sb, ha, hb, exsx, invx, bpad, Wcat, bcat):
    return pl.pallas_call(
        _k3_body,
        grid=(GRID,),
        in_specs=[
            pl.BlockSpec((MBLK, 128), lambda i: (i, 0)),
            pl.BlockSpec((MBLK, 128), lambda i: (i, 0)),
            pl.BlockSpec((MBLK, 128), lambda i: (i, 0)),
            pl.BlockSpec((MBLK, 128), lambda i: (i, 0)),
            pl.BlockSpec((MBLK, HID), lambda i: (i, 0)),
            pl.BlockSpec((MBLK, HID), lambda i: (i, 0)),
            pl.BlockSpec((8, HID), lambda i: (0, 0)),
            pl.BlockSpec((HID, 512), lambda i: (0, 0)),
            pl.BlockSpec((8, 512), lambda i: (0, 0)),
        ],
        out_specs=[pl.BlockSpec((MBLK, 128), lambda i: (i, 0))] * 4,
        out_shape=[jax.ShapeDtypeStruct((N, 128), jnp.float32)] * 4,
    )(sa, sb, ha, hb, exsx, invx, bpad, Wcat, bcat)


def _k4_body(sa_ref, sb_ref, inv_ref, skip_ref, x_ref, w_ref, b_ref, o_ref):
    acc = jnp.dot(x_ref[...], w_ref[...], preferred_element_type=jnp.float32)
    x3 = jnp.concatenate([sa_ref[...][:, 0:64], sb_ref[...][:, 0:64]], axis=1)
    o_ref[...] = jnp.tanh(x3 * inv_ref[...]
                          + skip_ref[...] + acc + b_ref[0:1, :])


def _k4(sa, sb, invx, skip, x, Wres, bpad):
    return pl.pallas_call(
        _k4_body,
        grid=(GRID,),
        in_specs=[
            pl.BlockSpec((MBLK, 128), lambda i: (i, 0)),
            pl.BlockSpec((MBLK, 128), lambda i: (i, 0)),
            pl.BlockSpec((MBLK, 128), lambda i: (i, 0)),
            pl.BlockSpec((MBLK, 128), lambda i: (i, 0)),
            pl.BlockSpec((MBLK, F_IN), lambda i: (i, 0)),
            pl.BlockSpec((F_IN, 128), lambda i: (0, 0)),
            pl.BlockSpec((8, 128), lambda i: (0, 0)),
        ],
        out_specs=pl.BlockSpec((MBLK, 128), lambda i: (i, 0)),
        out_shape=jax.ShapeDtypeStruct((N, 128), jnp.float32),
    )(sa, sb, invx, skip, x, Wres, bpad)


def _zero_vbuf(vbuf, rows):
    z = jnp.zeros((16,), jnp.float32)
    nv = vbuf.shape[1] // 16

    def zr(r, _):
        for j in range(nv):
            vbuf[r, pl.ds(j * 16, 16)] = z
        return 0

    lax.fori_loop(0, rows, zr, 0)


def _zero_acc(zb, acc_sh, sub):
    zr = zb.shape[0]
    _zero_vbuf(zb, zr)

    @pl.when(sub < NSUB - 1)
    def _():
        for k in range(RA // zr):
            pltpu.sync_copy(zb, acc_sh.at[pl.ds(sub * RA + k * zr, zr)])

    @pl.when(sub == NSUB - 1)
    def _():
        for k in range(RATAIL // zr):
            pltpu.sync_copy(zb, acc_sh.at[pl.ds((NSUB - 1) * RA + k * zr,
                                                zr)])
        rem = RATAIL % zr
        if rem:
            pltpu.sync_copy(zb.at[pl.ds(0, rem)],
                            acc_sh.at[pl.ds(N - rem, rem)])


def _drain_acc(acc_sh, out_hbm, core, sub, sem):
    @pl.when(sub < NSUB - 1)
    def _():
        pltpu.async_copy(acc_sh.at[pl.ds(sub * RA, RA)],
                         out_hbm.at[core, pl.ds(sub * RA, RA)], sem).wait()

    @pl.when(sub == NSUB - 1)
    def _():
        pltpu.async_copy(acc_sh.at[pl.ds((NSUB - 1) * RA, RATAIL)],
                         out_hbm.at[core, pl.ds((NSUB - 1) * RA, RATAIL)],
                         sem).wait()


def _sc_a_body(src_hbm, dst_hbm, t_hbm, ex_hbm, den_hbm,
               sidx0, didx0, sidx1, didx1, rs0, rd0, rs1, rd1, exs, bufex,
               acc_sh, sem0, sem1):
    core = lax.axis_index("c")
    sub = lax.axis_index("s")
    wid = core * NSUB + sub
    _zero_acc(rs0, acc_sh, sub)
    _zero_vbuf(exs, CHA)
    plsc.subcore_barrier()

    trips = (NCHUNKA - wid + NW - 1) // NW
    bufs = [(rs0, rd0, sidx0, didx0, sem0), (rs1, rd1, sidx1, didx1, sem1)]

    def fire_idx(b, kk):
        rs, rd, sidx, didx, sem = bufs[b]
        base = (wid + kk * NW) * CHA
        pltpu.async_copy(src_hbm.at[pl.ds(base, CHA)], sidx, sem)
        pltpu.async_copy(dst_hbm.at[pl.ds(base, CHA)], didx, sem)

    def wait_idx(b, kk):
        rs, rd, sidx, didx, sem = bufs[b]
        base = (wid + kk * NW) * CHA
        pltpu.make_async_copy(src_hbm.at[pl.ds(base, CHA)], sidx,
                              sem).wait()
        pltpu.make_async_copy(dst_hbm.at[pl.ds(base, CHA)], didx,
                              sem).wait()

    def issue(b):
        rs, rd, sidx, didx, sem = bufs[b]
        pltpu.async_copy(t_hbm.at[sidx], rs, sem)
        pltpu.async_copy(t_hbm.at[didx], rd, sem)

    fire_idx(0, 0)
    wait_idx(0, 0)
    issue(0)

    def step(b, kk, _):
        rs, rd, sidx, didx, sem = bufs[b]

        @pl.when(kk + 1 < trips)
        def _():
            fire_idx(1 - b, kk + 1)

        pltpu.make_async_copy(t_hbm.at[sidx], rs, sem).wait()
        pltpu.make_async_copy(t_hbm.at[didx], rd, sem).wait()

        def ew(i, _):
            for u in range(4):
                e = i * 4 + u
                t = rs[e, pl.ds(0, 16)] + rd[e, pl.ds(16, 16)]
                al = jnp.maximum(t, 0.2 * t)
                ex = jnp.exp(al - rd[e, pl.ds(32, 16)])
                exs[e, pl.ds(0, 16)] = ex
                bufex[e >> 3, pl.ds((e & 7) * 16, 16)] = ex
            return 0

        lax.fori_loop(0, CHA // 4, ew, 0)
        ci = wid + kk * NW
        dex = pltpu.async_copy(bufex, ex_hbm.at[pl.ds(ci * CHA8, CHA8)],
                               sem0)
        pltpu.sync_copy(exs, acc_sh.at[didx], add=True)
        dex.wait()

        @pl.when(kk + 1 < trips)
        def _():
            wait_idx(1 - b, kk + 1)
            issue(1 - b)

        return 0

    def loop(kk, _):
        @pl.when(kk % 2 == 0)
        def _():
            step(0, kk, 0)

        @pl.when(kk % 2 == 1)
        def _():
            step(1, kk, 0)

        return 0

    lax.fori_loop(0, trips, loop, 0)
    plsc.subcore_barrier()
    _drain_acc(acc_sh, den_hbm, core, sub, sem0)


_sc_a = pl.kernel(
    _sc_a_body,
    out_type=[
        jax.ShapeDtypeStruct((EX_ROWS, 128), jnp.float32),
        jax.ShapeDtypeStruct((NCORE, N, 128), jnp.float32),
    ],
    mesh=_mesh,
    compiler_params=_sc_params,
    scratch_types=[
        pltpu.VMEM((CHA,), jnp.int32),
        pltpu.VMEM((CHA,), jnp.int32),
        pltpu.VMEM((CHA,), jnp.int32),
        pltpu.VMEM((CHA,), jnp.int32),
        pltpu.VMEM((CHA, 128), jnp.float32),
        pltpu.VMEM((CHA, 128), jnp.float32),
        pltpu.VMEM((CHA, 128), jnp.float32),
        pltpu.VMEM((CHA, 128), jnp.float32),
        pltpu.VMEM((CHA, 128), jnp.float32),
        pltpu.VMEM((CHA8, 128), jnp.float32),
        pltpu.VMEM_SHARED((N, 128), jnp.float32),
        pltpu.SemaphoreType.DMA,
        pltpu.SemaphoreType.DMA,
    ],
)


def _make_sc_b(hpc):
    vph = 8 // hpc

    def body(src_hbm, dst_hbm, ha_hbm, hb_hbm, ex_hbm,
             out_hbm, sidx0, didx0, sidx1, didx1, rows0, rows1, exb0, exb1,
             acc_sh, sem0, sem1):
        core = lax.axis_index("c")
        sub = lax.axis_index("s")
        _zero_acc(rows0, acc_sh, sub)
        plsc.subcore_barrier()

        trips = (NCHUNK - sub + NSUB - 1) // NSUB
        zi = jnp.zeros((16,), jnp.int32)
        bufs = [(rows0, exb0, sidx0, didx0, sem0),
                (rows1, exb1, sidx1, didx1, sem1)]

        def fire_idx(b, kk):
            rows, exb, sidx, didx, sem = bufs[b]
            base = (sub + kk * NSUB) * CH
            pltpu.async_copy(src_hbm.at[pl.ds(base, CH)], sidx, sem)
            pltpu.async_copy(dst_hbm.at[pl.ds(base, CH)], didx, sem)

        def wait_idx(b, kk):
            rows, exb, sidx, didx, sem = bufs[b]
            base = (sub + kk * NSUB) * CH
            pltpu.make_async_copy(src_hbm.at[pl.ds(base, CH)], sidx,
                                  sem).wait()
            pltpu.make_async_copy(dst_hbm.at[pl.ds(base, CH)], didx,
                                  sem).wait()

        def issue(b, kk):
            rows, exb, sidx, didx, sem = bufs[b]

            @pl.when(core == 0)
            def _():
                pltpu.async_copy(ha_hbm.at[sidx], rows, sem)

            @pl.when(core == 1)
            def _():
                pltpu.async_copy(hb_hbm.at[sidx], rows, sem)

            ci = sub + kk * NSUB
            pltpu.async_copy(ex_hbm.at[pl.ds(ci * CH8, CH8)], exb, sem)

        fire_idx(0, 0)
        wait_idx(0, 0)
        issue(0, 0)

        def step(b, kk, _):
            rows, exb, sidx, didx, sem = bufs[b]

            @pl.when(kk + 1 < trips)
            def _():
                fire_idx(1 - b, kk + 1)

            pltpu.make_async_copy(ha_hbm.at[sidx], rows, sem).wait()
            ci = sub + kk * NSUB
            pltpu.make_async_copy(ex_hbm.at[pl.ds(ci * CH8, CH8)], exb,
                                  sem).wait()

            hoff = hpc * core if hpc > 1 else 0

            def ew(i, _):
                ws = []
                for u in range(4):
                    e = i * 4 + u
                    r8 = zi + (e >> 3)
                    l0 = zi + ((e & 7) * 16 + hoff)
                    ws.append([plsc.load_gather(exb, [r8, l0 + j])
                               for j in range(hpc)])
                for u in range(4):
                    e = i * 4 + u
                    for j in range(hpc):
                        for v in range(vph):
                            col = (j * vph + v) * 16
                            rows[e, pl.ds(col, 16)] = \
                                rows[e, pl.ds(col, 16)] * ws[u][j]
                return 0

            lax.fori_loop(0, CH // 4, ew, 0)
            pltpu.sync_copy(rows, acc_sh.at[didx], add=True)

            @pl.when(kk + 1 < trips)
            def _():
                wait_idx(1 - b, kk + 1)
                issue(1 - b, kk + 1)

            return 0

        def loop(kk, _):
            @pl.when(kk % 2 == 0)
            def _():
                step(0, kk, 0)

            @pl.when(kk % 2 == 1)
            def _():
                step(1, kk, 0)

            return 0

        lax.fori_loop(0, trips, loop, 0)
        plsc.subcore_barrier()
        _drain_acc(acc_sh, out_hbm, core, sub, sem0)

    return pl.kernel(
        body,
        out_type=jax.ShapeDtypeStruct((NCORE, N, 128), jnp.float32),
        mesh=_mesh,
        compiler_params=_sc_params,
        scratch_types=[
            pltpu.VMEM((CH,), jnp.int32),
            pltpu.VMEM((CH,), jnp.int32),
            pltpu.VMEM((CH,), jnp.int32),
            pltpu.VMEM((CH,), jnp.int32),
            pltpu.VMEM((CH, 128), jnp.float32),
            pltpu.VMEM((CH, 128), jnp.float32),
            pltpu.VMEM((CH8, 128), jnp.float32),
            pltpu.VMEM((CH8, 128), jnp.float32),
            pltpu.VMEM_SHARED((N, 128), jnp.float32),
            pltpu.SemaphoreType.DMA,
            pltpu.SemaphoreType.DMA,
        ],
    )


_sc_b1 = _make_sc_b(4)
_sc_b23 = _make_sc_b(1)


def _sc_a3_body(src_hbm, dst_hbm, q_hbm, k_hbm, ex_hbm, den_hbm,
                sidx0, didx0, sidx1, didx1, qb0, kb0, qb1, kb1, exs, bufex,
                alb, tmp, acc_sh, sem0, sem1):
    core = lax.axis_index("c")
    sub = lax.axis_index("s")
    wid = core * NSUB + sub
    _zero_acc(qb0, acc_sh, sub)
    _zero_vbuf(exs, CHA)
    plsc.subcore_barrier()

    trips = (NCHUNKA - wid + NW - 1) // NW
    lanes = lax.iota(jnp.int32, 16)
    bufs = [(qb0, kb0, sidx0, didx0, sem0), (qb1, kb1, sidx1, didx1, sem1)]

    def fire_idx(b, kk):
        qb, kb, sidx, didx, sem = bufs[b]
        base = (wid + kk * NW) * CHA
        pltpu.async_copy(src_hbm.at[pl.ds(base, CHA)], sidx, sem)
        pltpu.async_copy(dst_hbm.at[pl.ds(base, CHA)], didx, sem)

    def wait_idx(b, kk):
        qb, kb, sidx, didx, sem = bufs[b]
        base = (wid + kk * NW) * CHA
        pltpu.make_async_copy(src_hbm.at[pl.ds(base, CHA)], sidx,
                              sem).wait()
        pltpu.make_async_copy(dst_hbm.at[pl.ds(base, CHA)], didx,
                              sem).wait()

    def issue(b):
        qb, kb, sidx, didx, sem = bufs[b]
        pltpu.async_copy(q_hbm.at[didx], qb, sem)
        pltpu.async_copy(k_hbm.at[sidx], kb, sem)

    fire_idx(0, 0)
    wait_idx(0, 0)
    issue(0)

    def step(b, kk, _):
        qb, kb, sidx, didx, sem = bufs[b]

        @pl.when(kk + 1 < trips)
        def _():
            fire_idx(1 - b, kk + 1)

        pltpu.make_async_copy(q_hbm.at[didx], qb, sem).wait()
        pltpu.make_async_copy(k_hbm.at[sidx], kb, sem).wait()

        def dot_edge(i, _):
            for u in range(4):
                e = i * 4 + u
                pr = [qb[e, pl.ds(v * 16, 16)] * kb[e, pl.ds(v * 16, 16)]
                      for v in range(8)]
                acc = ((pr[0] + pr[1]) + (pr[2] + pr[3])) + \
                      ((pr[4] + pr[5]) + (pr[6] + pr[7]))
                tmp[e & 15, :] = acc
            return 0

        for g in range(CHA // 16):
            lax.fori_loop(g * 4, g * 4 + 4, dot_edge, 0)
            red = plsc.load_gather(tmp, [lanes, jnp.zeros((16,), jnp.int32)])
            for c in range(1, 16):
                red = red + plsc.load_gather(
                    tmp, [lanes, jnp.zeros((16,), jnp.int32) + c])
            alb[pl.ds(g * 16, 16)] = red

        for g in range(CHA // 16):
            ev = jnp.exp(alb[pl.ds(g * 16, 16)])
            plsc.store_scatter(exs, [g * 16 + lanes,
                                     jnp.zeros((16,), jnp.int32)], ev)
            plsc.store_scatter(bufex, [(g * 16 + lanes) >> 3,
                                       ((g * 16 + lanes) & 7) * 16], ev)

        ci = wid + kk * NW
        dex = pltpu.async_copy(bufex, ex_hbm.at[pl.ds(ci * CHA8, CHA8)],
                               sem0)
        pltpu.sync_copy(exs, acc_sh.at[didx], add=True)
        dex.wait()

        @pl.when(kk + 1 < trips)
        def _():
            wait_idx(1 - b, kk + 1)
            issue(1 - b)

        return 0

    def loop(kk, _):
        @pl.when(kk % 2 == 0)
        def _():
            step(0, kk, 0)

        @pl.when(kk % 2 == 1)
        def _():
            step(1, kk, 0)

        return 0

    lax.fori_loop(0, trips, loop, 0)
    plsc.subcore_barrier()
    _drain_acc(acc_sh, den_hbm, core, sub, sem0)


_sc_a3 = pl.kernel(
    _sc_a3_body,
    out_type=[
        jax.ShapeDtypeStruct((EX_ROWS, 128), jnp.float32),
        jax.ShapeDtypeStruct((NCORE, N, 128), jnp.float32),
    ],
    mesh=_mesh,
    compiler_params=_sc_params,
    scratch_types=[
        pltpu.VMEM((CHA,), jnp.int32),
        pltpu.VMEM((CHA,), jnp.int32),
        pltpu.VMEM((CHA,), jnp.int32),
        pltpu.VMEM((CHA,), jnp.int32),
        pltpu.VMEM((CHA, 128), jnp.float32),
        pltpu.VMEM((CHA, 128), jnp.float32),
        pltpu.VMEM((CHA, 128), jnp.float32),
        pltpu.VMEM((CHA, 128), jnp.float32),
        pltpu.VMEM((CHA, 128), jnp.float32),
        pltpu.VMEM((CHA8, 128), jnp.float32),
        pltpu.VMEM((CHA,), jnp.float32),
        pltpu.VMEM((16, 16), jnp.float32),
        pltpu.VMEM_SHARED((N, 128), jnp.float32),
        pltpu.SemaphoreType.DMA,
        pltpu.SemaphoreType.DMA,
    ],
)


def _leaky(v):
    return jnp.maximum(v, 0.2 * v)


def _den_slice(denp, h):
    return denp[0, :, 0:h] + denp[1, :, 0:h]


def kernel(x, edge_index, W1, att_src1, att_dst1, b1, W2, att_src2, att_dst2,
           b2, Wq, bq, Wk, bk, Wv, bv, Wskip, bskip, Wres, bres):
    src = edge_index[0]
    dst = edge_index[1]
    f32 = jnp.float32

    heads1, ch1 = att_src1.shape
    eye1 = jnp.eye(heads1, dtype=f32)
    blk_s = (att_src1[:, :, None] * eye1[:, None, :]).reshape(HID, heads1)
    blk_d = (att_dst1[:, :, None] * eye1[:, None, :]).reshape(HID, heads1)
    A1 = jnp.zeros((HID, 128), f32)
    A1 = A1.at[:, 0:8].set(blk_s).at[:, 16:24].set(blk_d)
    A2 = jnp.zeros((HID, 128), f32)
    A2 = A2.at[:, 0:1].set(att_src2.T).at[:, 16:17].set(att_dst2.T)

    scale = 1.0 / jnp.sqrt(jnp.float32(NC))
    Wcat = jnp.concatenate([Wq * scale, Wk, Wv, Wskip], axis=1)
    bcat = jnp.tile(jnp.concatenate([bq * scale, bk, bv, bskip])[None, :],
                    (8, 1))
    b1p = jnp.tile(b1[None, :], (8, 1))
    b2p = jnp.tile(b2[None, :], (8, 1))
    bresp = jnp.tile(bres[None, :], (8, 1))

    h1a, h1b, a1 = _k1(x, W1, A1)
    asrc1 = a1[:, 0:8]
    adst1 = a1[:, 16:24]
    gmax1 = jnp.max(asrc1, axis=0, keepdims=True)
    c1 = _leaky(adst1 + gmax1)
    exs1 = jnp.exp(_leaky(asrc1 + adst1) - c1)
    T1 = a1.at[:, 32:40].set(c1)

    ex1, den1p = _sc_a(src, dst, T1)
    den1 = _den_slice(den1p, 8) + exs1
    inv1 = 1.0 / (den1 + 1e-16)
    exsx1 = jnp.repeat(exs1, ch1, axis=1)
    invx1 = jnp.repeat(inv1, ch1, axis=1)

    scat1 = _sc_b1(src, dst, h1a, h1b, ex1)

    h2a, h2b, a2 = _k2(scat1[0], scat1[1], h1a, h1b, exsx1, invx1, b1p,
                       W2, A2)
    asrc2 = a2[:, 0:1]
    adst2 = a2[:, 16:17]
    gmax2 = jnp.max(asrc2, axis=0, keepdims=True)
    c2 = _leaky(adst2 + gmax2)
    exs2 = jnp.exp(_leaky(asrc2 + adst2) - c2)
    T2 = a2.at[:, 32:33].set(c2)

    ex2, den2p = _sc_a(src, dst, T2)
    den2 = _den_slice(den2p, 1) + exs2
    inv2 = 1.0 / (den2 + 1e-16)
    exsx2 = jnp.broadcast_to(exs2, (N, HID))
    invx2 = jnp.broadcast_to(inv2, (N, HID))

    scat2 = _sc_b23(src, dst, h2a, h2b, ex2)

    q3, k3, v3, skip3 = _k3(scat2[0], scat2[1], h2a, h2b, exsx2, invx2, b2p,
                            Wcat, bcat)

    ex3, den3p = _sc_a3(src, dst, q3, k3)
    den3 = _den_slice(den3p, 1)
    inv3 = 1.0 / (den3 + 1e-16)
    invx3 = jnp.broadcast_to(inv3, (N, 128))

    v3a = jnp.pad(v3[:, 0:64], ((0, 0), (0, 64)))
    v3b = jnp.pad(v3[:, 64:128], ((0, 0), (0, 64)))
    scat3 = _sc_b23(src, dst, v3a, v3b, ex3)

    return _k4(scat3[0], scat3[1], invx3, skip3, x, Wres, bresp)


# speedup vs baseline: 20.5731x; 1.0915x over previous
import jax
import jax.numpy as jnp
from jax import lax
from jax.experimental import pallas as pl
from jax.experimental.pallas import tpu as pltpu
from jax.experimental.pallas import tpu_sc as plsc

N = 10000
E = 160000
F_IN = 256
HID = 256
NC = 128

MBLK = 400
GRID = N // MBLK
CH = 128
CH8 = CH // 8
NCHUNK = E // CH
NSUB = 16
NCORE = 2
NW = NCORE * NSUB
RA = 640
RATAIL = N - (NSUB - 1) * RA
CHA = 64
CHA8 = CHA // 8
NCHUNKA = E // CHA
EX_ROWS = E // 8

_mesh = plsc.VectorSubcoreMesh(core_axis_name="c", subcore_axis_name="s")
_sc_params = pltpu.CompilerParams(needs_layout_passes=False)


def _k1_body(x_ref, w_ref, a_ref, ha_ref, hb_ref, a1_ref):
    h = jnp.dot(x_ref[...], w_ref[...], preferred_element_type=jnp.float32)
    ha_ref[...] = h[:, :128]
    hb_ref[...] = h[:, 128:]
    a1_ref[...] = jnp.dot(h, a_ref[...], preferred_element_type=jnp.float32)


def _k1(x, W1, A1):
    return pl.pallas_call(
        _k1_body,
        grid=(GRID,),
        in_specs=[
            pl.BlockSpec((MBLK, F_IN), lambda i: (i, 0)),
            pl.BlockSpec((F_IN, HID), lambda i: (0, 0)),
            pl.BlockSpec((HID, 128), lambda i: (0, 0)),
        ],
        out_specs=[
            pl.BlockSpec((MBLK, 128), lambda i: (i, 0)),
            pl.BlockSpec((MBLK, 128), lambda i: (i, 0)),
            pl.BlockSpec((MBLK, 128), lambda i: (i, 0)),
        ],
        out_shape=[
            jax.ShapeDtypeStruct((N, 128), jnp.float32),
            jax.ShapeDtypeStruct((N, 128), jnp.float32),
            jax.ShapeDtypeStruct((N, 128), jnp.float32),
        ],
    )(x, W1, A1)


def _elu(v):
    return jnp.where(v > 0, v, jnp.exp(v) - 1.0)


def _k2_body(sa_ref, sb_ref, ha_ref, hb_ref, exs_ref, inv_ref, b_ref,
             w_ref, a_ref, h2a_ref, h2b_ref, a2_ref):
    scat = jnp.concatenate([sa_ref[...], sb_ref[...]], axis=1)
    h = jnp.concatenate([ha_ref[...], hb_ref[...]], axis=1)
    x1 = _elu((scat + h * exs_ref[...]) * inv_ref[...] + b_ref[0:1, :])
    h2 = jnp.dot(x1, w_ref[...], preferred_element_type=jnp.float32)
    h2a_ref[...] = h2[:, :128]
    h2b_ref[...] = h2[:, 128:]
    a2_ref[...] = jnp.dot(h2, a_ref[...], preferred_element_type=jnp.float32)


def _k2(sa, sb, ha, hb, exsx, invx, bpad, W2, A2):
    return pl.pallas_call(
        _k2_body,
        grid=(GRID,),
        in_specs=[
            pl.BlockSpec((MBLK, 128), lambda i: (i, 0)),
            pl.BlockSpec((MBLK, 128), lambda i: (i, 0)),
            pl.BlockSpec((MBLK, 128), lambda i: (i, 0)),
            pl.BlockSpec((MBLK, 128), lambda i: (i, 0)),
            pl.BlockSpec((MBLK, HID), lambda i: (i, 0)),
            pl.BlockSpec((MBLK, HID), lambda i: (i, 0)),
            pl.BlockSpec((8, HID), lambda i: (0, 0)),
            pl.BlockSpec((HID, HID), lambda i: (0, 0)),
            pl.BlockSpec((HID, 128), lambda i: (0, 0)),
        ],
        out_specs=[
            pl.BlockSpec((MBLK, 128), lambda i: (i, 0)),
            pl.BlockSpec((MBLK, 128), lambda i: (i, 0)),
            pl.BlockSpec((MBLK, 128), lambda i: (i, 0)),
        ],
        out_shape=[
            jax.ShapeDtypeStruct((N, 128), jnp.float32),
            jax.ShapeDtypeStruct((N, 128), jnp.float32),
            jax.ShapeDtypeStruct((N, 128), jnp.float32),
        ],
    )(sa, sb, ha, hb, exsx, invx, bpad, W2, A2)


def _k3_body(sa_ref, sb_ref, ha_ref, hb_ref, exs_ref, inv_ref, b_ref,
             w_ref, bc_ref, q_ref, k_ref, v_ref, skip_ref):
    scat = jnp.concatenate([sa_ref[...], sb_ref[...]], axis=1)
    h = jnp.concatenate([ha_ref[...], hb_ref[...]], axis=1)
    x2 = _elu((scat + h * exs_ref[...]) * inv_ref[...] + b_ref[0:1, :])
    y = jnp.dot(x2, w_ref[...], preferred_element_type=jnp.float32)
    y = y + bc_ref[0:1, :]
    q_ref[...] = y[:, 0:128]
    k_ref[...] = y[:, 128:256]
    v_ref[...] = y[:, 256:384]
    skip_ref[...] = y[:, 384:512]


def _k3(sa, sb, ha, hb, exsx, invx, bpad, Wcat, bcat):
    return pl.pallas_call(
        _k3_body,
        grid=(GRID,),
        in_specs=[
            pl.BlockSpec((MBLK, 128), lambda i: (i, 0)),
            pl.BlockSpec((MBLK, 128), lambda i: (i, 0)),
            pl.BlockSpec((MBLK, 128), lambda i: (i, 0)),
            pl.BlockSpec((MBLK, 128), lambda i: (i, 0)),
            pl.BlockSpec((MBLK, HID), lambda i: (i, 0)),
            pl.BlockSpec((MBLK, HID), lambda i: (i, 0)),
            pl.BlockSpec((8, HID), lambda i: (0, 0)),
            pl.BlockSpec((HID, 512), lambda i: (0, 0)),
            pl.BlockSpec((8, 512), lambda i: (0, 0)),
        ],
        out_specs=[pl.BlockSpec((MBLK, 128), lambda i: (i, 0))] * 4,
        out_shape=[jax.ShapeDtypeStruct((N, 128), jnp.float32)] * 4,
    )(sa, sb, ha, hb, exsx, invx, bpad, Wcat, bcat)


def _k4_body(sa_ref, sb_ref, inv_ref, skip_ref, x_ref, w_ref, b_ref, o_ref):
    acc = jnp.dot(x_ref[...], w_ref[...], preferred_element_type=jnp.float32)
    x3 = jnp.concatenate([sa_ref[...][:, 0:64], sb_ref[...][:, 0:64]], axis=1)
    o_ref[...] = jnp.tanh(x3 * inv_ref[...]
                          + skip_ref[...] + acc + b_ref[0:1, :])


def _k4(sa, sb, invx, skip, x, Wres, bpad):
    return pl.pallas_call(
        _k4_body,
        grid=(GRID,),
        in_specs=[
            pl.BlockSpec((MBLK, 128), lambda i: (i, 0)),
            pl.BlockSpec((MBLK, 128), lambda i: (i, 0)),
            pl.BlockSpec((MBLK, 128), lambda i: (i, 0)),
            pl.BlockSpec((MBLK, 128), lambda i: (i, 0)),
            pl.BlockSpec((MBLK, F_IN), lambda i: (i, 0)),
            pl.BlockSpec((F_IN, 128), lambda i: (0, 0)),
            pl.BlockSpec((8, 128), lambda i: (0, 0)),
        ],
        out_specs=pl.BlockSpec((MBLK, 128), lambda i: (i, 0)),
        out_shape=jax.ShapeDtypeStruct((N, 128), jnp.float32),
    )(sa, sb, invx, skip, x, Wres, bpad)


def _zero_vbuf(vbuf, rows):
    z = jnp.zeros((16,), jnp.float32)
    nv = vbuf.shape[1] // 16

    def zr(r, _):
        for j in range(nv):
            vbuf[r, pl.ds(j * 16, 16)] = z
        return 0

    lax.fori_loop(0, rows, zr, 0)


def _zero_acc(zb, acc_sh, sub):
    zr = zb.shape[0]
    _zero_vbuf(zb, zr)

    @pl.when(sub < NSUB - 1)
    def _():
        for k in range(RA // zr):
            pltpu.sync_copy(zb, acc_sh.at[pl.ds(sub * RA + k * zr, zr)])

    @pl.when(sub == NSUB - 1)
    def _():
        for k in range(RATAIL // zr):
            pltpu.sync_copy(zb, acc_sh.at[pl.ds((NSUB - 1) * RA + k * zr,
                                                zr)])
        rem = RATAIL % zr
        if rem:
            pltpu.sync_copy(zb.at[pl.ds(0, rem)],
                            acc_sh.at[pl.ds(N - rem, rem)])


def _drain_acc(acc_sh, out_hbm, core, sub, sem):
    @pl.when(sub < NSUB - 1)
    def _():
        pltpu.async_copy(acc_sh.at[pl.ds(sub * RA, RA)],
                         out_hbm.at[core, pl.ds(sub * RA, RA)], sem).wait()

    @pl.when(sub == NSUB - 1)
    def _():
        pltpu.async_copy(acc_sh.at[pl.ds((NSUB - 1) * RA, RATAIL)],
                         out_hbm.at[core, pl.ds((NSUB - 1) * RA, RATAIL)],
                         sem).wait()


def _sc_a_body(src_hbm, dst_hbm, t_hbm, ex_hbm, den_hbm,
               sidx0, didx0, sidx1, didx1, rs0, rd0, rs1, rd1, exs, bufex,
               acc_sh, sem0, sem1):
    core = lax.axis_index("c")
    sub = lax.axis_index("s")
    wid = core * NSUB + sub
    _zero_acc(rs0, acc_sh, sub)
    _zero_vbuf(exs, CHA)
    plsc.subcore_barrier()

    trips = (NCHUNKA - wid + NW - 1) // NW
    bufs = [(rs0, rd0, sidx0, didx0, sem0), (rs1, rd1, sidx1, didx1, sem1)]

    def fire_idx(b, kk):
        rs, rd, sidx, didx, sem = bufs[b]
        base = (wid + kk * NW) * CHA
        pltpu.async_copy(src_hbm.at[pl.ds(base, CHA)], sidx, sem)
        pltpu.async_copy(dst_hbm.at[pl.ds(base, CHA)], didx, sem)

    def wait_idx(b, kk):
        rs, rd, sidx, didx, sem = bufs[b]
        base = (wid + kk * NW) * CHA
        pltpu.make_async_copy(src_hbm.at[pl.ds(base, CHA)], sidx,
                              sem).wait()
        pltpu.make_async_copy(dst_hbm.at[pl.ds(base, CHA)], didx,
                              sem).wait()

    def issue(b):
        rs, rd, sidx, didx, sem = bufs[b]
        pltpu.async_copy(t_hbm.at[sidx], rs, sem)
        pltpu.async_copy(t_hbm.at[didx], rd, sem)

    fire_idx(0, 0)
    wait_idx(0, 0)
    issue(0)

    def step(b, kk, _):
        rs, rd, sidx, didx, sem = bufs[b]

        @pl.when(kk + 1 < trips)
        def _():
            fire_idx(1 - b, kk + 1)

        pltpu.make_async_copy(t_hbm.at[sidx], rs, sem).wait()
        pltpu.make_async_copy(t_hbm.at[didx], rd, sem).wait()

        def ew(i, _):
            for u in range(4):
                e = i * 4 + u
                t = rs[e, pl.ds(0, 16)] + rd[e, pl.ds(16, 16)]
                al = jnp.maximum(t, 0.2 * t)
                ex = jnp.exp(al - rd[e, pl.ds(32, 16)])
                exs[e, pl.ds(0, 16)] = ex
                bufex[e >> 3, pl.ds((e & 7) * 16, 16)] = ex
            return 0

        lax.fori_loop(0, CHA // 4, ew, 0)
        ci = wid + kk * NW
        dex = pltpu.async_copy(bufex, ex_hbm.at[pl.ds(ci * CHA8, CHA8)],
                               sem0)
        pltpu.sync_copy(exs, acc_sh.at[didx], add=True)
        dex.wait()

        @pl.when(kk + 1 < trips)
        def _():
            wait_idx(1 - b, kk + 1)
            issue(1 - b)

        return 0

    def loop(kk, _):
        @pl.when(kk % 2 == 0)
        def _():
            step(0, kk, 0)

        @pl.when(kk % 2 == 1)
        def _():
            step(1, kk, 0)

        return 0

    lax.fori_loop(0, trips, loop, 0)
    plsc.subcore_barrier()
    _drain_acc(acc_sh, den_hbm, core, sub, sem0)


_sc_a = pl.kernel(
    _sc_a_body,
    out_type=[
        jax.ShapeDtypeStruct((EX_ROWS, 128), jnp.float32),
        jax.ShapeDtypeStruct((NCORE, N, 128), jnp.float32),
    ],
    mesh=_mesh,
    compiler_params=_sc_params,
    scratch_types=[
        pltpu.VMEM((CHA,), jnp.int32),
        pltpu.VMEM((CHA,), jnp.int32),
        pltpu.VMEM((CHA,), jnp.int32),
        pltpu.VMEM((CHA,), jnp.int32),
        pltpu.VMEM((CHA, 128), jnp.float32),
        pltpu.VMEM((CHA, 128), jnp.float32),
        pltpu.VMEM((CHA, 128), jnp.float32),
        pltpu.VMEM((CHA, 128), jnp.float32),
        pltpu.VMEM((CHA, 128), jnp.float32),
        pltpu.VMEM((CHA8, 128), jnp.float32),
        pltpu.VMEM_SHARED((N, 128), jnp.float32),
        pltpu.SemaphoreType.DMA,
        pltpu.SemaphoreType.DMA,
    ],
)


def _make_sc_b(hpc):
    vph = 8 // hpc

    def body(src_hbm, dst_hbm, ha_hbm, hb_hbm, ex_hbm,
             out_hbm, sidx0, didx0, sidx1, didx1, rows0, rows1, exb0, exb1,
             acc_sh, sem0, sem1):
        core = lax.axis_index("c")
        sub = lax.axis_index("s")
        _zero_acc(rows0, acc_sh, sub)
        plsc.subcore_barrier()

        trips = (NCHUNK - sub + NSUB - 1) // NSUB
        zi = jnp.zeros((16,), jnp.int32)
        bufs = [(rows0, exb0, sidx0, didx0, sem0),
                (rows1, exb1, sidx1, didx1, sem1)]

        def fire_idx(b, kk):
            rows, exb, sidx, didx, sem = bufs[b]
            base = (sub + kk * NSUB) * CH
            pltpu.async_copy(src_hbm.at[pl.ds(base, CH)], sidx, sem)
            pltpu.async_copy(dst_hbm.at[pl.ds(base, CH)], didx, sem)

        def wait_idx(b, kk):
            rows, exb, sidx, didx, sem = bufs[b]
            base = (sub + kk * NSUB) * CH
            pltpu.make_async_copy(src_hbm.at[pl.ds(base, CH)], sidx,
                                  sem).wait()
            pltpu.make_async_copy(dst_hbm.at[pl.ds(base, CH)], didx,
                                  sem).wait()

        def issue(b, kk):
            rows, exb, sidx, didx, sem = bufs[b]

            @pl.when(core == 0)
            def _():
                pltpu.async_copy(ha_hbm.at[sidx], rows, sem)

            @pl.when(core == 1)
            def _():
                pltpu.async_copy(hb_hbm.at[sidx], rows, sem)

            ci = sub + kk * NSUB
            pltpu.async_copy(ex_hbm.at[pl.ds(ci * CH8, CH8)], exb, sem)

        fire_idx(0, 0)
        wait_idx(0, 0)
        issue(0, 0)

        def step(b, kk, _):
            rows, exb, sidx, didx, sem = bufs[b]
            rown, exbn, sidxn, didxn, semn = bufs[1 - b]

            @pl.when(kk >= 1)
            def _():
                pltpu.make_async_copy(rown, acc_sh.at[didxn], semn).wait()

            @pl.when(kk + 1 < trips)
            def _():
                fire_idx(1 - b, kk + 1)

            pltpu.make_async_copy(ha_hbm.at[sidx], rows, sem).wait()
            ci = sub + kk * NSUB
            pltpu.make_async_copy(ex_hbm.at[pl.ds(ci * CH8, CH8)], exb,
                                  sem).wait()

            hoff = hpc * core if hpc > 1 else 0

            def ew(i, _):
                ws = []
                for u in range(4):
                    e = i * 4 + u
                    r8 = zi + (e >> 3)
                    l0 = zi + ((e & 7) * 16 + hoff)
                    ws.append([plsc.load_gather(exb, [r8, l0 + j])
                               for j in range(hpc)])
                for u in range(4):
                    e = i * 4 + u
                    for j in range(hpc):
                        for v in range(vph):
                            col = (j * vph + v) * 16
                            rows[e, pl.ds(col, 16)] = \
                                rows[e, pl.ds(col, 16)] * ws[u][j]
                return 0

            lax.fori_loop(0, CH // 4, ew, 0)
            pltpu.async_copy(rows, acc_sh.at[didx], sem, add=True)

            @pl.when(kk + 1 < trips)
            def _():
                wait_idx(1 - b, kk + 1)
                issue(1 - b, kk + 1)

            return 0

        def loop(kk, _):
            @pl.when(kk % 2 == 0)
            def _():
                step(0, kk, 0)

            @pl.when(kk % 2 == 1)
            def _():
                step(1, kk, 0)

            return 0

        lax.fori_loop(0, trips, loop, 0)

        @pl.when((trips - 1) % 2 == 0)
        def _():
            pltpu.make_async_copy(rows0, acc_sh.at[didx0], sem0).wait()

        @pl.when((trips - 1) % 2 == 1)
        def _():
            pltpu.make_async_copy(rows1, acc_sh.at[didx1], sem1).wait()

        plsc.subcore_barrier()
        _drain_acc(acc_sh, out_hbm, core, sub, sem0)

    return pl.kernel(
        body,
        out_type=jax.ShapeDtypeStruct((NCORE, N, 128), jnp.float32),
        mesh=_mesh,
        compiler_params=_sc_params,
        scratch_types=[
            pltpu.VMEM((CH,), jnp.int32),
            pltpu.VMEM((CH,), jnp.int32),
            pltpu.VMEM((CH,), jnp.int32),
            pltpu.VMEM((CH,), jnp.int32),
            pltpu.VMEM((CH, 128), jnp.float32),
            pltpu.VMEM((CH, 128), jnp.float32),
            pltpu.VMEM((CH8, 128), jnp.float32),
            pltpu.VMEM((CH8, 128), jnp.float32),
            pltpu.VMEM_SHARED((N, 128), jnp.float32),
            pltpu.SemaphoreType.DMA,
            pltpu.SemaphoreType.DMA,
        ],
    )


_sc_b1 = _make_sc_b(4)
_sc_b23 = _make_sc_b(1)


def _sc_a3_body(src_hbm, dst_hbm, q_hbm, k_hbm, ex_hbm, den_hbm,
                sidx0, didx0, sidx1, didx1, qb0, kb0, qb1, kb1, exs, bufex,
                alb, tmp, acc_sh, sem0, sem1):
    core = lax.axis_index("c")
    sub = lax.axis_index("s")
    wid = core * NSUB + sub
    _zero_acc(qb0, acc_sh, sub)
    _zero_vbuf(exs, CHA)
    plsc.subcore_barrier()

    trips = (NCHUNKA - wid + NW - 1) // NW
    lanes = lax.iota(jnp.int32, 16)
    bufs = [(qb0, kb0, sidx0, didx0, sem0), (qb1, kb1, sidx1, didx1, sem1)]

    def fire_idx(b, kk):
        qb, kb, sidx, didx, sem = bufs[b]
        base = (wid + kk * NW) * CHA
        pltpu.async_copy(src_hbm.at[pl.ds(base, CHA)], sidx, sem)
        pltpu.async_copy(dst_hbm.at[pl.ds(base, CHA)], didx, sem)

    def wait_idx(b, kk):
        qb, kb, sidx, didx, sem = bufs[b]
        base = (wid + kk * NW) * CHA
        pltpu.make_async_copy(src_hbm.at[pl.ds(base, CHA)], sidx,
                              sem).wait()
        pltpu.make_async_copy(dst_hbm.at[pl.ds(base, CHA)], didx,
                              sem).wait()

    def issue(b):
        qb, kb, sidx, didx, sem = bufs[b]
        pltpu.async_copy(q_hbm.at[didx], qb, sem)
        pltpu.async_copy(k_hbm.at[sidx], kb, sem)

    fire_idx(0, 0)
    wait_idx(0, 0)
    issue(0)

    def step(b, kk, _):
        qb, kb, sidx, didx, sem = bufs[b]

        @pl.when(kk + 1 < trips)
        def _():
            fire_idx(1 - b, kk + 1)

        pltpu.make_async_copy(q_hbm.at[didx], qb, sem).wait()
        pltpu.make_async_copy(k_hbm.at[sidx], kb, sem).wait()

        def dot_edge(i, _):
            for u in range(4):
                e = i * 4 + u
                pr = [qb[e, pl.ds(v * 16, 16)] * kb[e, pl.ds(v * 16, 16)]
                      for v in range(8)]
                acc = ((pr[0] + pr[1]) + (pr[2] + pr[3])) + \
                      ((pr[4] + pr[5]) + (pr[6] + pr[7]))
                tmp[e & 15, :] = acc
            return 0

        for g in range(CHA // 16):
            lax.fori_loop(g * 4, g * 4 + 4, dot_edge, 0)
            red = plsc.load_gather(tmp, [lanes, jnp.zeros((16,), jnp.int32)])
            for c in range(1, 16):
                red = red + plsc.load_gather(
                    tmp, [lanes, jnp.zeros((16,), jnp.int32) + c])
            alb[pl.ds(g * 16, 16)] = red

        for g in range(CHA // 16):
            ev = jnp.exp(alb[pl.ds(g * 16, 16)])
            plsc.store_scatter(exs, [g * 16 + lanes,
                                     jnp.zeros((16,), jnp.int32)], ev)
            plsc.store_scatter(bufex, [(g * 16 + lanes) >> 3,
                                       ((g * 16 + lanes) & 7) * 16], ev)

        ci = wid + kk * NW
        dex = pltpu.async_copy(bufex, ex_hbm.at[pl.ds(ci * CHA8, CHA8)],
                               sem0)
        pltpu.sync_copy(exs, acc_sh.at[didx], add=True)
        dex.wait()

        @pl.when(kk + 1 < trips)
        def _():
            wait_idx(1 - b, kk + 1)
            issue(1 - b)

        return 0

    def loop(kk, _):
        @pl.when(kk % 2 == 0)
        def _():
            step(0, kk, 0)

        @pl.when(kk % 2 == 1)
        def _():
            step(1, kk, 0)

        return 0

    lax.fori_loop(0, trips, loop, 0)
    plsc.subcore_barrier()
    _drain_acc(acc_sh, den_hbm, core, sub, sem0)


_sc_a3 = pl.kernel(
    _sc_a3_body,
    out_type=[
        jax.ShapeDtypeStruct((EX_ROWS, 128), jnp.float32),
        jax.ShapeDtypeStruct((NCORE, N, 128), jnp.float32),
    ],
    mesh=_mesh,
    compiler_params=_sc_params,
    scratch_types=[
        pltpu.VMEM((CHA,), jnp.int32),
        pltpu.VMEM((CHA,), jnp.int32),
        pltpu.VMEM((CHA,), jnp.int32),
        pltpu.VMEM((CHA,), jnp.int32),
        pltpu.VMEM((CHA, 128), jnp.float32),
        pltpu.VMEM((CHA, 128), jnp.float32),
        pltpu.VMEM((CHA, 128), jnp.float32),
        pltpu.VMEM((CHA, 128), jnp.float32),
        pltpu.VMEM((CHA, 128), jnp.float32),
        pltpu.VMEM((CHA8, 128), jnp.float32),
        pltpu.VMEM((CHA,), jnp.float32),
        pltpu.VMEM((16, 16), jnp.float32),
        pltpu.VMEM_SHARED((N, 128), jnp.float32),
        pltpu.SemaphoreType.DMA,
        pltpu.SemaphoreType.DMA,
    ],
)


def _leaky(v):
    return jnp.maximum(v, 0.2 * v)


def _den_slice(denp, h):
    return denp[0, :, 0:h] + denp[1, :, 0:h]


def kernel(x, edge_index, W1, att_src1, att_dst1, b1, W2, att_src2, att_dst2,
           b2, Wq, bq, Wk, bk, Wv, bv, Wskip, bskip, Wres, bres):
    src = edge_index[0]
    dst = edge_index[1]
    f32 = jnp.float32

    heads1, ch1 = att_src1.shape
    eye1 = jnp.eye(heads1, dtype=f32)
    blk_s = (att_src1[:, :, None] * eye1[:, None, :]).reshape(HID, heads1)
    blk_d = (att_dst1[:, :, None] * eye1[:, None, :]).reshape(HID, heads1)
    A1 = jnp.zeros((HID, 128), f32)
    A1 = A1.at[:, 0:8].set(blk_s).at[:, 16:24].set(blk_d)
    A2 = jnp.zeros((HID, 128), f32)
    A2 = A2.at[:, 0:1].set(att_src2.T).at[:, 16:17].set(att_dst2.T)

    scale = 1.0 / jnp.sqrt(jnp.float32(NC))
    Wcat = jnp.concatenate([Wq * scale, Wk, Wv, Wskip], axis=1)
    bcat = jnp.tile(jnp.concatenate([bq * scale, bk, bv, bskip])[None, :],
                    (8, 1))
    b1p = jnp.tile(b1[None, :], (8, 1))
    b2p = jnp.tile(b2[None, :], (8, 1))
    bresp = jnp.tile(bres[None, :], (8, 1))

    h1a, h1b, a1 = _k1(x, W1, A1)
    asrc1 = a1[:, 0:8]
    adst1 = a1[:, 16:24]
    gmax1 = jnp.max(asrc1, axis=0, keepdims=True)
    c1 = _leaky(adst1 + gmax1)
    exs1 = jnp.exp(_leaky(asrc1 + adst1) - c1)
    T1 = a1.at[:, 32:40].set(c1)

    ex1, den1p = _sc_a(src, dst, T1)
    den1 = _den_slice(den1p, 8) + exs1
    inv1 = 1.0 / (den1 + 1e-16)
    exsx1 = jnp.repeat(exs1, ch1, axis=1)
    invx1 = jnp.repeat(inv1, ch1, axis=1)

    scat1 = _sc_b1(src, dst, h1a, h1b, ex1)

    h2a, h2b, a2 = _k2(scat1[0], scat1[1], h1a, h1b, exsx1, invx1, b1p,
                       W2, A2)
    asrc2 = a2[:, 0:1]
    adst2 = a2[:, 16:17]
    gmax2 = jnp.max(asrc2, axis=0, keepdims=True)
    c2 = _leaky(adst2 + gmax2)
    exs2 = jnp.exp(_leaky(asrc2 + adst2) - c2)
    T2 = a2.at[:, 32:33].set(c2)

    ex2, den2p = _sc_a(src, dst, T2)
    den2 = _den_slice(den2p, 1) + exs2
    inv2 = 1.0 / (den2 + 1e-16)
    exsx2 = jnp.broadcast_to(exs2, (N, HID))
    invx2 = jnp.broadcast_to(inv2, (N, HID))

    scat2 = _sc_b23(src, dst, h2a, h2b, ex2)

    q3, k3, v3, skip3 = _k3(scat2[0], scat2[1], h2a, h2b, exsx2, invx2, b2p,
                            Wcat, bcat)

    ex3, den3p = _sc_a3(src, dst, q3, k3)
    den3 = _den_slice(den3p, 1)
    inv3 = 1.0 / (den3 + 1e-16)
    invx3 = jnp.broadcast_to(inv3, (N, 128))

    v3a = jnp.pad(v3[:, 0:64], ((0, 0), (0, 64)))
    v3b = jnp.pad(v3[:, 64:128], ((0, 0), (0, 64)))
    scat3 = _sc_b23(src, dst, v3a, v3b, ex3)

    return _k4(scat3[0], scat3[1], invx3, skip3, x, Wres, bresp)


# speedup vs baseline: 20.6390x; 1.0032x over previous
import jax
import jax.numpy as jnp
from jax import lax
from jax.experimental import pallas as pl
from jax.experimental.pallas import tpu as pltpu
from jax.experimental.pallas import tpu_sc as plsc

N = 10000
E = 160000
F_IN = 256
HID = 256
NC = 128

MBLK = 400
GRID = N // MBLK
CH = 128
CH8 = CH // 8
NCHUNK = E // CH
NSUB = 16
NCORE = 2
NW = NCORE * NSUB
RA = 640
RATAIL = N - (NSUB - 1) * RA
CHA = 64
CHA8 = CHA // 8
NCHUNKA = E // CHA
EX_ROWS = E // 8

_mesh = plsc.VectorSubcoreMesh(core_axis_name="c", subcore_axis_name="s")
_sc_params = pltpu.CompilerParams(needs_layout_passes=False)


def _k1_body(x_ref, w_ref, a_ref, ha_ref, hb_ref, a1_ref):
    h = jnp.dot(x_ref[...], w_ref[...], preferred_element_type=jnp.float32)
    ha_ref[...] = h[:, :128]
    hb_ref[...] = h[:, 128:]
    a1_ref[...] = jnp.dot(h, a_ref[...], preferred_element_type=jnp.float32)


def _k1(x, W1, A1):
    return pl.pallas_call(
        _k1_body,
        grid=(GRID,),
        in_specs=[
            pl.BlockSpec((MBLK, F_IN), lambda i: (i, 0)),
            pl.BlockSpec((F_IN, HID), lambda i: (0, 0)),
            pl.BlockSpec((HID, 128), lambda i: (0, 0)),
        ],
        out_specs=[
            pl.BlockSpec((MBLK, 128), lambda i: (i, 0)),
            pl.BlockSpec((MBLK, 128), lambda i: (i, 0)),
            pl.BlockSpec((MBLK, 128), lambda i: (i, 0)),
        ],
        out_shape=[
            jax.ShapeDtypeStruct((N, 128), jnp.float32),
            jax.ShapeDtypeStruct((N, 128), jnp.float32),
            jax.ShapeDtypeStruct((N, 128), jnp.float32),
        ],
    )(x, W1, A1)


def _elu(v):
    return jnp.where(v > 0, v, jnp.exp(v) - 1.0)


def _k2_body(sa_ref, sb_ref, ha_ref, hb_ref, exs_ref, inv_ref, b_ref,
             w_ref, a_ref, h2a_ref, h2b_ref, a2_ref):
    scat = jnp.concatenate([sa_ref[...], sb_ref[...]], axis=1)
    h = jnp.concatenate([ha_ref[...], hb_ref[...]], axis=1)
    x1 = _elu((scat + h * exs_ref[...]) * inv_ref[...] + b_ref[0:1, :])
    h2 = jnp.dot(x1, w_ref[...], preferred_element_type=jnp.float32)
    h2a_ref[...] = h2[:, :128]
    h2b_ref[...] = h2[:, 128:]
    a2_ref[...] = jnp.dot(h2, a_ref[...], preferred_element_type=jnp.float32)


def _k2(sa, sb, ha, hb, exsx, invx, bpad, W2, A2):
    return pl.pallas_call(
        _k2_body,
        grid=(GRID,),
        in_specs=[
            pl.BlockSpec((MBLK, 128), lambda i: (i, 0)),
            pl.BlockSpec((MBLK, 128), lambda i: (i, 0)),
            pl.BlockSpec((MBLK, 128), lambda i: (i, 0)),
            pl.BlockSpec((MBLK, 128), lambda i: (i, 0)),
            pl.BlockSpec((MBLK, HID), lambda i: (i, 0)),
            pl.BlockSpec((MBLK, HID), lambda i: (i, 0)),
            pl.BlockSpec((8, HID), lambda i: (0, 0)),
            pl.BlockSpec((HID, HID), lambda i: (0, 0)),
            pl.BlockSpec((HID, 128), lambda i: (0, 0)),
        ],
        out_specs=[
            pl.BlockSpec((MBLK, 128), lambda i: (i, 0)),
            pl.BlockSpec((MBLK, 128), lambda i: (i, 0)),
            pl.BlockSpec((MBLK, 128), lambda i: (i, 0)),
        ],
        out_shape=[
            jax.ShapeDtypeStruct((N, 128), jnp.float32),
            jax.ShapeDtypeStruct((N, 128), jnp.float32),
            jax.ShapeDtypeStruct((N, 128), jnp.float32),
        ],
    )(sa, sb, ha, hb, exsx, invx, bpad, W2, A2)


def _k3_body(sa_ref, sb_ref, ha_ref, hb_ref, exs_ref, inv_ref, b_ref,
             w_ref, bc_ref, q_ref, k_ref, v_ref, skip_ref):
    scat = jnp.concatenate([sa_ref[...], sb_ref[...]], axis=1)
    h = jnp.concatenate([ha_ref[...], hb_ref[...]], axis=1)
    x2 = _elu((scat + h * exs_ref[...]) * inv_ref[...] + b_ref[0:1, :])
    y = jnp.dot(x2, w_ref[...], preferred_element_type=jnp.float32)
    y = y + bc_ref[0:1, :]
    q_ref[...] = y[:, 0:128]
    k_ref[...] = y[:, 128:256]
    v_ref[...] = y[:, 256:384]
    skip_ref[...] = y[:, 384:512]


def _k3(sa, sb, ha, hb, exsx, invx, bpad, Wcat, bcat):
    return pl.pallas_call(
        _k3_body,
        grid=(GRID,),
        in_specs=[
            pl.BlockSpec((MBLK, 128), lambda i: (i, 0)),
            pl.BlockSpec((MBLK, 128), lambda i: (i, 0)),
            pl.BlockSpec((MBLK, 128), lambda i: (i, 0)),
            pl.BlockSpec((MBLK, 128), lambda i: (i, 0)),
            pl.BlockSpec((MBLK, HID), lambda i: (i, 0)),
            pl.BlockSpec((MBLK, HID), lambda i: (i, 0)),
            pl.BlockSpec((8, HID), lambda i: (0, 0)),
            pl.BlockSpec((HID, 512), lambda i: (0, 0)),
            pl.BlockSpec((8, 512), lambda i: (0, 0)),
        ],
        out_specs=[pl.BlockSpec((MBLK, 128), lambda i: (i, 0))] * 4,
        out_shape=[jax.ShapeDtypeStruct((N, 128), jnp.float32)] * 4,
    )(sa, sb, ha, hb, exsx, invx, bpad, Wcat, bcat)


def _k4_body(sa_ref, sb_ref, inv_ref, skip_ref, x_ref, w_ref, b_ref, o_ref):
    acc = jnp.dot(x_ref[...], w_ref[...], preferred_element_type=jnp.float32)
    x3 = jnp.concatenate([sa_ref[...][:, 0:64], sb_ref[...][:, 0:64]], axis=1)
    o_ref[...] = jnp.tanh(x3 * inv_ref[...]
                          + skip_ref[...] + acc + b_ref[0:1, :])


def _k4(sa, sb, invx, skip, x, Wres, bpad):
    return pl.pallas_call(
        _k4_body,
        grid=(GRID,),
        in_specs=[
            pl.BlockSpec((MBLK, 128), lambda i: (i, 0)),
            pl.BlockSpec((MBLK, 128), lambda i: (i, 0)),
            pl.BlockSpec((MBLK, 128), lambda i: (i, 0)),
            pl.BlockSpec((MBLK, 128), lambda i: (i, 0)),
            pl.BlockSpec((MBLK, F_IN), lambda i: (i, 0)),
            pl.BlockSpec((F_IN, 128), lambda i: (0, 0)),
            pl.BlockSpec((8, 128), lambda i: (0, 0)),
        ],
        out_specs=pl.BlockSpec((MBLK, 128), lambda i: (i, 0)),
        out_shape=jax.ShapeDtypeStruct((N, 128), jnp.float32),
    )(sa, sb, invx, skip, x, Wres, bpad)


def _zero_vbuf(vbuf, rows):
    z = jnp.zeros((16,), jnp.float32)
    nv = vbuf.shape[1] // 16

    def zr(r, _):
        for j in range(nv):
            vbuf[r, pl.ds(j * 16, 16)] = z
        return 0

    lax.fori_loop(0, rows, zr, 0)


def _zero_acc(zb, acc_sh, sub):
    zr = zb.shape[0]
    _zero_vbuf(zb, zr)

    @pl.when(sub < NSUB - 1)
    def _():
        for k in range(RA // zr):
            pltpu.sync_copy(zb, acc_sh.at[pl.ds(sub * RA + k * zr, zr)])

    @pl.when(sub == NSUB - 1)
    def _():
        for k in range(RATAIL // zr):
            pltpu.sync_copy(zb, acc_sh.at[pl.ds((NSUB - 1) * RA + k * zr,
                                                zr)])
        rem = RATAIL % zr
        if rem:
            pltpu.sync_copy(zb.at[pl.ds(0, rem)],
                            acc_sh.at[pl.ds(N - rem, rem)])


def _drain_acc(acc_sh, out_hbm, core, sub, sem):
    @pl.when(sub < NSUB - 1)
    def _():
        pltpu.async_copy(acc_sh.at[pl.ds(sub * RA, RA)],
                         out_hbm.at[core, pl.ds(sub * RA, RA)], sem).wait()

    @pl.when(sub == NSUB - 1)
    def _():
        pltpu.async_copy(acc_sh.at[pl.ds((NSUB - 1) * RA, RATAIL)],
                         out_hbm.at[core, pl.ds((NSUB - 1) * RA, RATAIL)],
                         sem).wait()


def _sc_a_body(src_hbm, dst_hbm, t_hbm, ex_hbm, den_hbm,
               sidx0, didx0, sidx1, didx1, rs0, rd0, rs1, rd1, exs, bufex,
               acc_sh, sem0, sem1, sem2):
    core = lax.axis_index("c")
    sub = lax.axis_index("s")
    wid = core * NSUB + sub
    _zero_acc(rs0, acc_sh, sub)
    _zero_vbuf(exs, CHA)
    plsc.subcore_barrier()

    trips = (NCHUNKA - wid + NW - 1) // NW
    bufs = [(rs0, rd0, sidx0, didx0, sem0), (rs1, rd1, sidx1, didx1, sem1)]

    def fire_idx(b, kk):
        rs, rd, sidx, didx, sem = bufs[b]
        base = (wid + kk * NW) * CHA
        pltpu.async_copy(src_hbm.at[pl.ds(base, CHA)], sidx, sem)
        pltpu.async_copy(dst_hbm.at[pl.ds(base, CHA)], didx, sem)

    def wait_idx(b, kk):
        rs, rd, sidx, didx, sem = bufs[b]
        base = (wid + kk * NW) * CHA
        pltpu.make_async_copy(src_hbm.at[pl.ds(base, CHA)], sidx,
                              sem).wait()
        pltpu.make_async_copy(dst_hbm.at[pl.ds(base, CHA)], didx,
                              sem).wait()

    def issue(b):
        rs, rd, sidx, didx, sem = bufs[b]
        pltpu.async_copy(t_hbm.at[sidx], rs, sem)
        pltpu.async_copy(t_hbm.at[didx], rd, sem)

    fire_idx(0, 0)
    wait_idx(0, 0)
    issue(0)

    def step(b, kk, _):
        rs, rd, sidx, didx, sem = bufs[b]

        @pl.when(kk + 1 < trips)
        def _():
            fire_idx(1 - b, kk + 1)

        pltpu.make_async_copy(t_hbm.at[sidx], rs, sem).wait()
        pltpu.make_async_copy(t_hbm.at[didx], rd, sem).wait()

        def ew(i, _):
            for u in range(4):
                e = i * 4 + u
                t = rs[e, pl.ds(0, 16)] + rd[e, pl.ds(16, 16)]
                al = jnp.maximum(t, 0.2 * t)
                ex = jnp.exp(al - rd[e, pl.ds(32, 16)])
                exs[e, pl.ds(0, 16)] = ex
                bufex[e >> 3, pl.ds((e & 7) * 16, 16)] = ex
            return 0

        lax.fori_loop(0, CHA // 4, ew, 0)
        ci = wid + kk * NW
        dex = pltpu.async_copy(bufex, ex_hbm.at[pl.ds(ci * CHA8, CHA8)],
                               sem2)
        pltpu.sync_copy(exs, acc_sh.at[didx], add=True)
        dex.wait()

        @pl.when(kk + 1 < trips)
        def _():
            wait_idx(1 - b, kk + 1)
            issue(1 - b)

        return 0

    def loop(kk, _):
        @pl.when(kk % 2 == 0)
        def _():
            step(0, kk, 0)

        @pl.when(kk % 2 == 1)
        def _():
            step(1, kk, 0)

        return 0

    lax.fori_loop(0, trips, loop, 0)
    plsc.subcore_barrier()
    _drain_acc(acc_sh, den_hbm, core, sub, sem0)


_sc_a = pl.kernel(
    _sc_a_body,
    out_type=[
        jax.ShapeDtypeStruct((EX_ROWS, 128), jnp.float32),
        jax.ShapeDtypeStruct((NCORE, N, 128), jnp.float32),
    ],
    mesh=_mesh,
    compiler_params=_sc_params,
    scratch_types=[
        pltpu.VMEM((CHA,), jnp.int32),
        pltpu.VMEM((CHA,), jnp.int32),
        pltpu.VMEM((CHA,), jnp.int32),
        pltpu.VMEM((CHA,), jnp.int32),
        pltpu.VMEM((CHA, 128), jnp.float32),
        pltpu.VMEM((CHA, 128), jnp.float32),
        pltpu.VMEM((CHA, 128), jnp.float32),
        pltpu.VMEM((CHA, 128), jnp.float32),
        pltpu.VMEM((CHA, 128), jnp.float32),
        pltpu.VMEM((CHA8, 128), jnp.float32),
        pltpu.VMEM_SHARED((N, 128), jnp.float32),
        pltpu.SemaphoreType.DMA,
        pltpu.SemaphoreType.DMA,
        pltpu.SemaphoreType.DMA,
    ],
)


def _make_sc_b(hpc):
    vph = 8 // hpc

    def body(src_hbm, dst_hbm, ha_hbm, hb_hbm, ex_hbm,
             out_hbm, sidx0, didx0, sidx1, didx1, rows0, rows1, exb0, exb1,
             acc_sh, sem0, sem1):
        core = lax.axis_index("c")
        sub = lax.axis_index("s")
        _zero_acc(rows0, acc_sh, sub)
        plsc.subcore_barrier()

        trips = (NCHUNK - sub + NSUB - 1) // NSUB
        zi = jnp.zeros((16,), jnp.int32)
        bufs = [(rows0, exb0, sidx0, didx0, sem0),
                (rows1, exb1, sidx1, didx1, sem1)]

        def fire_idx(b, kk):
            rows, exb, sidx, didx, sem = bufs[b]
            base = (sub + kk * NSUB) * CH
            pltpu.async_copy(src_hbm.at[pl.ds(base, CH)], sidx, sem)
            pltpu.async_copy(dst_hbm.at[pl.ds(base, CH)], didx, sem)

        def wait_idx(b, kk):
            rows, exb, sidx, didx, sem = bufs[b]
            base = (sub + kk * NSUB) * CH
            pltpu.make_async_copy(src_hbm.at[pl.ds(base, CH)], sidx,
                                  sem).wait()
            pltpu.make_async_copy(dst_hbm.at[pl.ds(base, CH)], didx,
                                  sem).wait()

        def issue(b, kk):
            rows, exb, sidx, didx, sem = bufs[b]

            @pl.when(core == 0)
            def _():
                pltpu.async_copy(ha_hbm.at[sidx], rows, sem)

            @pl.when(core == 1)
            def _():
                pltpu.async_copy(hb_hbm.at[sidx], rows, sem)

            ci = sub + kk * NSUB
            pltpu.async_copy(ex_hbm.at[pl.ds(ci * CH8, CH8)], exb, sem)

        fire_idx(0, 0)
        wait_idx(0, 0)
        issue(0, 0)

        def step(b, kk, _):
            rows, exb, sidx, didx, sem = bufs[b]
            rown, exbn, sidxn, didxn, semn = bufs[1 - b]

            @pl.when(kk >= 1)
            def _():
                pltpu.make_async_copy(rown, acc_sh.at[didxn], semn).wait()

            @pl.when(kk + 1 < trips)
            def _():
                fire_idx(1 - b, kk + 1)

            pltpu.make_async_copy(ha_hbm.at[sidx], rows, sem).wait()
            ci = sub + kk * NSUB
            pltpu.make_async_copy(ex_hbm.at[pl.ds(ci * CH8, CH8)], exb,
                                  sem).wait()

            hoff = hpc * core if hpc > 1 else 0

            def ew(i, _):
                ws = []
                for u in range(4):
                    e = i * 4 + u
                    r8 = zi + (e >> 3)
                    l0 = zi + ((e & 7) * 16 + hoff)
                    ws.append([plsc.load_gather(exb, [r8, l0 + j])
                               for j in range(hpc)])
                for u in range(4):
                    e = i * 4 + u
                    for j in range(hpc):
                        for v in range(vph):
                            col = (j * vph + v) * 16
                            rows[e, pl.ds(col, 16)] = \
                                rows[e, pl.ds(col, 16)] * ws[u][j]
                return 0

            lax.fori_loop(0, CH // 4, ew, 0)
            pltpu.async_copy(rows, acc_sh.at[didx], sem, add=True)

            @pl.when(kk + 1 < trips)
            def _():
                wait_idx(1 - b, kk + 1)
                issue(1 - b, kk + 1)

            return 0

        def loop(kk, _):
            @pl.when(kk % 2 == 0)
            def _():
                step(0, kk, 0)

            @pl.when(kk % 2 == 1)
            def _():
                step(1, kk, 0)

            return 0

        lax.fori_loop(0, trips, loop, 0)

        @pl.when((trips - 1) % 2 == 0)
        def _():
            pltpu.make_async_copy(rows0, acc_sh.at[didx0], sem0).wait()

        @pl.when((trips - 1) % 2 == 1)
        def _():
            pltpu.make_async_copy(rows1, acc_sh.at[didx1], sem1).wait()

        plsc.subcore_barrier()
        _drain_acc(acc_sh, out_hbm, core, sub, sem0)

    return pl.kernel(
        body,
        out_type=jax.ShapeDtypeStruct((NCORE, N, 128), jnp.float32),
        mesh=_mesh,
        compiler_params=_sc_params,
        scratch_types=[
            pltpu.VMEM((CH,), jnp.int32),
            pltpu.VMEM((CH,), jnp.int32),
            pltpu.VMEM((CH,), jnp.int32),
            pltpu.VMEM((CH,), jnp.int32),
            pltpu.VMEM((CH, 128), jnp.float32),
            pltpu.VMEM((CH, 128), jnp.float32),
            pltpu.VMEM((CH8, 128), jnp.float32),
            pltpu.VMEM((CH8, 128), jnp.float32),
            pltpu.VMEM_SHARED((N, 128), jnp.float32),
            pltpu.SemaphoreType.DMA,
            pltpu.SemaphoreType.DMA,
        ],
    )


_sc_b1 = _make_sc_b(4)
_sc_b23 = _make_sc_b(1)


def _sc_a3_body(src_hbm, dst_hbm, q_hbm, k_hbm, ex_hbm, den_hbm,
                sidx0, didx0, sidx1, didx1, qb0, kb0, qb1, kb1, exs, bufex,
                alb, tmp, acc_sh, sem0, sem1, sem2):
    core = lax.axis_index("c")
    sub = lax.axis_index("s")
    wid = core * NSUB + sub
    _zero_acc(qb0, acc_sh, sub)
    _zero_vbuf(exs, CHA)
    plsc.subcore_barrier()

    trips = (NCHUNKA - wid + NW - 1) // NW
    lanes = lax.iota(jnp.int32, 16)
    bufs = [(qb0, kb0, sidx0, didx0, sem0), (qb1, kb1, sidx1, didx1, sem1)]

    def fire_idx(b, kk):
        qb, kb, sidx, didx, sem = bufs[b]
        base = (wid + kk * NW) * CHA
        pltpu.async_copy(src_hbm.at[pl.ds(base, CHA)], sidx, sem)
        pltpu.async_copy(dst_hbm.at[pl.ds(base, CHA)], didx, sem)

    def wait_idx(b, kk):
        qb, kb, sidx, didx, sem = bufs[b]
        base = (wid + kk * NW) * CHA
        pltpu.make_async_copy(src_hbm.at[pl.ds(base, CHA)], sidx,
                              sem).wait()
        pltpu.make_async_copy(dst_hbm.at[pl.ds(base, CHA)], didx,
                              sem).wait()

    def issue(b):
        qb, kb, sidx, didx, sem = bufs[b]
        pltpu.async_copy(q_hbm.at[didx], qb, sem)
        pltpu.async_copy(k_hbm.at[sidx], kb, sem)

    fire_idx(0, 0)
    wait_idx(0, 0)
    issue(0)

    def step(b, kk, _):
        qb, kb, sidx, didx, sem = bufs[b]

        @pl.when(kk + 1 < trips)
        def _():
            fire_idx(1 - b, kk + 1)

        pltpu.make_async_copy(q_hbm.at[didx], qb, sem).wait()
        pltpu.make_async_copy(k_hbm.at[sidx], kb, sem).wait()

        def dot_edge(i, _):
            for u in range(4):
                e = i * 4 + u
                pr = [qb[e, pl.ds(v * 16, 16)] * kb[e, pl.ds(v * 16, 16)]
                      for v in range(8)]
                acc = ((pr[0] + pr[1]) + (pr[2] + pr[3])) + \
                      ((pr[4] + pr[5]) + (pr[6] + pr[7]))
                tmp[e & 15, :] = acc
            return 0

        for g in range(CHA // 16):
            lax.fori_loop(g * 4, g * 4 + 4, dot_edge, 0)
            red = plsc.load_gather(tmp, [lanes, jnp.zeros((16,), jnp.int32)])
            for c in range(1, 16):
                red = red + plsc.load_gather(
                    tmp, [lanes, jnp.zeros((16,), jnp.int32) + c])
            alb[pl.ds(g * 16, 16)] = red

        for g in range(CHA // 16):
            ev = jnp.exp(alb[pl.ds(g * 16, 16)])
            plsc.store_scatter(exs, [g * 16 + lanes,
                                     jnp.zeros((16,), jnp.int32)], ev)
            plsc.store_scatter(bufex, [(g * 16 + lanes) >> 3,
                                       ((g * 16 + lanes) & 7) * 16], ev)

        ci = wid + kk * NW
        dex = pltpu.async_copy(bufex, ex_hbm.at[pl.ds(ci * CHA8, CHA8)],
                               sem2)
        pltpu.sync_copy(exs, acc_sh.at[didx], add=True)
        dex.wait()

        @pl.when(kk + 1 < trips)
        def _():
            wait_idx(1 - b, kk + 1)
            issue(1 - b)

        return 0

    def loop(kk, _):
        @pl.when(kk % 2 == 0)
        def _():
            step(0, kk, 0)

        @pl.when(kk % 2 == 1)
        def _():
            step(1, kk, 0)

        return 0

    lax.fori_loop(0, trips, loop, 0)
    plsc.subcore_barrier()
    _drain_acc(acc_sh, den_hbm, core, sub, sem0)


_sc_a3 = pl.kernel(
    _sc_a3_body,
    out_type=[
        jax.ShapeDtypeStruct((EX_ROWS, 128), jnp.float32),
        jax.ShapeDtypeStruct((NCORE, N, 128), jnp.float32),
    ],
    mesh=_mesh,
    compiler_params=_sc_params,
    scratch_types=[
        pltpu.VMEM((CHA,), jnp.int32),
        pltpu.VMEM((CHA,), jnp.int32),
        pltpu.VMEM((CHA,), jnp.int32),
        pltpu.VMEM((CHA,), jnp.int32),
        pltpu.VMEM((CHA, 128), jnp.float32),
        pltpu.VMEM((CHA, 128), jnp.float32),
        pltpu.VMEM((CHA, 128), jnp.float32),
        pltpu.VMEM((CHA, 128), jnp.float32),
        pltpu.VMEM((CHA, 128), jnp.float32),
        pltpu.VMEM((CHA8, 128), jnp.float32),
        pltpu.VMEM((CHA,), jnp.float32),
        pltpu.VMEM((16, 16), jnp.float32),
        pltpu.VMEM_SHARED((N, 128), jnp.float32),
        pltpu.SemaphoreType.DMA,
        pltpu.SemaphoreType.DMA,
        pltpu.SemaphoreType.DMA,
    ],
)


def _leaky(v):
    return jnp.maximum(v, 0.2 * v)


def _den_slice(denp, h):
    return denp[0, :, 0:h] + denp[1, :, 0:h]


def kernel(x, edge_index, W1, att_src1, att_dst1, b1, W2, att_src2, att_dst2,
           b2, Wq, bq, Wk, bk, Wv, bv, Wskip, bskip, Wres, bres):
    src = edge_index[0]
    dst = edge_index[1]
    f32 = jnp.float32

    heads1, ch1 = att_src1.shape
    eye1 = jnp.eye(heads1, dtype=f32)
    blk_s = (att_src1[:, :, None] * eye1[:, None, :]).reshape(HID, heads1)
    blk_d = (att_dst1[:, :, None] * eye1[:, None, :]).reshape(HID, heads1)
    A1 = jnp.zeros((HID, 128), f32)
    A1 = A1.at[:, 0:8].set(blk_s).at[:, 16:24].set(blk_d)
    A2 = jnp.zeros((HID, 128), f32)
    A2 = A2.at[:, 0:1].set(att_src2.T).at[:, 16:17].set(att_dst2.T)

    scale = 1.0 / jnp.sqrt(jnp.float32(NC))
    Wcat = jnp.concatenate([Wq * scale, Wk, Wv, Wskip], axis=1)
    bcat = jnp.tile(jnp.concatenate([bq * scale, bk, bv, bskip])[None, :],
                    (8, 1))
    b1p = jnp.tile(b1[None, :], (8, 1))
    b2p = jnp.tile(b2[None, :], (8, 1))
    bresp = jnp.tile(bres[None, :], (8, 1))

    h1a, h1b, a1 = _k1(x, W1, A1)
    asrc1 = a1[:, 0:8]
    adst1 = a1[:, 16:24]
    gmax1 = jnp.max(asrc1, axis=0, keepdims=True)
    c1 = _leaky(adst1 + gmax1)
    exs1 = jnp.exp(_leaky(asrc1 + adst1) - c1)
    T1 = a1.at[:, 32:40].set(c1)

    ex1, den1p = _sc_a(src, dst, T1)
    den1 = _den_slice(den1p, 8) + exs1
    inv1 = 1.0 / (den1 + 1e-16)
    exsx1 = jnp.repeat(exs1, ch1, axis=1)
    invx1 = jnp.repeat(inv1, ch1, axis=1)

    scat1 = _sc_b1(src, dst, h1a, h1b, ex1)

    h2a, h2b, a2 = _k2(scat1[0], scat1[1], h1a, h1b, exsx1, invx1, b1p,
                       W2, A2)
    asrc2 = a2[:, 0:1]
    adst2 = a2[:, 16:17]
    gmax2 = jnp.max(asrc2, axis=0, keepdims=True)
    c2 = _leaky(adst2 + gmax2)
    exs2 = jnp.exp(_leaky(asrc2 + adst2) - c2)
    T2 = a2.at[:, 32:33].set(c2)

    ex2, den2p = _sc_a(src, dst, T2)
    den2 = _den_slice(den2p, 1) + exs2
    inv2 = 1.0 / (den2 + 1e-16)
    exsx2 = jnp.broadcast_to(exs2, (N, HID))
    invx2 = jnp.broadcast_to(inv2, (N, HID))

    scat2 = _sc_b23(src, dst, h2a, h2b, ex2)

    q3, k3, v3, skip3 = _k3(scat2[0], scat2[1], h2a, h2b, exsx2, invx2, b2p,
                            Wcat, bcat)

    ex3, den3p = _sc_a3(src, dst, q3, k3)
    den3 = _den_slice(den3p, 1)
    inv3 = 1.0 / (den3 + 1e-16)
    invx3 = jnp.broadcast_to(inv3, (N, 128))

    v3a = jnp.pad(v3[:, 0:64], ((0, 0), (0, 64)))
    v3b = jnp.pad(v3[:, 64:128], ((0, 0), (0, 64)))
    scat3 = _sc_b23(src, dst, v3a, v3b, ex3)

    return _k4(scat3[0], scat3[1], invx3, skip3, x, Wres, bresp)
